# Initial kernel scaffold; baseline (speedup 1.0000x reference)
import jax
import jax.numpy as jnp
from jax.experimental import pallas as pl


def kernel(x, edge_index, W_enc, b_enc, W0, b0, W1, b1):
    raise NotImplementedError("write your pallas kernel here")


# speedup vs baseline: 13.3084x; 13.3084x over previous
import functools

import jax
import jax.numpy as jnp
from jax import lax
from jax.experimental import pallas as pl
from jax.experimental.pallas import tpu as pltpu
from jax.experimental.pallas import tpu_sc as plsc

N = 10000
E = 320000
NC, NS = 2, 16
NW = NC * NS
EB = 128
K = -(-E // (NW * EB))
EPAD = NW * K * EB
NP = 10112
JUNK = 10008
RPW = NP // NS
T64 = RPW // 64
REM = RPW - T64 * 64

_mesh = plsc.VectorSubcoreMesh(
    core_axis_name="c", subcore_axis_name="s", num_cores=NC, num_subcores=NS)

_f32 = jnp.float32
_sc_params = pltpu.CompilerParams(use_tc_tiling_on_sc=False)
_HI = jax.lax.Precision.HIGHEST


def _vdot(a, b):
  return jnp.sum(a * b)


def _fill(ref, rows, cols, value):
  def body(i, _):
    r = i // (cols // 16)
    c0 = (i % (cols // 16)) * 16
    ref[r, pl.ds(c0, 16)] = jnp.full((16,), value, _f32)
    return _
  lax.fori_loop(0, rows * (cols // 16), body, 0)


def _zero_acc(acc, s, zero_v):
  base = s * RPW
  for t in range(T64):
    pltpu.sync_copy(zero_v, acc.at[pl.ds(base + t * 64, 64)])
  pltpu.sync_copy(zero_v.at[pl.ds(0, REM)],
                  acc.at[pl.ds(base + T64 * 64, REM)])


def _writeback(acc, out, c, s):
  base = s * RPW
  pltpu.sync_copy(acc.at[pl.ds(base, RPW)], out.at[c, pl.ds(base, RPW)])


@functools.partial(
    pl.kernel,
    out_type=(jax.ShapeDtypeStruct((NC, NP, 16), _f32),
              jax.ShapeDtypeStruct((NC, NP, 16), _f32)),
    mesh=_mesh,
    compiler_params=_sc_params,
    scratch_types=[
        pltpu.VMEM((K, EB), jnp.int32),
        pltpu.VMEM((K, EB), jnp.int32),
        pltpu.VMEM((EB, 16), _f32),
        pltpu.VMEM((64, 16), _f32),
        pltpu.VMEM_SHARED((NP, 16), _f32),
        pltpu.VMEM_SHARED((NP, 16), _f32),
    ],
)
def _deg_kernel(src_hbm, dst_hbm, dout_hbm, din_hbm,
                sidx, didx, ones_v, zero16, acc_o, acc_i):
  c = lax.axis_index("c")
  s = lax.axis_index("s")
  w = s * NC + c
  pltpu.sync_copy(src_hbm.at[w], sidx)
  pltpu.sync_copy(dst_hbm.at[w], didx)
  _fill(ones_v, EB, 16, 1.0)
  _fill(zero16, 64, 16, 0.0)
  base = s * RPW
  for t in range(T64):
    pltpu.sync_copy(zero16, acc_o.at[pl.ds(base + t * 64, 64)])
    pltpu.sync_copy(zero16, acc_i.at[pl.ds(base + t * 64, 64)])
  pltpu.sync_copy(zero16.at[pl.ds(0, REM)], acc_o.at[pl.ds(base + T64 * 64, REM)])
  pltpu.sync_copy(zero16.at[pl.ds(0, REM)], acc_i.at[pl.ds(base + T64 * 64, REM)])
  plsc.subcore_barrier()

  def body(k, _):
    pltpu.sync_copy(ones_v, acc_o.at[sidx.at[k]], add=True)
    pltpu.sync_copy(ones_v, acc_i.at[didx.at[k]], add=True)
    return _
  lax.fori_loop(0, K, body, 0)
  plsc.subcore_barrier()
  pltpu.sync_copy(acc_o.at[pl.ds(base, RPW)], dout_hbm.at[c, pl.ds(base, RPW)])
  pltpu.sync_copy(acc_i.at[pl.ds(base, RPW)], din_hbm.at[c, pl.ds(base, RPW)])


@functools.partial(
    pl.kernel,
    out_type=(jax.ShapeDtypeStruct((NC, NP, 16), _f32),
              jax.ShapeDtypeStruct((NC, NP, 16), _f32)),
    mesh=_mesh,
    compiler_params=_sc_params,
    scratch_types=[
        pltpu.VMEM((K, EB), jnp.int32),
        pltpu.VMEM((K, EB), jnp.int32),
        pltpu.VMEM((EB, 16), _f32),
        pltpu.VMEM((64, 16), _f32),
        pltpu.VMEM_SHARED((NP, 16), _f32),
        pltpu.VMEM_SHARED((NP, 16), _f32),
        pltpu.SemaphoreType.DMA,
    ],
)
def _vpass_kernel(v16_hbm, src_hbm, dst_hbm, win_hbm, wout_hbm,
                  sidx, didx, rows, zero16, acc_f, acc_r, sem):
  c = lax.axis_index("c")
  s = lax.axis_index("s")
  w = s * NC + c
  pltpu.sync_copy(src_hbm.at[w], sidx)
  pltpu.sync_copy(dst_hbm.at[w], didx)
  _fill(zero16, 64, 16, 0.0)
  base = s * RPW
  for t in range(T64):
    pltpu.sync_copy(zero16, acc_f.at[pl.ds(base + t * 64, 64)])
    pltpu.sync_copy(zero16, acc_r.at[pl.ds(base + t * 64, 64)])
  pltpu.sync_copy(zero16.at[pl.ds(0, REM)], acc_f.at[pl.ds(base + T64 * 64, REM)])
  pltpu.sync_copy(zero16.at[pl.ds(0, REM)], acc_r.at[pl.ds(base + T64 * 64, REM)])
  plsc.subcore_barrier()

  def body(k, _):
    pltpu.async_copy(v16_hbm.at[sidx.at[k]], rows, sem).wait()
    pltpu.sync_copy(rows, acc_f.at[didx.at[k]], add=True)
    pltpu.async_copy(v16_hbm.at[didx.at[k]], rows, sem).wait()
    pltpu.sync_copy(rows, acc_r.at[sidx.at[k]], add=True)
    return _
  lax.fori_loop(0, K, body, 0)
  plsc.subcore_barrier()
  pltpu.sync_copy(acc_f.at[pl.ds(base, RPW)], win_hbm.at[c, pl.ds(base, RPW)])
  pltpu.sync_copy(acc_r.at[pl.ds(base, RPW)], wout_hbm.at[c, pl.ds(base, RPW)])


@functools.lru_cache(maxsize=None)
def _make_block_kernel(nblocks):
  out_type = tuple(jax.ShapeDtypeStruct((NC, NP, 128), _f32)
                   for _ in range(nblocks))

  @functools.partial(
      pl.kernel,
      out_type=out_type,
      mesh=_mesh,
      compiler_params=_sc_params,
      scratch_types=[
          pltpu.VMEM((K, EB), jnp.int32),
          pltpu.VMEM((K, EB), jnp.int32),
          pltpu.VMEM((EB, 128), _f32),
          pltpu.VMEM((64, 128), _f32),
          pltpu.VMEM_SHARED((NP, 128), _f32),
          pltpu.SemaphoreType.DMA,
      ],
  )
  def _block_kernel(*refs):
    z_hbm = refs[:nblocks]
    src_hbm, dst_hbm = refs[nblocks], refs[nblocks + 1]
    outs = refs[nblocks + 2:2 * nblocks + 2]
    sidx, didx, rows, zero_v, acc, sem = refs[2 * nblocks + 2:]
    c = lax.axis_index("c")
    s = lax.axis_index("s")
    w = s * NC + c
    pltpu.sync_copy(src_hbm.at[w], sidx)
    pltpu.sync_copy(dst_hbm.at[w], didx)
    _fill(zero_v, 64, 128, 0.0)
    for b in range(nblocks):
      _zero_acc(acc, s, zero_v)
      plsc.subcore_barrier()

      def body(k, _, zb=z_hbm[b]):
        pltpu.async_copy(zb.at[sidx.at[k]], rows, sem).wait()
        pltpu.sync_copy(rows, acc.at[didx.at[k]], add=True)
        return _
      lax.fori_loop(0, K, body, 0)
      plsc.subcore_barrier()
      _writeback(acc, outs[b], c, s)
      plsc.subcore_barrier()

  return _block_kernel


def _msgpass(blocks, src_t, dst_t):
  kern = _make_block_kernel(len(blocks))
  outs = kern(*blocks, src_t, dst_t)
  return [(o[0] + o[1])[:N] for o in outs]


NBLK = 25
BR = N // NBLK


def _dot(a, b):
  return jnp.dot(a, b, precision=_HI)


@functools.lru_cache(maxsize=None)
def _make_stage_kernel(encoder, with_next, with_rank, ncross):
  def body(*refs):
    idx = 0
    if encoder:
      x_ref, W_ref, b_ref = refs[0:3]
    else:
      rc_ref, xhp_ref, b_ref = refs[0:3]
    idx = 3
    dinv_ref, dout_inv_ref = refs[idx:idx + 2]; idx += 2
    if with_next:
      Wn_ref = refs[idx]; idx += 1
    cross_in = []
    for _ in range(ncross):
      cross_in.append((refs[idx], refs[idx + 1])); idx += 2
    h_ref, n_ref, rs_ref = refs[idx:idx + 3]; idx += 3
    if with_next:
      xh_ref, zc_ref = refs[idx:idx + 2]; idx += 2
    zs_ref = refs[idx]; idx += 1
    if with_rank:
      G_ref, cs_ref = refs[idx:idx + 2]; idx += 2
    cross_out = refs[idx:idx + ncross]

    i = pl.program_id(0)
    dinv = dinv_ref[...]
    if encoder:
      h = _dot(x_ref[...], W_ref[...]) + b_ref[...]
    else:
      pre = dinv * (rc_ref[0] + rc_ref[1]) + dinv * dinv * xhp_ref[...] + b_ref[...]
      h = jnp.maximum(pre, 0.0)
    h_ref[...] = h
    n_ref[...] = jnp.sum(h * h, axis=1, keepdims=True)
    rs_ref[...] = jnp.sum(jnp.abs(h), axis=1, keepdims=True)
    if with_next:
      xh = _dot(h, Wn_ref[...])
      xh_ref[...] = xh
      zc_ref[...] = dinv * xh
    zs_ref[...] = h * dout_inv_ref[...]

    @pl.when(i == 0)
    def _init():
      if with_rank:
        G_ref[...] = jnp.zeros_like(G_ref)
        cs_ref[...] = jnp.zeros_like(cs_ref)
      for co in cross_out:
        co[0, 0] = 0.0

    if with_rank:
      G_ref[...] += _dot(h.T, h)
      cs_ref[...] += jnp.sum(jnp.abs(h), axis=0, keepdims=True)
    for (zp_ref, r_ref), co in zip(cross_in, cross_out):
      co[0, 0] += jnp.sum(zp_ref[...] * (r_ref[0] + r_ref[1]))

  row = lambda i: (i, 0)
  full = lambda i: (0, 0)
  full3 = lambda i: (0, i, 0)
  in_specs = []
  if encoder:
    in_specs += [pl.BlockSpec((BR, 128), row), pl.BlockSpec((128, 128), full),
                 pl.BlockSpec((1, 128), full)]
  else:
    in_specs += [pl.BlockSpec((2, BR, 128), full3), pl.BlockSpec((BR, 128), row),
                 pl.BlockSpec((1, 128), full)]
  in_specs += [pl.BlockSpec((BR, 1), row)] * 2
  if with_next:
    in_specs += [pl.BlockSpec((128, 128), full)]
  for _ in range(ncross):
    in_specs += [pl.BlockSpec((BR, 128), row), pl.BlockSpec((2, BR, 128), full3)]

  out_specs = [pl.BlockSpec((BR, 128), row), pl.BlockSpec((BR, 1), row),
               pl.BlockSpec((BR, 1), row)]
  out_shape = [jax.ShapeDtypeStruct((N, 128), _f32),
               jax.ShapeDtypeStruct((N, 1), _f32),
               jax.ShapeDtypeStruct((N, 1), _f32)]
  if with_next:
    out_specs += [pl.BlockSpec((BR, 128), row)] * 2
    out_shape += [jax.ShapeDtypeStruct((N, 128), _f32)] * 2
  out_specs += [pl.BlockSpec((BR, 128), row)]
  out_shape += [jax.ShapeDtypeStruct((N, 128), _f32)]
  if with_rank:
    out_specs += [pl.BlockSpec((128, 128), full), pl.BlockSpec((1, 128), full)]
    out_shape += [jax.ShapeDtypeStruct((128, 128), _f32),
                  jax.ShapeDtypeStruct((1, 128), _f32)]
  out_specs += [pl.BlockSpec((1, 1), full, memory_space=pltpu.SMEM)] * ncross
  out_shape += [jax.ShapeDtypeStruct((1, 1), _f32)] * ncross

  return pl.pallas_call(body, grid=(NBLK,), in_specs=in_specs,
                        out_specs=out_specs, out_shape=out_shape)


def _cross2(zA, rA, zB, rB):
  def body(zA_ref, rA_ref, zB_ref, rB_ref, a_ref, b_ref):
    i = pl.program_id(0)

    @pl.when(i == 0)
    def _init():
      a_ref[0, 0] = 0.0
      b_ref[0, 0] = 0.0
    a_ref[0, 0] += jnp.sum(zA_ref[...] * (rA_ref[0] + rA_ref[1]))
    b_ref[0, 0] += jnp.sum(zB_ref[...] * (rB_ref[0] + rB_ref[1]))

  row = lambda i: (i, 0)
  full3 = lambda i: (0, i, 0)
  smem = pl.BlockSpec((1, 1), lambda i: (0, 0), memory_space=pltpu.SMEM)
  out = pl.pallas_call(
      body, grid=(NBLK,),
      in_specs=[pl.BlockSpec((BR, 128), row), pl.BlockSpec((2, BR, 128), full3),
                pl.BlockSpec((BR, 128), row), pl.BlockSpec((2, BR, 128), full3)],
      out_specs=[smem, smem],
      out_shape=[jax.ShapeDtypeStruct((1, 1), _f32)] * 2)(zA, rA, zB, rB)
  return out[0][0, 0], out[1][0, 0]


NS_ITERS = 25


def _rank_kernel(G, g, r, cnorm2, rnorm2, sgn):
  def body(G_ref, g_ref, r_ref, sc_ref, I_ref, nu_ref, rank_ref):
    I = I_ref[...]

    def trsqrt(S):
      c = jnp.sum(S * I)
      Y, Z = S / c, I
      for _ in range(NS_ITERS):
        T = 0.5 * (3.0 * I - _dot(Z, Y))
        Y, Z = _dot(Y, T), _dot(T, Z)
      return jnp.sum(Y * I) * jnp.sqrt(c)

    G = G_ref[...]
    nu1 = trsqrt(G)
    nu_ref[0, 0] = nu1
    g_ = g_ref[...]
    r_ = r_ref[...]
    cn2, rn2, sg = sc_ref[0], sc_ref[1], sc_ref[2]
    gr = _dot(g_.T, r_) + _dot(r_.T, g_)
    M = (G / (nu1 * nu1)
         - (sg / (nu1 * jnp.sqrt(cn2 * rn2))) * gr
         + _dot(r_.T, r_) / rn2)
    rank_ref[0, 0] = trsqrt(M)

  nu, rank = pl.pallas_call(
      body,
      in_specs=[pl.BlockSpec(memory_space=pltpu.VMEM),
                pl.BlockSpec(memory_space=pltpu.VMEM),
                pl.BlockSpec(memory_space=pltpu.VMEM),
                pl.BlockSpec(memory_space=pltpu.SMEM),
                pl.BlockSpec(memory_space=pltpu.VMEM)],
      out_specs=[pl.BlockSpec(memory_space=pltpu.SMEM),
                 pl.BlockSpec(memory_space=pltpu.SMEM)],
      out_shape=[jax.ShapeDtypeStruct((1, 1), _f32)] * 2,
  )(G, g.reshape(1, 128), r.reshape(1, 128), jnp.stack([cnorm2, rnorm2, sgn]),
    jnp.eye(128, dtype=_f32))
  return rank[0, 0]


def _rank_diff(h, G, rs, cs):
  i = jnp.argmax(rs[:, 0])
  j = jnp.argmax(cs[0, :])
  r = lax.dynamic_slice(h, (i, 0), (1, 128))[0]
  g = lax.dynamic_slice(G, (0, j), (128, 1))[:, 0]
  cnorm2 = G[j, j]
  rnorm2 = jnp.sum(r * r)
  sgn = jnp.where(h[i, j] < 0, -1.0, 1.0)
  return _rank_kernel(G, g, r, cnorm2, rnorm2, sgn)


def kernel(x, edge_index, W_enc, b_enc, W0, b0, W1, b1):
  src = edge_index[0]
  dst = edge_index[1]
  pad = EPAD - E
  src_t = jnp.concatenate([src, jnp.zeros((pad,), jnp.int32)]).reshape(NW, K, EB)
  dst_t = jnp.concatenate([dst, jnp.full((pad,), JUNK, jnp.int32)]).reshape(NW, K, EB)

  dout_p, din_p = _deg_kernel(src_t, dst_t)
  dout = (dout_p[0] + dout_p[1])[:N, 0] + 1.0
  dout = dout.at[0].add(-float(pad))
  din = (din_p[0] + din_p[1])[:N, 0] + 1.0
  v = lax.rsqrt(dout)
  dinv = lax.rsqrt(din)
  dout_inv = 1.0 / dout
  col = lambda a: a.reshape(N, 1)

  v16 = jnp.zeros((NP, 16), _f32).at[:N, :].set(v[:, None])
  win_p, wout_p = _vpass_kernel(v16, src_t, dst_t)
  wsum = v * ((win_p[0] + win_p[1])[:N, 0] + (wout_p[0] + wout_p[1])[:N, 0])
  degsum = (dout - 1.0) + (din - 1.0)

  def e_rw(n, nf, cross):
    return 0.5 * (_vdot(degsum, n) - 2.0 * cross) / nf

  def e_sym(n, nf, cross):
    return 0.5 * (_vdot(n * dout_inv, wsum) - 2.0 * cross) / nf

  enc = _make_stage_kernel(True, True, False, 0)
  x0, n0c, rs0, xh0, zc0, zs0 = enc(x, W_enc, b_enc.reshape(1, 128),
                                    col(dinv), col(dout_inv), W0)
  n0 = n0c[:, 0]
  nf0 = jnp.sum(n0)
  rc0, rsym0 = _make_block_kernel(2)(zc0, zs0, src_t, dst_t)

  st1 = _make_stage_kernel(False, True, True, 1)
  (h1, n1c, rs1, xh1, zc1, zs1, G1, cs1, x_sym0) = st1(
      rc0, xh0, b0.reshape(1, 128), col(dinv), col(dout_inv), W1, zs0, rsym0)
  n1 = n1c[:, 0]
  nf1 = jnp.sum(n1)
  e0 = e_sym(n0, nf0, x_sym0[0, 0])
  rc1, rrw1, rsym1 = _make_block_kernel(3)(zc1, h1, zs1, src_t, dst_t)
  rank1 = _rank_diff(h1, G1, rs1, cs1)

  st2 = _make_stage_kernel(False, False, True, 2)
  (h2, n2c, rs2, zs2, G2, cs2, x_rw1, x_sym1) = st2(
      rc1, xh1, b1.reshape(1, 128), col(dinv), col(dout_inv),
      h1, rrw1, zs1, rsym1)
  n2 = n2c[:, 0]
  nf2 = jnp.sum(n2)
  erw1 = e_rw(n1, nf1, x_rw1[0, 0])
  esym1 = e_sym(n1, nf1, x_sym1[0, 0])
  rrw2, rsym2 = _make_block_kernel(2)(h2, zs2, src_t, dst_t)
  rank2 = _rank_diff(h2, G2, rs2, cs2)
  x_rw2, x_sym2 = _cross2(h2, rrw2, zs2, rsym2)
  erw2 = e_rw(n2, nf2, x_rw2)
  esym2 = e_sym(n2, nf2, x_sym2)

  return (h2,
          jnp.stack([e0, erw1, erw2]),
          jnp.stack([esym1, esym2]),
          jnp.stack([rank1, rank2]))


# speedup vs baseline: 14.8014x; 1.1122x over previous
import functools

import jax
import jax.numpy as jnp
from jax import lax
from jax.experimental import pallas as pl
from jax.experimental.pallas import tpu as pltpu
from jax.experimental.pallas import tpu_sc as plsc

N = 10000
E = 320000
NC, NS = 2, 16
NW = NC * NS
EB = 64
K = -(-E // (NW * EB))
K = K + (K % 2)
EPAD = NW * K * EB
NP = 10112
JUNK = 10008
RPW = NP // NS
T64 = RPW // 64
REM = RPW - T64 * 64

_mesh = plsc.VectorSubcoreMesh(
    core_axis_name="c", subcore_axis_name="s", num_cores=NC, num_subcores=NS)

_f32 = jnp.float32
_sc_params = pltpu.CompilerParams(use_tc_tiling_on_sc=False)
_HI = jax.lax.Precision.HIGHEST


def _vdot(a, b):
  return jnp.sum(a * b)


def _fill(ref, rows, cols, value):
  def body(i, _):
    r = i // (cols // 16)
    c0 = (i % (cols // 16)) * 16
    ref[r, pl.ds(c0, 16)] = jnp.full((16,), value, _f32)
    return _
  lax.fori_loop(0, rows * (cols // 16), body, 0)


def _zero_acc(acc, s, zero_v):
  base = s * RPW
  for t in range(T64):
    pltpu.sync_copy(zero_v, acc.at[pl.ds(base + t * 64, 64)])
  pltpu.sync_copy(zero_v.at[pl.ds(0, REM)],
                  acc.at[pl.ds(base + T64 * 64, REM)])


def _writeback(acc, out, c, s):
  base = s * RPW
  pltpu.sync_copy(acc.at[pl.ds(base, RPW)], out.at[c, pl.ds(base, RPW)])


@functools.partial(
    pl.kernel,
    out_type=(jax.ShapeDtypeStruct((NC, NP, 16), _f32),
              jax.ShapeDtypeStruct((NC, NP, 16), _f32)),
    mesh=_mesh,
    compiler_params=_sc_params,
    scratch_types=[
        pltpu.VMEM((K, EB), jnp.int32),
        pltpu.VMEM((K, EB), jnp.int32),
        pltpu.VMEM((EB, 16), _f32),
        pltpu.VMEM((64, 16), _f32),
        pltpu.VMEM_SHARED((NP, 16), _f32),
        pltpu.VMEM_SHARED((NP, 16), _f32),
    ],
)
def _deg_kernel(src_hbm, dst_hbm, dout_hbm, din_hbm,
                sidx, didx, ones_v, zero16, acc_o, acc_i):
  c = lax.axis_index("c")
  s = lax.axis_index("s")
  w = s * NC + c
  pltpu.sync_copy(src_hbm.at[w], sidx)
  pltpu.sync_copy(dst_hbm.at[w], didx)
  _fill(ones_v, EB, 16, 1.0)
  _fill(zero16, 64, 16, 0.0)
  base = s * RPW
  for t in range(T64):
    pltpu.sync_copy(zero16, acc_o.at[pl.ds(base + t * 64, 64)])
    pltpu.sync_copy(zero16, acc_i.at[pl.ds(base + t * 64, 64)])
  pltpu.sync_copy(zero16.at[pl.ds(0, REM)], acc_o.at[pl.ds(base + T64 * 64, REM)])
  pltpu.sync_copy(zero16.at[pl.ds(0, REM)], acc_i.at[pl.ds(base + T64 * 64, REM)])
  plsc.subcore_barrier()

  def body(k, _):
    pltpu.sync_copy(ones_v, acc_o.at[sidx.at[k]], add=True)
    pltpu.sync_copy(ones_v, acc_i.at[didx.at[k]], add=True)
    return _
  lax.fori_loop(0, K, body, 0)
  plsc.subcore_barrier()
  pltpu.sync_copy(acc_o.at[pl.ds(base, RPW)], dout_hbm.at[c, pl.ds(base, RPW)])
  pltpu.sync_copy(acc_i.at[pl.ds(base, RPW)], din_hbm.at[c, pl.ds(base, RPW)])


@functools.partial(
    pl.kernel,
    out_type=(jax.ShapeDtypeStruct((NC, NP, 16), _f32),
              jax.ShapeDtypeStruct((NC, NP, 16), _f32)),
    mesh=_mesh,
    compiler_params=_sc_params,
    scratch_types=[
        pltpu.VMEM((K, EB), jnp.int32),
        pltpu.VMEM((K, EB), jnp.int32),
        pltpu.VMEM((EB, 16), _f32),
        pltpu.VMEM((64, 16), _f32),
        pltpu.VMEM_SHARED((NP, 16), _f32),
        pltpu.VMEM_SHARED((NP, 16), _f32),
        pltpu.SemaphoreType.DMA,
    ],
)
def _vpass_kernel(v16_hbm, src_hbm, dst_hbm, win_hbm, wout_hbm,
                  sidx, didx, rows, zero16, acc_f, acc_r, sem):
  c = lax.axis_index("c")
  s = lax.axis_index("s")
  w = s * NC + c
  pltpu.sync_copy(src_hbm.at[w], sidx)
  pltpu.sync_copy(dst_hbm.at[w], didx)
  _fill(zero16, 64, 16, 0.0)
  base = s * RPW
  for t in range(T64):
    pltpu.sync_copy(zero16, acc_f.at[pl.ds(base + t * 64, 64)])
    pltpu.sync_copy(zero16, acc_r.at[pl.ds(base + t * 64, 64)])
  pltpu.sync_copy(zero16.at[pl.ds(0, REM)], acc_f.at[pl.ds(base + T64 * 64, REM)])
  pltpu.sync_copy(zero16.at[pl.ds(0, REM)], acc_r.at[pl.ds(base + T64 * 64, REM)])
  plsc.subcore_barrier()

  def body(k, _):
    pltpu.async_copy(v16_hbm.at[sidx.at[k]], rows, sem).wait()
    pltpu.sync_copy(rows, acc_f.at[didx.at[k]], add=True)
    pltpu.async_copy(v16_hbm.at[didx.at[k]], rows, sem).wait()
    pltpu.sync_copy(rows, acc_r.at[sidx.at[k]], add=True)
    return _
  lax.fori_loop(0, K, body, 0)
  plsc.subcore_barrier()
  pltpu.sync_copy(acc_f.at[pl.ds(base, RPW)], win_hbm.at[c, pl.ds(base, RPW)])
  pltpu.sync_copy(acc_r.at[pl.ds(base, RPW)], wout_hbm.at[c, pl.ds(base, RPW)])


@functools.lru_cache(maxsize=None)
def _make_block_kernel(nblocks):
  out_type = tuple(jax.ShapeDtypeStruct((NC, NP, 128), _f32)
                   for _ in range(nblocks))

  @functools.partial(
      pl.kernel,
      out_type=out_type,
      mesh=_mesh,
      compiler_params=_sc_params,
      scratch_types=[
          pltpu.VMEM((K, EB), jnp.int32),
          pltpu.VMEM((K, EB), jnp.int32),
          pltpu.VMEM((EB, 128), _f32),
          pltpu.VMEM((EB, 128), _f32),
          pltpu.VMEM((64, 128), _f32),
          pltpu.VMEM_SHARED((NP, 128), _f32),
          pltpu.SemaphoreType.DMA,
          pltpu.SemaphoreType.DMA,
      ],
  )
  def _block_kernel(*refs):
    z_hbm = refs[:nblocks]
    src_hbm, dst_hbm = refs[nblocks], refs[nblocks + 1]
    outs = refs[nblocks + 2:2 * nblocks + 2]
    sidx, didx, rows0, rows1, zero_v, acc, sem0, sem1 = refs[2 * nblocks + 2:]
    c = lax.axis_index("c")
    s = lax.axis_index("s")
    w = s * NC + c
    pltpu.sync_copy(src_hbm.at[w], sidx)
    pltpu.sync_copy(dst_hbm.at[w], didx)
    _fill(zero_v, 64, 128, 0.0)
    for b in range(nblocks):
      _zero_acc(acc, s, zero_v)
      plsc.subcore_barrier()
      zb = z_hbm[b]
      pltpu.async_copy(zb.at[sidx.at[0]], rows0, sem0)

      def pair(t, _, zb=zb):
        k0 = 2 * t
        pltpu.async_copy(zb.at[sidx.at[k0 + 1]], rows1, sem1)
        pltpu.make_async_copy(zb.at[sidx.at[k0]], rows0, sem0).wait()
        pltpu.sync_copy(rows0, acc.at[didx.at[k0]], add=True)

        @pl.when(k0 + 2 < K)
        def _fire():
          pltpu.async_copy(zb.at[sidx.at[k0 + 2]], rows0, sem0)
        pltpu.make_async_copy(zb.at[sidx.at[k0 + 1]], rows1, sem1).wait()
        pltpu.sync_copy(rows1, acc.at[didx.at[k0 + 1]], add=True)
        return _
      lax.fori_loop(0, K // 2, pair, 0)
      if K % 2:
        pltpu.make_async_copy(zb.at[sidx.at[K - 1]], rows0, sem0).wait()
        pltpu.sync_copy(rows0, acc.at[didx.at[K - 1]], add=True)
      plsc.subcore_barrier()
      _writeback(acc, outs[b], c, s)
      plsc.subcore_barrier()

  return _block_kernel


def _msgpass(blocks, src_t, dst_t):
  kern = _make_block_kernel(len(blocks))
  outs = kern(*blocks, src_t, dst_t)
  return [(o[0] + o[1])[:N] for o in outs]


NBLK = 25
BR = N // NBLK


def _dot(a, b):
  return jnp.dot(a, b, precision=_HI)


@functools.lru_cache(maxsize=None)
def _make_stage_kernel(encoder, with_next, with_rank, ncross):
  def body(*refs):
    idx = 0
    if encoder:
      x_ref, W_ref, b_ref = refs[0:3]
    else:
      rc_ref, xhp_ref, b_ref = refs[0:3]
    idx = 3
    dinv_ref, dout_inv_ref = refs[idx:idx + 2]; idx += 2
    if with_next:
      Wn_ref = refs[idx]; idx += 1
    cross_in = []
    for _ in range(ncross):
      cross_in.append((refs[idx], refs[idx + 1])); idx += 2
    h_ref, n_ref, rs_ref = refs[idx:idx + 3]; idx += 3
    if with_next:
      xh_ref, zc_ref = refs[idx:idx + 2]; idx += 2
    zs_ref = refs[idx]; idx += 1
    if with_rank:
      G_ref, cs_ref = refs[idx:idx + 2]; idx += 2
    cross_out = refs[idx:idx + ncross]

    i = pl.program_id(0)
    dinv = dinv_ref[...]
    if encoder:
      h = _dot(x_ref[...], W_ref[...]) + b_ref[...]
    else:
      pre = dinv * (rc_ref[0] + rc_ref[1]) + dinv * dinv * xhp_ref[...] + b_ref[...]
      h = jnp.maximum(pre, 0.0)
    h_ref[...] = h
    n_ref[...] = jnp.sum(h * h, axis=1, keepdims=True)
    rs_ref[...] = jnp.sum(jnp.abs(h), axis=1, keepdims=True)
    if with_next:
      xh = _dot(h, Wn_ref[...])
      xh_ref[...] = xh
      zc_ref[...] = dinv * xh
    zs_ref[...] = h * dout_inv_ref[...]

    @pl.when(i == 0)
    def _init():
      if with_rank:
        G_ref[...] = jnp.zeros_like(G_ref)
        cs_ref[...] = jnp.zeros_like(cs_ref)
      for co in cross_out:
        co[0, 0] = 0.0

    if with_rank:
      G_ref[...] += _dot(h.T, h)
      cs_ref[...] += jnp.sum(jnp.abs(h), axis=0, keepdims=True)
    for (zp_ref, r_ref), co in zip(cross_in, cross_out):
      co[0, 0] += jnp.sum(zp_ref[...] * (r_ref[0] + r_ref[1]))

  row = lambda i: (i, 0)
  full = lambda i: (0, 0)
  full3 = lambda i: (0, i, 0)
  in_specs = []
  if encoder:
    in_specs += [pl.BlockSpec((BR, 128), row), pl.BlockSpec((128, 128), full),
                 pl.BlockSpec((1, 128), full)]
  else:
    in_specs += [pl.BlockSpec((2, BR, 128), full3), pl.BlockSpec((BR, 128), row),
                 pl.BlockSpec((1, 128), full)]
  in_specs += [pl.BlockSpec((BR, 1), row)] * 2
  if with_next:
    in_specs += [pl.BlockSpec((128, 128), full)]
  for _ in range(ncross):
    in_specs += [pl.BlockSpec((BR, 128), row), pl.BlockSpec((2, BR, 128), full3)]

  out_specs = [pl.BlockSpec((BR, 128), row), pl.BlockSpec((BR, 1), row),
               pl.BlockSpec((BR, 1), row)]
  out_shape = [jax.ShapeDtypeStruct((N, 128), _f32),
               jax.ShapeDtypeStruct((N, 1), _f32),
               jax.ShapeDtypeStruct((N, 1), _f32)]
  if with_next:
    out_specs += [pl.BlockSpec((BR, 128), row)] * 2
    out_shape += [jax.ShapeDtypeStruct((N, 128), _f32)] * 2
  out_specs += [pl.BlockSpec((BR, 128), row)]
  out_shape += [jax.ShapeDtypeStruct((N, 128), _f32)]
  if with_rank:
    out_specs += [pl.BlockSpec((128, 128), full), pl.BlockSpec((1, 128), full)]
    out_shape += [jax.ShapeDtypeStruct((128, 128), _f32),
                  jax.ShapeDtypeStruct((1, 128), _f32)]
  out_specs += [pl.BlockSpec((1, 1), full, memory_space=pltpu.SMEM)] * ncross
  out_shape += [jax.ShapeDtypeStruct((1, 1), _f32)] * ncross

  return pl.pallas_call(body, grid=(NBLK,), in_specs=in_specs,
                        out_specs=out_specs, out_shape=out_shape)


def _cross2(zA, rA, zB, rB):
  def body(zA_ref, rA_ref, zB_ref, rB_ref, a_ref, b_ref):
    i = pl.program_id(0)

    @pl.when(i == 0)
    def _init():
      a_ref[0, 0] = 0.0
      b_ref[0, 0] = 0.0
    a_ref[0, 0] += jnp.sum(zA_ref[...] * (rA_ref[0] + rA_ref[1]))
    b_ref[0, 0] += jnp.sum(zB_ref[...] * (rB_ref[0] + rB_ref[1]))

  row = lambda i: (i, 0)
  full3 = lambda i: (0, i, 0)
  smem = pl.BlockSpec((1, 1), lambda i: (0, 0), memory_space=pltpu.SMEM)
  out = pl.pallas_call(
      body, grid=(NBLK,),
      in_specs=[pl.BlockSpec((BR, 128), row), pl.BlockSpec((2, BR, 128), full3),
                pl.BlockSpec((BR, 128), row), pl.BlockSpec((2, BR, 128), full3)],
      out_specs=[smem, smem],
      out_shape=[jax.ShapeDtypeStruct((1, 1), _f32)] * 2)(zA, rA, zB, rB)
  return out[0][0, 0], out[1][0, 0]


NS_ITERS = 25


def _rank_kernel(G, g, r, cnorm2, rnorm2, sgn):
  def body(G_ref, g_ref, r_ref, sc_ref, I_ref, nu_ref, rank_ref):
    I = I_ref[...]

    def trsqrt(S):
      c = jnp.sum(S * I)
      Y, Z = S / c, I
      for _ in range(NS_ITERS):
        T = 0.5 * (3.0 * I - _dot(Z, Y))
        Y, Z = _dot(Y, T), _dot(T, Z)
      return jnp.sum(Y * I) * jnp.sqrt(c)

    G = G_ref[...]
    nu1 = trsqrt(G)
    nu_ref[0, 0] = nu1
    g_ = g_ref[...]
    r_ = r_ref[...]
    cn2, rn2, sg = sc_ref[0], sc_ref[1], sc_ref[2]
    gr = _dot(g_.T, r_) + _dot(r_.T, g_)
    M = (G / (nu1 * nu1)
         - (sg / (nu1 * jnp.sqrt(cn2 * rn2))) * gr
         + _dot(r_.T, r_) / rn2)
    rank_ref[0, 0] = trsqrt(M)

  nu, rank = pl.pallas_call(
      body,
      in_specs=[pl.BlockSpec(memory_space=pltpu.VMEM),
                pl.BlockSpec(memory_space=pltpu.VMEM),
                pl.BlockSpec(memory_space=pltpu.VMEM),
                pl.BlockSpec(memory_space=pltpu.SMEM),
                pl.BlockSpec(memory_space=pltpu.VMEM)],
      out_specs=[pl.BlockSpec(memory_space=pltpu.SMEM),
                 pl.BlockSpec(memory_space=pltpu.SMEM)],
      out_shape=[jax.ShapeDtypeStruct((1, 1), _f32)] * 2,
  )(G, g.reshape(1, 128), r.reshape(1, 128), jnp.stack([cnorm2, rnorm2, sgn]),
    jnp.eye(128, dtype=_f32))
  return rank[0, 0]


def _rank_diff(h, G, rs, cs):
  i = jnp.argmax(rs[:, 0])
  j = jnp.argmax(cs[0, :])
  r = lax.dynamic_slice(h, (i, 0), (1, 128))[0]
  g = lax.dynamic_slice(G, (0, j), (128, 1))[:, 0]
  cnorm2 = G[j, j]
  rnorm2 = jnp.sum(r * r)
  sgn = jnp.where(h[i, j] < 0, -1.0, 1.0)
  return _rank_kernel(G, g, r, cnorm2, rnorm2, sgn)


def kernel(x, edge_index, W_enc, b_enc, W0, b0, W1, b1):
  src = edge_index[0]
  dst = edge_index[1]
  pad = EPAD - E
  src_t = jnp.concatenate([src, jnp.zeros((pad,), jnp.int32)]).reshape(NW, K, EB)
  dst_t = jnp.concatenate([dst, jnp.full((pad,), JUNK, jnp.int32)]).reshape(NW, K, EB)

  dout_p, din_p = _deg_kernel(src_t, dst_t)
  dout = (dout_p[0] + dout_p[1])[:N, 0] + 1.0
  dout = dout.at[0].add(-float(pad))
  din = (din_p[0] + din_p[1])[:N, 0] + 1.0
  v = lax.rsqrt(dout)
  dinv = lax.rsqrt(din)
  dout_inv = 1.0 / dout
  col = lambda a: a.reshape(N, 1)

  v16 = jnp.zeros((NP, 16), _f32).at[:N, :].set(v[:, None])
  win_p, wout_p = _vpass_kernel(v16, src_t, dst_t)
  wsum = v * ((win_p[0] + win_p[1])[:N, 0] + (wout_p[0] + wout_p[1])[:N, 0])
  degsum = (dout - 1.0) + (din - 1.0)

  def e_rw(n, nf, cross):
    return 0.5 * (_vdot(degsum, n) - 2.0 * cross) / nf

  def e_sym(n, nf, cross):
    return 0.5 * (_vdot(n * dout_inv, wsum) - 2.0 * cross) / nf

  enc = _make_stage_kernel(True, True, False, 0)
  x0, n0c, rs0, xh0, zc0, zs0 = enc(x, W_enc, b_enc.reshape(1, 128),
                                    col(dinv), col(dout_inv), W0)
  n0 = n0c[:, 0]
  nf0 = jnp.sum(n0)
  rc0, rsym0 = _make_block_kernel(2)(zc0, zs0, src_t, dst_t)

  st1 = _make_stage_kernel(False, True, True, 1)
  (h1, n1c, rs1, xh1, zc1, zs1, G1, cs1, x_sym0) = st1(
      rc0, xh0, b0.reshape(1, 128), col(dinv), col(dout_inv), W1, zs0, rsym0)
  n1 = n1c[:, 0]
  nf1 = jnp.sum(n1)
  e0 = e_sym(n0, nf0, x_sym0[0, 0])
  rc1, rrw1, rsym1 = _make_block_kernel(3)(zc1, h1, zs1, src_t, dst_t)
  rank1 = _rank_diff(h1, G1, rs1, cs1)

  st2 = _make_stage_kernel(False, False, True, 2)
  (h2, n2c, rs2, zs2, G2, cs2, x_rw1, x_sym1) = st2(
      rc1, xh1, b1.reshape(1, 128), col(dinv), col(dout_inv),
      h1, rrw1, zs1, rsym1)
  n2 = n2c[:, 0]
  nf2 = jnp.sum(n2)
  erw1 = e_rw(n1, nf1, x_rw1[0, 0])
  esym1 = e_sym(n1, nf1, x_sym1[0, 0])
  rrw2, rsym2 = _make_block_kernel(2)(h2, zs2, src_t, dst_t)
  rank2 = _rank_diff(h2, G2, rs2, cs2)
  x_rw2, x_sym2 = _cross2(h2, rrw2, zs2, rsym2)
  erw2 = e_rw(n2, nf2, x_rw2)
  esym2 = e_sym(n2, nf2, x_sym2)

  return (h2,
          jnp.stack([e0, erw1, erw2]),
          jnp.stack([esym1, esym2]),
          jnp.stack([rank1, rank2]))


# speedup vs baseline: 15.3623x; 1.0379x over previous
import functools

import jax
import jax.numpy as jnp
from jax import lax
from jax.experimental import pallas as pl
from jax.experimental.pallas import tpu as pltpu
from jax.experimental.pallas import tpu_sc as plsc

N = 10000
E = 320000
NC, NS = 2, 16
NW = NC * NS
EB = 64
K = -(-E // (NW * EB))
K = K + (K % 2)
EPAD = NW * K * EB
NP = 10112
JUNK = 10008
RPW = NP // NS
T64 = RPW // 64
REM = RPW - T64 * 64

_mesh = plsc.VectorSubcoreMesh(
    core_axis_name="c", subcore_axis_name="s", num_cores=NC, num_subcores=NS)

_f32 = jnp.float32
_sc_params = pltpu.CompilerParams(use_tc_tiling_on_sc=False)
_HI = jax.lax.Precision.HIGHEST


def _vdot(a, b):
  return jnp.sum(a * b)


def _fill(ref, rows, cols, value):
  def body(i, _):
    r = i // (cols // 16)
    c0 = (i % (cols // 16)) * 16
    ref[r, pl.ds(c0, 16)] = jnp.full((16,), value, _f32)
    return _
  lax.fori_loop(0, rows * (cols // 16), body, 0)


def _zero_acc(acc, s, zero_v):
  base = s * RPW
  for t in range(T64):
    pltpu.sync_copy(zero_v, acc.at[pl.ds(base + t * 64, 64)])
  pltpu.sync_copy(zero_v.at[pl.ds(0, REM)],
                  acc.at[pl.ds(base + T64 * 64, REM)])


def _writeback(acc, out, c, s):
  base = s * RPW
  pltpu.sync_copy(acc.at[pl.ds(base, RPW)], out.at[c, pl.ds(base, RPW)])


@functools.partial(
    pl.kernel,
    out_type=(jax.ShapeDtypeStruct((NC, NP, 16), _f32),
              jax.ShapeDtypeStruct((NC, NP, 16), _f32)),
    mesh=_mesh,
    compiler_params=_sc_params,
    scratch_types=[
        pltpu.VMEM((K, EB), jnp.int32),
        pltpu.VMEM((K, EB), jnp.int32),
        pltpu.VMEM((EB, 16), _f32),
        pltpu.VMEM((64, 16), _f32),
        pltpu.VMEM_SHARED((NP, 16), _f32),
        pltpu.VMEM_SHARED((NP, 16), _f32),
    ],
)
def _deg_kernel(src_hbm, dst_hbm, dout_hbm, din_hbm,
                sidx, didx, ones_v, zero16, acc_o, acc_i):
  c = lax.axis_index("c")
  s = lax.axis_index("s")
  w = s * NC + c
  pltpu.sync_copy(src_hbm.at[w], sidx)
  pltpu.sync_copy(dst_hbm.at[w], didx)
  _fill(ones_v, EB, 16, 1.0)
  _fill(zero16, 64, 16, 0.0)
  base = s * RPW
  for t in range(T64):
    pltpu.sync_copy(zero16, acc_o.at[pl.ds(base + t * 64, 64)])
    pltpu.sync_copy(zero16, acc_i.at[pl.ds(base + t * 64, 64)])
  pltpu.sync_copy(zero16.at[pl.ds(0, REM)], acc_o.at[pl.ds(base + T64 * 64, REM)])
  pltpu.sync_copy(zero16.at[pl.ds(0, REM)], acc_i.at[pl.ds(base + T64 * 64, REM)])
  plsc.subcore_barrier()

  def body(k, _):
    pltpu.sync_copy(ones_v, acc_o.at[sidx.at[k]], add=True)
    pltpu.sync_copy(ones_v, acc_i.at[didx.at[k]], add=True)
    return _
  lax.fori_loop(0, K, body, 0)
  plsc.subcore_barrier()
  pltpu.sync_copy(acc_o.at[pl.ds(base, RPW)], dout_hbm.at[c, pl.ds(base, RPW)])
  pltpu.sync_copy(acc_i.at[pl.ds(base, RPW)], din_hbm.at[c, pl.ds(base, RPW)])


@functools.partial(
    pl.kernel,
    out_type=(jax.ShapeDtypeStruct((NC, NP, 16), _f32),
              jax.ShapeDtypeStruct((NC, NP, 16), _f32)),
    mesh=_mesh,
    compiler_params=_sc_params,
    scratch_types=[
        pltpu.VMEM((K, EB), jnp.int32),
        pltpu.VMEM((K, EB), jnp.int32),
        pltpu.VMEM((EB, 16), _f32),
        pltpu.VMEM((EB, 16), _f32),
        pltpu.VMEM((64, 16), _f32),
        pltpu.VMEM_SHARED((NP, 16), _f32),
        pltpu.VMEM_SHARED((NP, 16), _f32),
        pltpu.SemaphoreType.DMA,
        pltpu.SemaphoreType.DMA,
    ],
)
def _vpass_kernel(v16_hbm, src_hbm, dst_hbm, win_hbm, wout_hbm,
                  sidx, didx, rows_f, rows_r, zero16, acc_f, acc_r, sem_f, sem_r):
  c = lax.axis_index("c")
  s = lax.axis_index("s")
  w = s * NC + c
  pltpu.sync_copy(src_hbm.at[w], sidx)
  pltpu.sync_copy(dst_hbm.at[w], didx)
  _fill(zero16, 64, 16, 0.0)
  base = s * RPW
  for t in range(T64):
    pltpu.sync_copy(zero16, acc_f.at[pl.ds(base + t * 64, 64)])
    pltpu.sync_copy(zero16, acc_r.at[pl.ds(base + t * 64, 64)])
  pltpu.sync_copy(zero16.at[pl.ds(0, REM)], acc_f.at[pl.ds(base + T64 * 64, REM)])
  pltpu.sync_copy(zero16.at[pl.ds(0, REM)], acc_r.at[pl.ds(base + T64 * 64, REM)])
  plsc.subcore_barrier()

  pltpu.async_copy(v16_hbm.at[sidx.at[0]], rows_f, sem_f)
  pltpu.async_copy(v16_hbm.at[didx.at[0]], rows_r, sem_r)

  def body(k, _):
    pltpu.make_async_copy(v16_hbm.at[sidx.at[k]], rows_f, sem_f).wait()
    pltpu.sync_copy(rows_f, acc_f.at[didx.at[k]], add=True)

    @pl.when(k + 1 < K)
    def _f():
      pltpu.async_copy(v16_hbm.at[sidx.at[k + 1]], rows_f, sem_f)
    pltpu.make_async_copy(v16_hbm.at[didx.at[k]], rows_r, sem_r).wait()
    pltpu.sync_copy(rows_r, acc_r.at[sidx.at[k]], add=True)

    @pl.when(k + 1 < K)
    def _r():
      pltpu.async_copy(v16_hbm.at[didx.at[k + 1]], rows_r, sem_r)
    return _
  lax.fori_loop(0, K, body, 0)
  plsc.subcore_barrier()
  pltpu.sync_copy(acc_f.at[pl.ds(base, RPW)], win_hbm.at[c, pl.ds(base, RPW)])
  pltpu.sync_copy(acc_r.at[pl.ds(base, RPW)], wout_hbm.at[c, pl.ds(base, RPW)])


@functools.lru_cache(maxsize=None)
def _make_block_kernel(nblocks):
  out_type = tuple(jax.ShapeDtypeStruct((NC, NP, 128), _f32)
                   for _ in range(nblocks))

  @functools.partial(
      pl.kernel,
      out_type=out_type,
      mesh=_mesh,
      compiler_params=_sc_params,
      scratch_types=[
          pltpu.VMEM((K, EB), jnp.int32),
          pltpu.VMEM((K, EB), jnp.int32),
          pltpu.VMEM((EB, 128), _f32),
          pltpu.VMEM((EB, 128), _f32),
          pltpu.VMEM((64, 128), _f32),
          pltpu.VMEM_SHARED((NP, 128), _f32),
          pltpu.SemaphoreType.DMA,
          pltpu.SemaphoreType.DMA,
          pltpu.SemaphoreType.DMA,
          pltpu.SemaphoreType.DMA,
      ],
  )
  def _block_kernel(*refs):
    z_hbm = refs[:nblocks]
    src_hbm, dst_hbm = refs[nblocks], refs[nblocks + 1]
    outs = refs[nblocks + 2:2 * nblocks + 2]
    (sidx, didx, rows0, rows1, zero_v, acc,
     sem0, sem1, semw0, semw1) = refs[2 * nblocks + 2:]
    c = lax.axis_index("c")
    s = lax.axis_index("s")
    w = s * NC + c
    pltpu.sync_copy(src_hbm.at[w], sidx)
    pltpu.sync_copy(dst_hbm.at[w], didx)
    _fill(zero_v, 64, 128, 0.0)
    for b in range(nblocks):
      _zero_acc(acc, s, zero_v)
      plsc.subcore_barrier()
      zb = z_hbm[b]
      pltpu.async_copy(zb.at[sidx.at[0]], rows0, sem0)
      pltpu.async_copy(zb.at[sidx.at[1]], rows1, sem1)

      def pair(t, _, zb=zb):
        k0 = 2 * t
        pltpu.make_async_copy(zb.at[sidx.at[k0]], rows0, sem0).wait()
        pltpu.async_copy(rows0, acc.at[didx.at[k0]], semw0, add=True)

        @pl.when(k0 + 2 < K)
        def _fire():
          pltpu.make_async_copy(rows0, acc.at[didx.at[k0]], semw0).wait()
          pltpu.async_copy(zb.at[sidx.at[k0 + 2]], rows0, sem0)
        pltpu.make_async_copy(zb.at[sidx.at[k0 + 1]], rows1, sem1).wait()
        pltpu.async_copy(rows1, acc.at[didx.at[k0 + 1]], semw1, add=True)

        @pl.when(k0 + 3 < K)
        def _fire1():
          pltpu.make_async_copy(rows1, acc.at[didx.at[k0 + 1]], semw1).wait()
          pltpu.async_copy(zb.at[sidx.at[k0 + 3]], rows1, sem1)
        return _
      lax.fori_loop(0, K // 2, pair, 0)
      pltpu.make_async_copy(rows0, acc.at[didx.at[K - 2]], semw0).wait()
      pltpu.make_async_copy(rows1, acc.at[didx.at[K - 1]], semw1).wait()
      plsc.subcore_barrier()
      _writeback(acc, outs[b], c, s)
      plsc.subcore_barrier()

  return _block_kernel


def _msgpass(blocks, src_t, dst_t):
  kern = _make_block_kernel(len(blocks))
  outs = kern(*blocks, src_t, dst_t)
  return [(o[0] + o[1])[:N] for o in outs]


NBLK = 25
BR = N // NBLK


def _dot(a, b):
  return jnp.dot(a, b, precision=_HI)


@functools.lru_cache(maxsize=None)
def _make_stage_kernel(encoder, with_next, with_rank, ncross):
  def body(*refs):
    idx = 0
    if encoder:
      x_ref, W_ref, b_ref = refs[0:3]
    else:
      rc_ref, xhp_ref, b_ref = refs[0:3]
    idx = 3
    dinv_ref, dout_inv_ref = refs[idx:idx + 2]; idx += 2
    if with_next:
      Wn_ref = refs[idx]; idx += 1
    cross_in = []
    for _ in range(ncross):
      cross_in.append((refs[idx], refs[idx + 1])); idx += 2
    h_ref, n_ref, rs_ref = refs[idx:idx + 3]; idx += 3
    if with_next:
      xh_ref, zc_ref = refs[idx:idx + 2]; idx += 2
    zs_ref = refs[idx]; idx += 1
    if with_rank:
      G_ref, cs_ref = refs[idx:idx + 2]; idx += 2
    cross_out = refs[idx:idx + ncross]

    i = pl.program_id(0)
    dinv = dinv_ref[...]
    if encoder:
      h = _dot(x_ref[...], W_ref[...]) + b_ref[...]
    else:
      pre = dinv * (rc_ref[0] + rc_ref[1]) + dinv * dinv * xhp_ref[...] + b_ref[...]
      h = jnp.maximum(pre, 0.0)
    h_ref[...] = h
    n_ref[...] = jnp.sum(h * h, axis=1, keepdims=True)
    rs_ref[...] = jnp.sum(jnp.abs(h), axis=1, keepdims=True)
    if with_next:
      xh = _dot(h, Wn_ref[...])
      xh_ref[...] = xh
      zc_ref[...] = dinv * xh
    zs_ref[...] = h * dout_inv_ref[...]

    @pl.when(i == 0)
    def _init():
      if with_rank:
        G_ref[...] = jnp.zeros_like(G_ref)
        cs_ref[...] = jnp.zeros_like(cs_ref)
      for co in cross_out:
        co[0, 0] = 0.0

    if with_rank:
      G_ref[...] += _dot(h.T, h)
      cs_ref[...] += jnp.sum(jnp.abs(h), axis=0, keepdims=True)
    for (zp_ref, r_ref), co in zip(cross_in, cross_out):
      co[0, 0] += jnp.sum(zp_ref[...] * (r_ref[0] + r_ref[1]))

  row = lambda i: (i, 0)
  full = lambda i: (0, 0)
  full3 = lambda i: (0, i, 0)
  in_specs = []
  if encoder:
    in_specs += [pl.BlockSpec((BR, 128), row), pl.BlockSpec((128, 128), full),
                 pl.BlockSpec((1, 128), full)]
  else:
    in_specs += [pl.BlockSpec((2, BR, 128), full3), pl.BlockSpec((BR, 128), row),
                 pl.BlockSpec((1, 128), full)]
  in_specs += [pl.BlockSpec((BR, 1), row)] * 2
  if with_next:
    in_specs += [pl.BlockSpec((128, 128), full)]
  for _ in range(ncross):
    in_specs += [pl.BlockSpec((BR, 128), row), pl.BlockSpec((2, BR, 128), full3)]

  out_specs = [pl.BlockSpec((BR, 128), row), pl.BlockSpec((BR, 1), row),
               pl.BlockSpec((BR, 1), row)]
  out_shape = [jax.ShapeDtypeStruct((N, 128), _f32),
               jax.ShapeDtypeStruct((N, 1), _f32),
               jax.ShapeDtypeStruct((N, 1), _f32)]
  if with_next:
    out_specs += [pl.BlockSpec((BR, 128), row)] * 2
    out_shape += [jax.ShapeDtypeStruct((N, 128), _f32)] * 2
  out_specs += [pl.BlockSpec((BR, 128), row)]
  out_shape += [jax.ShapeDtypeStruct((N, 128), _f32)]
  if with_rank:
    out_specs += [pl.BlockSpec((128, 128), full), pl.BlockSpec((1, 128), full)]
    out_shape += [jax.ShapeDtypeStruct((128, 128), _f32),
                  jax.ShapeDtypeStruct((1, 128), _f32)]
  out_specs += [pl.BlockSpec((1, 1), full, memory_space=pltpu.SMEM)] * ncross
  out_shape += [jax.ShapeDtypeStruct((1, 1), _f32)] * ncross

  return pl.pallas_call(body, grid=(NBLK,), in_specs=in_specs,
                        out_specs=out_specs, out_shape=out_shape)


def _cross2(zA, rA, zB, rB):
  def body(zA_ref, rA_ref, zB_ref, rB_ref, a_ref, b_ref):
    i = pl.program_id(0)

    @pl.when(i == 0)
    def _init():
      a_ref[0, 0] = 0.0
      b_ref[0, 0] = 0.0
    a_ref[0, 0] += jnp.sum(zA_ref[...] * (rA_ref[0] + rA_ref[1]))
    b_ref[0, 0] += jnp.sum(zB_ref[...] * (rB_ref[0] + rB_ref[1]))

  row = lambda i: (i, 0)
  full3 = lambda i: (0, i, 0)
  smem = pl.BlockSpec((1, 1), lambda i: (0, 0), memory_space=pltpu.SMEM)
  out = pl.pallas_call(
      body, grid=(NBLK,),
      in_specs=[pl.BlockSpec((BR, 128), row), pl.BlockSpec((2, BR, 128), full3),
                pl.BlockSpec((BR, 128), row), pl.BlockSpec((2, BR, 128), full3)],
      out_specs=[smem, smem],
      out_shape=[jax.ShapeDtypeStruct((1, 1), _f32)] * 2)(zA, rA, zB, rB)
  return out[0][0, 0], out[1][0, 0]


NS_ITERS = 25


def _rank_kernel(G, g, r, cnorm2, rnorm2, sgn):
  def body(G_ref, g_ref, r_ref, sc_ref, I_ref, nu_ref, rank_ref):
    I = I_ref[...]

    def trsqrt(S):
      c = jnp.sum(S * I)
      Y, Z = S / c, I
      for _ in range(NS_ITERS):
        T = 0.5 * (3.0 * I - _dot(Z, Y))
        Y, Z = _dot(Y, T), _dot(T, Z)
      return jnp.sum(Y * I) * jnp.sqrt(c)

    G = G_ref[...]
    nu1 = trsqrt(G)
    nu_ref[0, 0] = nu1
    g_ = g_ref[...]
    r_ = r_ref[...]
    cn2, rn2, sg = sc_ref[0], sc_ref[1], sc_ref[2]
    gr = _dot(g_.T, r_) + _dot(r_.T, g_)
    M = (G / (nu1 * nu1)
         - (sg / (nu1 * jnp.sqrt(cn2 * rn2))) * gr
         + _dot(r_.T, r_) / rn2)
    rank_ref[0, 0] = trsqrt(M)

  nu, rank = pl.pallas_call(
      body,
      in_specs=[pl.BlockSpec(memory_space=pltpu.VMEM),
                pl.BlockSpec(memory_space=pltpu.VMEM),
                pl.BlockSpec(memory_space=pltpu.VMEM),
                pl.BlockSpec(memory_space=pltpu.SMEM),
                pl.BlockSpec(memory_space=pltpu.VMEM)],
      out_specs=[pl.BlockSpec(memory_space=pltpu.SMEM),
                 pl.BlockSpec(memory_space=pltpu.SMEM)],
      out_shape=[jax.ShapeDtypeStruct((1, 1), _f32)] * 2,
  )(G, g.reshape(1, 128), r.reshape(1, 128), jnp.stack([cnorm2, rnorm2, sgn]),
    jnp.eye(128, dtype=_f32))
  return rank[0, 0]


def _rank_diff(h, G, rs, cs):
  i = jnp.argmax(rs[:, 0])
  j = jnp.argmax(cs[0, :])
  r = lax.dynamic_slice(h, (i, 0), (1, 128))[0]
  g = lax.dynamic_slice(G, (0, j), (128, 1))[:, 0]
  cnorm2 = G[j, j]
  rnorm2 = jnp.sum(r * r)
  sgn = jnp.where(h[i, j] < 0, -1.0, 1.0)
  return _rank_kernel(G, g, r, cnorm2, rnorm2, sgn)


def kernel(x, edge_index, W_enc, b_enc, W0, b0, W1, b1):
  src = edge_index[0]
  dst = edge_index[1]
  pad = EPAD - E
  src_t = jnp.concatenate([src, jnp.zeros((pad,), jnp.int32)]).reshape(NW, K, EB)
  dst_t = jnp.concatenate([dst, jnp.full((pad,), JUNK, jnp.int32)]).reshape(NW, K, EB)

  dout_p, din_p = _deg_kernel(src_t, dst_t)
  dout = (dout_p[0] + dout_p[1])[:N, 0] + 1.0
  dout = dout.at[0].add(-float(pad))
  din = (din_p[0] + din_p[1])[:N, 0] + 1.0
  v = lax.rsqrt(dout)
  dinv = lax.rsqrt(din)
  dout_inv = 1.0 / dout
  col = lambda a: a.reshape(N, 1)

  v16 = jnp.zeros((NP, 16), _f32).at[:N, :].set(v[:, None])
  win_p, wout_p = _vpass_kernel(v16, src_t, dst_t)
  wsum = v * ((win_p[0] + win_p[1])[:N, 0] + (wout_p[0] + wout_p[1])[:N, 0])
  degsum = (dout - 1.0) + (din - 1.0)

  def e_rw(n, nf, cross):
    return 0.5 * (_vdot(degsum, n) - 2.0 * cross) / nf

  def e_sym(n, nf, cross):
    return 0.5 * (_vdot(n * dout_inv, wsum) - 2.0 * cross) / nf

  enc = _make_stage_kernel(True, True, False, 0)
  x0, n0c, rs0, xh0, zc0, zs0 = enc(x, W_enc, b_enc.reshape(1, 128),
                                    col(dinv), col(dout_inv), W0)
  n0 = n0c[:, 0]
  nf0 = jnp.sum(n0)
  rc0, rsym0 = _make_block_kernel(2)(zc0, zs0, src_t, dst_t)

  st1 = _make_stage_kernel(False, True, True, 1)
  (h1, n1c, rs1, xh1, zc1, zs1, G1, cs1, x_sym0) = st1(
      rc0, xh0, b0.reshape(1, 128), col(dinv), col(dout_inv), W1, zs0, rsym0)
  n1 = n1c[:, 0]
  nf1 = jnp.sum(n1)
  e0 = e_sym(n0, nf0, x_sym0[0, 0])
  rc1, rrw1, rsym1 = _make_block_kernel(3)(zc1, h1, zs1, src_t, dst_t)
  rank1 = _rank_diff(h1, G1, rs1, cs1)

  st2 = _make_stage_kernel(False, False, True, 2)
  (h2, n2c, rs2, zs2, G2, cs2, x_rw1, x_sym1) = st2(
      rc1, xh1, b1.reshape(1, 128), col(dinv), col(dout_inv),
      h1, rrw1, zs1, rsym1)
  n2 = n2c[:, 0]
  nf2 = jnp.sum(n2)
  erw1 = e_rw(n1, nf1, x_rw1[0, 0])
  esym1 = e_sym(n1, nf1, x_sym1[0, 0])
  rrw2, rsym2 = _make_block_kernel(2)(h2, zs2, src_t, dst_t)
  rank2 = _rank_diff(h2, G2, rs2, cs2)
  x_rw2, x_sym2 = _cross2(h2, rrw2, zs2, rsym2)
  erw2 = e_rw(n2, nf2, x_rw2)
  esym2 = e_sym(n2, nf2, x_sym2)

  return (h2,
          jnp.stack([e0, erw1, erw2]),
          jnp.stack([esym1, esym2]),
          jnp.stack([rank1, rank2]))


# speedup vs baseline: 16.4380x; 1.0700x over previous
import functools

import jax
import jax.numpy as jnp
from jax import lax
from jax.experimental import pallas as pl
from jax.experimental.pallas import tpu as pltpu
from jax.experimental.pallas import tpu_sc as plsc

N = 10000
E = 320000
NC, NS = 2, 16
NW = NC * NS
EB = 64
K = -(-E // (NW * EB))
K = K + (K % 2)
EPAD = NW * K * EB
NP = 10112
JUNK = 10008
RPW = NP // NS
T64 = RPW // 64
REM = RPW - T64 * 64

_mesh = plsc.VectorSubcoreMesh(
    core_axis_name="c", subcore_axis_name="s", num_cores=NC, num_subcores=NS)

_f32 = jnp.float32
_sc_params = pltpu.CompilerParams(use_tc_tiling_on_sc=False)
_HI = jax.lax.Precision.HIGHEST


def _vdot(a, b):
  return jnp.sum(a * b)


def _fill(ref, rows, cols, value):
  def body(i, _):
    r = i // (cols // 16)
    c0 = (i % (cols // 16)) * 16
    ref[r, pl.ds(c0, 16)] = jnp.full((16,), value, _f32)
    return _
  lax.fori_loop(0, rows * (cols // 16), body, 0)


def _zero_acc(acc, s, zero_v):
  base = s * RPW
  for t in range(T64):
    pltpu.sync_copy(zero_v, acc.at[pl.ds(base + t * 64, 64)])
  pltpu.sync_copy(zero_v.at[pl.ds(0, REM)],
                  acc.at[pl.ds(base + T64 * 64, REM)])


def _writeback(acc, out, c, s):
  base = s * RPW
  pltpu.sync_copy(acc.at[pl.ds(base, RPW)], out.at[c, pl.ds(base, RPW)])


@functools.partial(
    pl.kernel,
    out_type=(jax.ShapeDtypeStruct((NC, NP, 16), _f32),
              jax.ShapeDtypeStruct((NC, NP, 16), _f32)),
    mesh=_mesh,
    compiler_params=_sc_params,
    scratch_types=[
        pltpu.VMEM((K, EB), jnp.int32),
        pltpu.VMEM((K, EB), jnp.int32),
        pltpu.VMEM((EB, 16), _f32),
        pltpu.VMEM((64, 16), _f32),
        pltpu.VMEM_SHARED((NP, 16), _f32),
        pltpu.VMEM_SHARED((NP, 16), _f32),
    ],
)
def _deg_kernel(src_hbm, dst_hbm, dout_hbm, din_hbm,
                sidx, didx, ones_v, zero16, acc_o, acc_i):
  c = lax.axis_index("c")
  s = lax.axis_index("s")
  w = s * NC + c
  pltpu.sync_copy(src_hbm.at[w], sidx)
  pltpu.sync_copy(dst_hbm.at[w], didx)
  _fill(ones_v, EB, 16, 1.0)
  _fill(zero16, 64, 16, 0.0)
  base = s * RPW
  for t in range(T64):
    pltpu.sync_copy(zero16, acc_o.at[pl.ds(base + t * 64, 64)])
    pltpu.sync_copy(zero16, acc_i.at[pl.ds(base + t * 64, 64)])
  pltpu.sync_copy(zero16.at[pl.ds(0, REM)], acc_o.at[pl.ds(base + T64 * 64, REM)])
  pltpu.sync_copy(zero16.at[pl.ds(0, REM)], acc_i.at[pl.ds(base + T64 * 64, REM)])
  plsc.subcore_barrier()

  def body(k, _):
    pltpu.sync_copy(ones_v, acc_o.at[sidx.at[k]], add=True)
    pltpu.sync_copy(ones_v, acc_i.at[didx.at[k]], add=True)
    return _
  lax.fori_loop(0, K, body, 0)
  plsc.subcore_barrier()
  pltpu.sync_copy(acc_o.at[pl.ds(base, RPW)], dout_hbm.at[c, pl.ds(base, RPW)])
  pltpu.sync_copy(acc_i.at[pl.ds(base, RPW)], din_hbm.at[c, pl.ds(base, RPW)])


@functools.partial(
    pl.kernel,
    out_type=(jax.ShapeDtypeStruct((NC, NP, 16), _f32),
              jax.ShapeDtypeStruct((NC, NP, 16), _f32)),
    mesh=_mesh,
    compiler_params=_sc_params,
    scratch_types=[
        pltpu.VMEM((K, EB), jnp.int32),
        pltpu.VMEM((K, EB), jnp.int32),
        pltpu.VMEM((EB, 16), _f32),
        pltpu.VMEM((EB, 16), _f32),
        pltpu.VMEM((64, 16), _f32),
        pltpu.VMEM_SHARED((NP, 16), _f32),
        pltpu.VMEM_SHARED((NP, 16), _f32),
        pltpu.SemaphoreType.DMA,
        pltpu.SemaphoreType.DMA,
    ],
)
def _vpass_kernel(v16_hbm, src_hbm, dst_hbm, win_hbm, wout_hbm,
                  sidx, didx, rows_f, rows_r, zero16, acc_f, acc_r, sem_f, sem_r):
  c = lax.axis_index("c")
  s = lax.axis_index("s")
  w = s * NC + c
  pltpu.sync_copy(src_hbm.at[w], sidx)
  pltpu.sync_copy(dst_hbm.at[w], didx)
  _fill(zero16, 64, 16, 0.0)
  base = s * RPW
  for t in range(T64):
    pltpu.sync_copy(zero16, acc_f.at[pl.ds(base + t * 64, 64)])
    pltpu.sync_copy(zero16, acc_r.at[pl.ds(base + t * 64, 64)])
  pltpu.sync_copy(zero16.at[pl.ds(0, REM)], acc_f.at[pl.ds(base + T64 * 64, REM)])
  pltpu.sync_copy(zero16.at[pl.ds(0, REM)], acc_r.at[pl.ds(base + T64 * 64, REM)])
  plsc.subcore_barrier()

  pltpu.async_copy(v16_hbm.at[sidx.at[0]], rows_f, sem_f)
  pltpu.async_copy(v16_hbm.at[didx.at[0]], rows_r, sem_r)

  def body(k, _):
    pltpu.make_async_copy(v16_hbm.at[sidx.at[k]], rows_f, sem_f).wait()
    pltpu.sync_copy(rows_f, acc_f.at[didx.at[k]], add=True)

    @pl.when(k + 1 < K)
    def _f():
      pltpu.async_copy(v16_hbm.at[sidx.at[k + 1]], rows_f, sem_f)
    pltpu.make_async_copy(v16_hbm.at[didx.at[k]], rows_r, sem_r).wait()
    pltpu.sync_copy(rows_r, acc_r.at[sidx.at[k]], add=True)

    @pl.when(k + 1 < K)
    def _r():
      pltpu.async_copy(v16_hbm.at[didx.at[k + 1]], rows_r, sem_r)
    return _
  lax.fori_loop(0, K, body, 0)
  plsc.subcore_barrier()
  pltpu.sync_copy(acc_f.at[pl.ds(base, RPW)], win_hbm.at[c, pl.ds(base, RPW)])
  pltpu.sync_copy(acc_r.at[pl.ds(base, RPW)], wout_hbm.at[c, pl.ds(base, RPW)])


@functools.lru_cache(maxsize=None)
def _make_block_kernel(nblocks):
  out_type = tuple(jax.ShapeDtypeStruct((NC, NP, 128), _f32)
                   for _ in range(nblocks))

  @functools.partial(
      pl.kernel,
      out_type=out_type,
      mesh=_mesh,
      compiler_params=_sc_params,
      scratch_types=[
          pltpu.VMEM((K, EB), jnp.int32),
          pltpu.VMEM((K, EB), jnp.int32),
          pltpu.VMEM((EB, 128), _f32),
          pltpu.VMEM((EB, 128), _f32),
          pltpu.VMEM((64, 128), _f32),
          pltpu.VMEM_SHARED((NP, 128), _f32),
          pltpu.SemaphoreType.DMA,
          pltpu.SemaphoreType.DMA,
          pltpu.SemaphoreType.DMA,
          pltpu.SemaphoreType.DMA,
      ],
  )
  def _block_kernel(*refs):
    z_hbm = refs[:nblocks]
    src_hbm, dst_hbm = refs[nblocks], refs[nblocks + 1]
    outs = refs[nblocks + 2:2 * nblocks + 2]
    (sidx, didx, rows0, rows1, zero_v, acc,
     sem0, sem1, semw0, semw1) = refs[2 * nblocks + 2:]
    c = lax.axis_index("c")
    s = lax.axis_index("s")
    w = s * NC + c
    pltpu.sync_copy(src_hbm.at[w], sidx)
    pltpu.sync_copy(dst_hbm.at[w], didx)
    _fill(zero_v, 64, 128, 0.0)
    for b in range(nblocks):
      _zero_acc(acc, s, zero_v)
      plsc.subcore_barrier()
      zb = z_hbm[b]
      pltpu.async_copy(zb.at[sidx.at[0]], rows0, sem0)
      pltpu.async_copy(zb.at[sidx.at[1]], rows1, sem1)

      def pair(t, _, zb=zb):
        k0 = 2 * t
        pltpu.make_async_copy(zb.at[sidx.at[k0]], rows0, sem0).wait()
        pltpu.async_copy(rows0, acc.at[didx.at[k0]], semw0, add=True)

        @pl.when(k0 + 2 < K)
        def _fire():
          pltpu.make_async_copy(rows0, acc.at[didx.at[k0]], semw0).wait()
          pltpu.async_copy(zb.at[sidx.at[k0 + 2]], rows0, sem0)
        pltpu.make_async_copy(zb.at[sidx.at[k0 + 1]], rows1, sem1).wait()
        pltpu.async_copy(rows1, acc.at[didx.at[k0 + 1]], semw1, add=True)

        @pl.when(k0 + 3 < K)
        def _fire1():
          pltpu.make_async_copy(rows1, acc.at[didx.at[k0 + 1]], semw1).wait()
          pltpu.async_copy(zb.at[sidx.at[k0 + 3]], rows1, sem1)
        return _
      lax.fori_loop(0, K // 2, pair, 0)
      pltpu.make_async_copy(rows0, acc.at[didx.at[K - 2]], semw0).wait()
      pltpu.make_async_copy(rows1, acc.at[didx.at[K - 1]], semw1).wait()
      plsc.subcore_barrier()
      _writeback(acc, outs[b], c, s)
      plsc.subcore_barrier()

  return _block_kernel


def _msgpass(blocks, src_t, dst_t):
  kern = _make_block_kernel(len(blocks))
  outs = kern(*blocks, src_t, dst_t)
  return [(o[0] + o[1])[:N] for o in outs]


NBLK = 25
BR = N // NBLK


def _dot(a, b):
  return jnp.dot(a, b, precision=_HI)


@functools.lru_cache(maxsize=None)
def _make_stage_kernel(encoder, with_next, with_rank, ncross):
  def body(*refs):
    idx = 0
    if encoder:
      x_ref, W_ref, b_ref = refs[0:3]
    else:
      rc_ref, xhp_ref, b_ref = refs[0:3]
    idx = 3
    dinv_ref, dout_inv_ref = refs[idx:idx + 2]; idx += 2
    if with_next:
      Wn_ref = refs[idx]; idx += 1
    cross_in = []
    for _ in range(ncross):
      cross_in.append((refs[idx], refs[idx + 1])); idx += 2
    h_ref, n_ref, rs_ref = refs[idx:idx + 3]; idx += 3
    if with_next:
      xh_ref, zc_ref = refs[idx:idx + 2]; idx += 2
    zs_ref = refs[idx]; idx += 1
    if with_rank:
      G_ref, cs_ref = refs[idx:idx + 2]; idx += 2
    cross_out = refs[idx:idx + ncross]

    i = pl.program_id(0)
    dinv = dinv_ref[...]
    if encoder:
      h = _dot(x_ref[...], W_ref[...]) + b_ref[...]
    else:
      pre = dinv * (rc_ref[0] + rc_ref[1]) + dinv * dinv * xhp_ref[...] + b_ref[...]
      h = jnp.maximum(pre, 0.0)
    h_ref[...] = h
    n_ref[...] = jnp.sum(h * h, axis=1, keepdims=True)
    rs_ref[...] = jnp.sum(jnp.abs(h), axis=1, keepdims=True)
    if with_next:
      xh = _dot(h, Wn_ref[...])
      xh_ref[...] = xh
      zc_ref[...] = dinv * xh
    zs_ref[...] = h * dout_inv_ref[...]

    @pl.when(i == 0)
    def _init():
      if with_rank:
        G_ref[...] = jnp.zeros_like(G_ref)
        cs_ref[...] = jnp.zeros_like(cs_ref)
      for co in cross_out:
        co[0, 0] = 0.0

    if with_rank:
      G_ref[...] += _dot(h.T, h)
      cs_ref[...] += jnp.sum(jnp.abs(h), axis=0, keepdims=True)
    for (zp_ref, r_ref), co in zip(cross_in, cross_out):
      co[0, 0] += jnp.sum(zp_ref[...] * (r_ref[0] + r_ref[1]))

  row = lambda i: (i, 0)
  full = lambda i: (0, 0)
  full3 = lambda i: (0, i, 0)
  in_specs = []
  if encoder:
    in_specs += [pl.BlockSpec((BR, 128), row), pl.BlockSpec((128, 128), full),
                 pl.BlockSpec((1, 128), full)]
  else:
    in_specs += [pl.BlockSpec((2, BR, 128), full3), pl.BlockSpec((BR, 128), row),
                 pl.BlockSpec((1, 128), full)]
  in_specs += [pl.BlockSpec((BR, 1), row)] * 2
  if with_next:
    in_specs += [pl.BlockSpec((128, 128), full)]
  for _ in range(ncross):
    in_specs += [pl.BlockSpec((BR, 128), row), pl.BlockSpec((2, BR, 128), full3)]

  out_specs = [pl.BlockSpec((BR, 128), row), pl.BlockSpec((BR, 1), row),
               pl.BlockSpec((BR, 1), row)]
  out_shape = [jax.ShapeDtypeStruct((N, 128), _f32),
               jax.ShapeDtypeStruct((N, 1), _f32),
               jax.ShapeDtypeStruct((N, 1), _f32)]
  if with_next:
    out_specs += [pl.BlockSpec((BR, 128), row)] * 2
    out_shape += [jax.ShapeDtypeStruct((N, 128), _f32)] * 2
  out_specs += [pl.BlockSpec((BR, 128), row)]
  out_shape += [jax.ShapeDtypeStruct((N, 128), _f32)]
  if with_rank:
    out_specs += [pl.BlockSpec((128, 128), full), pl.BlockSpec((1, 128), full)]
    out_shape += [jax.ShapeDtypeStruct((128, 128), _f32),
                  jax.ShapeDtypeStruct((1, 128), _f32)]
  out_specs += [pl.BlockSpec((1, 1), full, memory_space=pltpu.SMEM)] * ncross
  out_shape += [jax.ShapeDtypeStruct((1, 1), _f32)] * ncross

  return pl.pallas_call(body, grid=(NBLK,), in_specs=in_specs,
                        out_specs=out_specs, out_shape=out_shape)


def _cross2(zA, rA, zB, rB):
  def body(zA_ref, rA_ref, zB_ref, rB_ref, a_ref, b_ref):
    i = pl.program_id(0)

    @pl.when(i == 0)
    def _init():
      a_ref[0, 0] = 0.0
      b_ref[0, 0] = 0.0
    a_ref[0, 0] += jnp.sum(zA_ref[...] * (rA_ref[0] + rA_ref[1]))
    b_ref[0, 0] += jnp.sum(zB_ref[...] * (rB_ref[0] + rB_ref[1]))

  row = lambda i: (i, 0)
  full3 = lambda i: (0, i, 0)
  smem = pl.BlockSpec((1, 1), lambda i: (0, 0), memory_space=pltpu.SMEM)
  out = pl.pallas_call(
      body, grid=(NBLK,),
      in_specs=[pl.BlockSpec((BR, 128), row), pl.BlockSpec((2, BR, 128), full3),
                pl.BlockSpec((BR, 128), row), pl.BlockSpec((2, BR, 128), full3)],
      out_specs=[smem, smem],
      out_shape=[jax.ShapeDtypeStruct((1, 1), _f32)] * 2)(zA, rA, zB, rB)
  return out[0][0, 0], out[1][0, 0]


NS_ITERS = 25


def _rank_kernel(G, g, r, cnorm2, rnorm2, sgn):
  def body(G_ref, g_ref, r_ref, sc_ref, I_ref, nu_ref, rank_ref):
    I = I_ref[...]

    def trsqrt(S):
      c = jnp.sum(S * I)
      Y, Z = S / c, I
      for _ in range(NS_ITERS):
        T = 0.5 * (3.0 * I - _dot(Z, Y))
        Y, Z = _dot(Y, T), _dot(T, Z)
      return jnp.sum(Y * I) * jnp.sqrt(c)

    G = G_ref[...]
    nu1 = trsqrt(G)
    nu_ref[0, 0] = nu1
    g_ = g_ref[...]
    r_ = r_ref[...]
    cn2, rn2, sg = sc_ref[0], sc_ref[1], sc_ref[2]
    gr = _dot(g_.T, r_) + _dot(r_.T, g_)
    M = (G / (nu1 * nu1)
         - (sg / (nu1 * jnp.sqrt(cn2 * rn2))) * gr
         + _dot(r_.T, r_) / rn2)
    rank_ref[0, 0] = trsqrt(M)

  nu, rank = pl.pallas_call(
      body,
      in_specs=[pl.BlockSpec(memory_space=pltpu.VMEM),
                pl.BlockSpec(memory_space=pltpu.VMEM),
                pl.BlockSpec(memory_space=pltpu.VMEM),
                pl.BlockSpec(memory_space=pltpu.SMEM),
                pl.BlockSpec(memory_space=pltpu.VMEM)],
      out_specs=[pl.BlockSpec(memory_space=pltpu.SMEM),
                 pl.BlockSpec(memory_space=pltpu.SMEM)],
      out_shape=[jax.ShapeDtypeStruct((1, 1), _f32)] * 2,
  )(G, g.reshape(1, 128), r.reshape(1, 128), jnp.stack([cnorm2, rnorm2, sgn]),
    jnp.eye(128, dtype=_f32))
  return rank[0, 0]


def _rank_diff(h, G, rs, cs):
  i = jnp.argmax(rs[:, 0])
  j = jnp.argmax(cs[0, :])
  r = lax.dynamic_slice(h, (i, 0), (1, 128))[0]
  g = lax.dynamic_slice(G, (0, j), (128, 1))[:, 0]
  cnorm2 = G[j, j]
  rnorm2 = jnp.sum(r * r)
  sgn = jnp.where(h[i, j] < 0, -1.0, 1.0)
  return _rank_kernel(G, g, r, cnorm2, rnorm2, sgn)


def kernel(x, edge_index, W_enc, b_enc, W0, b0, W1, b1):
  src = edge_index[0]
  dst = edge_index[1]
  pad = EPAD - E
  src_t = jnp.concatenate([src, jnp.zeros((pad,), jnp.int32)]).reshape(NW, K, EB)
  dst_t = jnp.concatenate([dst, jnp.full((pad,), JUNK, jnp.int32)]).reshape(NW, K, EB)

  dout_p, din_p = _deg_kernel(src_t, dst_t)
  dout = (dout_p[0] + dout_p[1])[:N, 0] + 1.0
  dout = dout.at[0].add(-float(pad))
  din = (din_p[0] + din_p[1])[:N, 0] + 1.0
  v = lax.rsqrt(dout)
  dinv = lax.rsqrt(din)
  dout_inv = 1.0 / dout
  col = lambda a: a.reshape(N, 1)

  v16 = jnp.zeros((NP, 16), _f32).at[:N, :].set(v[:, None])
  win_p, wout_p = _vpass_kernel(v16, src_t, dst_t)
  wsum = v * ((win_p[0] + win_p[1])[:N, 0] + (wout_p[0] + wout_p[1])[:N, 0])
  degsum = (dout - 1.0) + (din - 1.0)

  def e_rw(n, nf, cross):
    return 0.5 * (_vdot(degsum, n) - 2.0 * cross) / nf

  def e_sym(n, nf, cross):
    return 0.5 * (_vdot(n * dout_inv, wsum) - 2.0 * cross) / nf

  enc = _make_stage_kernel(True, True, False, 0)
  x0, n0c, rs0, xh0, zc0, zs0 = enc(x, W_enc, b_enc.reshape(1, 128),
                                    col(dinv), col(dout_inv), W0)
  n0 = n0c[:, 0]
  nf0 = jnp.sum(n0)
  (rc0,) = _make_block_kernel(1)(zc0, src_t, dst_t)
  (rsym0,) = _make_block_kernel(1)(zs0, src_t, dst_t)

  st1 = _make_stage_kernel(False, True, True, 1)
  (h1, n1c, rs1, xh1, zc1, zs1, G1, cs1, x_sym0) = st1(
      rc0, xh0, b0.reshape(1, 128), col(dinv), col(dout_inv), W1, zs0, rsym0)
  n1 = n1c[:, 0]
  nf1 = jnp.sum(n1)
  e0 = e_sym(n0, nf0, x_sym0[0, 0])
  (rc1,) = _make_block_kernel(1)(zc1, src_t, dst_t)
  rrw1, rsym1 = _make_block_kernel(2)(h1, zs1, src_t, dst_t)
  rank1 = _rank_diff(h1, G1, rs1, cs1)

  st2 = _make_stage_kernel(False, False, True, 2)
  (h2, n2c, rs2, zs2, G2, cs2, x_rw1, x_sym1) = st2(
      rc1, xh1, b1.reshape(1, 128), col(dinv), col(dout_inv),
      h1, rrw1, zs1, rsym1)
  n2 = n2c[:, 0]
  nf2 = jnp.sum(n2)
  erw1 = e_rw(n1, nf1, x_rw1[0, 0])
  esym1 = e_sym(n1, nf1, x_sym1[0, 0])
  rrw2, rsym2 = _make_block_kernel(2)(h2, zs2, src_t, dst_t)
  rank2 = _rank_diff(h2, G2, rs2, cs2)
  x_rw2, x_sym2 = _cross2(h2, rrw2, zs2, rsym2)
  erw2 = e_rw(n2, nf2, x_rw2)
  esym2 = e_sym(n2, nf2, x_sym2)

  return (h2,
          jnp.stack([e0, erw1, erw2]),
          jnp.stack([esym1, esym2]),
          jnp.stack([rank1, rank2]))


# speedup vs baseline: 30.7140x; 1.8685x over previous
import functools

import jax
import jax.numpy as jnp
from jax import lax
from jax.experimental import pallas as pl
from jax.experimental.pallas import tpu as pltpu
from jax.experimental.pallas import tpu_sc as plsc

N = 10000
E = 320000
NC, NS = 2, 16
NW = NC * NS
EB = 80
K = E // (NW * EB)
NP = 10112
RPW = NP // NS
T64 = RPW // 64
REM = RPW - T64 * 64

_mesh = plsc.VectorSubcoreMesh(
    core_axis_name="c", subcore_axis_name="s", num_cores=NC, num_subcores=NS)

_f32 = jnp.float32
_sc_params = pltpu.CompilerParams(use_tc_tiling_on_sc=False)
_HI = jax.lax.Precision.HIGHEST


def _vdot(a, b):
  return jnp.sum(a * b)


def _fill(ref, rows, cols, value):
  def body(i, _):
    r = i // (cols // 16)
    c0 = (i % (cols // 16)) * 16
    ref[r, pl.ds(c0, 16)] = jnp.full((16,), value, _f32)
    return _
  lax.fori_loop(0, rows * (cols // 16), body, 0)


def _zero_acc(acc, s, zero_v):
  base = s * RPW
  for t in range(T64):
    pltpu.sync_copy(zero_v, acc.at[pl.ds(base + t * 64, 64)])
  pltpu.sync_copy(zero_v.at[pl.ds(0, REM)],
                  acc.at[pl.ds(base + T64 * 64, REM)])


def _writeback(acc, out, c, s):
  base = s * RPW
  pltpu.sync_copy(acc.at[pl.ds(base, RPW)], out.at[c, pl.ds(base, RPW)])


@functools.partial(
    pl.kernel,
    out_type=(jax.ShapeDtypeStruct((NC, NP, 16), _f32),
              jax.ShapeDtypeStruct((NC, NP, 16), _f32)),
    mesh=_mesh,
    compiler_params=_sc_params,
    scratch_types=[
        pltpu.VMEM((K, EB), jnp.int32),
        pltpu.VMEM((K, EB), jnp.int32),
        pltpu.VMEM((EB, 16), _f32),
        pltpu.VMEM((64, 16), _f32),
        pltpu.VMEM_SHARED((NP, 16), _f32),
        pltpu.VMEM_SHARED((NP, 16), _f32),
    ],
)
def _deg_kernel(src_hbm, dst_hbm, dout_hbm, din_hbm,
                sidx, didx, ones_v, zero16, acc_o, acc_i):
  c = lax.axis_index("c")
  s = lax.axis_index("s")
  w = s * NC + c
  pltpu.sync_copy(src_hbm.at[w], sidx)
  pltpu.sync_copy(dst_hbm.at[w], didx)
  _fill(ones_v, EB, 16, 1.0)
  _fill(zero16, 64, 16, 0.0)
  base = s * RPW
  for t in range(T64):
    pltpu.sync_copy(zero16, acc_o.at[pl.ds(base + t * 64, 64)])
    pltpu.sync_copy(zero16, acc_i.at[pl.ds(base + t * 64, 64)])
  pltpu.sync_copy(zero16.at[pl.ds(0, REM)], acc_o.at[pl.ds(base + T64 * 64, REM)])
  pltpu.sync_copy(zero16.at[pl.ds(0, REM)], acc_i.at[pl.ds(base + T64 * 64, REM)])
  plsc.subcore_barrier()

  def body(k, _):
    pltpu.sync_copy(ones_v, acc_o.at[sidx.at[k]], add=True)
    pltpu.sync_copy(ones_v, acc_i.at[didx.at[k]], add=True)
    return _
  lax.fori_loop(0, K, body, 0)
  plsc.subcore_barrier()
  pltpu.sync_copy(acc_o.at[pl.ds(base, RPW)], dout_hbm.at[c, pl.ds(base, RPW)])
  pltpu.sync_copy(acc_i.at[pl.ds(base, RPW)], din_hbm.at[c, pl.ds(base, RPW)])


@functools.partial(
    pl.kernel,
    out_type=(jax.ShapeDtypeStruct((NC, NP, 16), _f32),
              jax.ShapeDtypeStruct((NC, NP, 16), _f32)),
    mesh=_mesh,
    compiler_params=_sc_params,
    scratch_types=[
        pltpu.VMEM((K, EB), jnp.int32),
        pltpu.VMEM((K, EB), jnp.int32),
        pltpu.VMEM((EB, 16), _f32),
        pltpu.VMEM((EB, 16), _f32),
        pltpu.VMEM((64, 16), _f32),
        pltpu.VMEM_SHARED((NP, 16), _f32),
        pltpu.VMEM_SHARED((NP, 16), _f32),
        pltpu.SemaphoreType.DMA,
        pltpu.SemaphoreType.DMA,
    ],
)
def _vpass_kernel(v16_hbm, src_hbm, dst_hbm, win_hbm, wout_hbm,
                  sidx, didx, rows_f, rows_r, zero16, acc_f, acc_r, sem_f, sem_r):
  c = lax.axis_index("c")
  s = lax.axis_index("s")
  w = s * NC + c
  pltpu.sync_copy(src_hbm.at[w], sidx)
  pltpu.sync_copy(dst_hbm.at[w], didx)
  _fill(zero16, 64, 16, 0.0)
  base = s * RPW
  for t in range(T64):
    pltpu.sync_copy(zero16, acc_f.at[pl.ds(base + t * 64, 64)])
    pltpu.sync_copy(zero16, acc_r.at[pl.ds(base + t * 64, 64)])
  pltpu.sync_copy(zero16.at[pl.ds(0, REM)], acc_f.at[pl.ds(base + T64 * 64, REM)])
  pltpu.sync_copy(zero16.at[pl.ds(0, REM)], acc_r.at[pl.ds(base + T64 * 64, REM)])
  plsc.subcore_barrier()

  pltpu.async_copy(v16_hbm.at[sidx.at[0]], rows_f, sem_f)
  pltpu.async_copy(v16_hbm.at[didx.at[0]], rows_r, sem_r)

  def body(k, _):
    pltpu.make_async_copy(v16_hbm.at[sidx.at[k]], rows_f, sem_f).wait()
    pltpu.sync_copy(rows_f, acc_f.at[didx.at[k]], add=True)

    @pl.when(k + 1 < K)
    def _f():
      pltpu.async_copy(v16_hbm.at[sidx.at[k + 1]], rows_f, sem_f)
    pltpu.make_async_copy(v16_hbm.at[didx.at[k]], rows_r, sem_r).wait()
    pltpu.sync_copy(rows_r, acc_r.at[sidx.at[k]], add=True)

    @pl.when(k + 1 < K)
    def _r():
      pltpu.async_copy(v16_hbm.at[didx.at[k + 1]], rows_r, sem_r)
    return _
  lax.fori_loop(0, K, body, 0)
  plsc.subcore_barrier()
  pltpu.sync_copy(acc_f.at[pl.ds(base, RPW)], win_hbm.at[c, pl.ds(base, RPW)])
  pltpu.sync_copy(acc_r.at[pl.ds(base, RPW)], wout_hbm.at[c, pl.ds(base, RPW)])


@functools.lru_cache(maxsize=None)
def _make_block_kernel(nblocks):
  out_type = tuple(jax.ShapeDtypeStruct((NC, NP, 128), _f32)
                   for _ in range(nblocks))

  @functools.partial(
      pl.kernel,
      out_type=out_type,
      mesh=_mesh,
      compiler_params=_sc_params,
      scratch_types=[
          pltpu.VMEM((K, EB), jnp.int32),
          pltpu.VMEM((K, EB), jnp.int32),
          pltpu.VMEM((EB, 128), _f32),
          pltpu.VMEM((EB, 128), _f32),
          pltpu.VMEM((64, 128), _f32),
          pltpu.VMEM_SHARED((NP, 128), _f32),
          pltpu.SemaphoreType.DMA,
          pltpu.SemaphoreType.DMA,
          pltpu.SemaphoreType.DMA,
          pltpu.SemaphoreType.DMA,
      ],
  )
  def _block_kernel(*refs):
    z_hbm = refs[:nblocks]
    src_hbm, dst_hbm = refs[nblocks], refs[nblocks + 1]
    outs = refs[nblocks + 2:2 * nblocks + 2]
    (sidx, didx, rows0, rows1, zero_v, acc,
     sem0, sem1, semw0, semw1) = refs[2 * nblocks + 2:]
    c = lax.axis_index("c")
    s = lax.axis_index("s")
    w = s * NC + c
    pltpu.sync_copy(src_hbm.at[w], sidx)
    pltpu.sync_copy(dst_hbm.at[w], didx)
    _fill(zero_v, 64, 128, 0.0)
    for b in range(nblocks):
      _zero_acc(acc, s, zero_v)
      plsc.subcore_barrier()
      zb = z_hbm[b]
      pltpu.async_copy(zb.at[sidx.at[0]], rows0, sem0)
      pltpu.async_copy(zb.at[sidx.at[1]], rows1, sem1)

      def pair(t, _, zb=zb):
        k0 = 2 * t
        pltpu.make_async_copy(zb.at[sidx.at[k0]], rows0, sem0).wait()
        pltpu.async_copy(rows0, acc.at[didx.at[k0]], semw0, add=True)

        @pl.when(k0 + 2 < K)
        def _fire():
          pltpu.make_async_copy(rows0, acc.at[didx.at[k0]], semw0).wait()
          pltpu.async_copy(zb.at[sidx.at[k0 + 2]], rows0, sem0)
        pltpu.make_async_copy(zb.at[sidx.at[k0 + 1]], rows1, sem1).wait()
        pltpu.async_copy(rows1, acc.at[didx.at[k0 + 1]], semw1, add=True)

        @pl.when(k0 + 3 < K)
        def _fire1():
          pltpu.make_async_copy(rows1, acc.at[didx.at[k0 + 1]], semw1).wait()
          pltpu.async_copy(zb.at[sidx.at[k0 + 3]], rows1, sem1)
        return _
      lax.fori_loop(0, K // 2, pair, 0)
      pltpu.make_async_copy(zb.at[sidx.at[K - 1]], rows0, sem0).wait()
      pltpu.async_copy(rows0, acc.at[didx.at[K - 1]], semw0, add=True)
      pltpu.make_async_copy(rows0, acc.at[didx.at[K - 1]], semw0).wait()
      pltpu.make_async_copy(rows1, acc.at[didx.at[K - 2]], semw1).wait()
      plsc.subcore_barrier()
      _writeback(acc, outs[b], c, s)
      plsc.subcore_barrier()

  return _block_kernel


def _msgpass(blocks, src_t, dst_t):
  kern = _make_block_kernel(len(blocks))
  outs = kern(*blocks, src_t, dst_t)
  return [(o[0] + o[1])[:N] for o in outs]


NBLK = 25
BR = N // NBLK


def _dot(a, b):
  return jnp.dot(a, b, precision=_HI)


@functools.lru_cache(maxsize=None)
def _make_stage_kernel(encoder, with_next, with_rank, ncross):
  def body(*refs):
    idx = 0
    if encoder:
      x_ref, W_ref, b_ref = refs[0:3]
    else:
      rc_ref, xhp_ref, b_ref = refs[0:3]
    idx = 3
    dinv_ref, dout_inv_ref = refs[idx:idx + 2]; idx += 2
    if with_next:
      Wn_ref = refs[idx]; idx += 1
    cross_in = []
    for _ in range(ncross):
      cross_in.append((refs[idx], refs[idx + 1])); idx += 2
    h_ref, n_ref, rs_ref = refs[idx:idx + 3]; idx += 3
    if with_next:
      xh_ref, zc_ref = refs[idx:idx + 2]; idx += 2
    zs_ref = refs[idx]; idx += 1
    if with_rank:
      G_ref, cs_ref = refs[idx:idx + 2]; idx += 2
    cross_out = refs[idx:idx + ncross]

    i = pl.program_id(0)
    dinv = dinv_ref[...]
    if encoder:
      h = _dot(x_ref[...], W_ref[...]) + b_ref[...]
    else:
      pre = dinv * (rc_ref[0] + rc_ref[1]) + dinv * dinv * xhp_ref[...] + b_ref[...]
      h = jnp.maximum(pre, 0.0)
    h_ref[...] = h
    n_ref[...] = jnp.sum(h * h, axis=1, keepdims=True)
    rs_ref[...] = jnp.sum(jnp.abs(h), axis=1, keepdims=True)
    if with_next:
      xh = _dot(h, Wn_ref[...])
      xh_ref[...] = xh
      zc_ref[...] = dinv * xh
    zs_ref[...] = h * dout_inv_ref[...]

    @pl.when(i == 0)
    def _init():
      if with_rank:
        G_ref[...] = jnp.zeros_like(G_ref)
        cs_ref[...] = jnp.zeros_like(cs_ref)
      for co in cross_out:
        co[0, 0] = 0.0

    if with_rank:
      G_ref[...] += _dot(h.T, h)
      cs_ref[...] += jnp.sum(jnp.abs(h), axis=0, keepdims=True)
    for (zp_ref, r_ref), co in zip(cross_in, cross_out):
      co[0, 0] += jnp.sum(zp_ref[...] * (r_ref[0] + r_ref[1]))

  row = lambda i: (i, 0)
  full = lambda i: (0, 0)
  full3 = lambda i: (0, i, 0)
  in_specs = []
  if encoder:
    in_specs += [pl.BlockSpec((BR, 128), row), pl.BlockSpec((128, 128), full),
                 pl.BlockSpec((1, 128), full)]
  else:
    in_specs += [pl.BlockSpec((2, BR, 128), full3), pl.BlockSpec((BR, 128), row),
                 pl.BlockSpec((1, 128), full)]
  in_specs += [pl.BlockSpec((BR, 1), row)] * 2
  if with_next:
    in_specs += [pl.BlockSpec((128, 128), full)]
  for _ in range(ncross):
    in_specs += [pl.BlockSpec((BR, 128), row), pl.BlockSpec((2, BR, 128), full3)]

  out_specs = [pl.BlockSpec((BR, 128), row), pl.BlockSpec((BR, 1), row),
               pl.BlockSpec((BR, 1), row)]
  out_shape = [jax.ShapeDtypeStruct((N, 128), _f32),
               jax.ShapeDtypeStruct((N, 1), _f32),
               jax.ShapeDtypeStruct((N, 1), _f32)]
  if with_next:
    out_specs += [pl.BlockSpec((BR, 128), row)] * 2
    out_shape += [jax.ShapeDtypeStruct((N, 128), _f32)] * 2
  out_specs += [pl.BlockSpec((BR, 128), row)]
  out_shape += [jax.ShapeDtypeStruct((N, 128), _f32)]
  if with_rank:
    out_specs += [pl.BlockSpec((128, 128), full), pl.BlockSpec((1, 128), full)]
    out_shape += [jax.ShapeDtypeStruct((128, 128), _f32),
                  jax.ShapeDtypeStruct((1, 128), _f32)]
  out_specs += [pl.BlockSpec((1, 1), full, memory_space=pltpu.SMEM)] * ncross
  out_shape += [jax.ShapeDtypeStruct((1, 1), _f32)] * ncross

  return pl.pallas_call(body, grid=(NBLK,), in_specs=in_specs,
                        out_specs=out_specs, out_shape=out_shape)


def _cross2(zA, rA, zB, rB):
  def body(zA_ref, rA_ref, zB_ref, rB_ref, a_ref, b_ref):
    i = pl.program_id(0)

    @pl.when(i == 0)
    def _init():
      a_ref[0, 0] = 0.0
      b_ref[0, 0] = 0.0
    a_ref[0, 0] += jnp.sum(zA_ref[...] * (rA_ref[0] + rA_ref[1]))
    b_ref[0, 0] += jnp.sum(zB_ref[...] * (rB_ref[0] + rB_ref[1]))

  row = lambda i: (i, 0)
  full3 = lambda i: (0, i, 0)
  smem = pl.BlockSpec((1, 1), lambda i: (0, 0), memory_space=pltpu.SMEM)
  out = pl.pallas_call(
      body, grid=(NBLK,),
      in_specs=[pl.BlockSpec((BR, 128), row), pl.BlockSpec((2, BR, 128), full3),
                pl.BlockSpec((BR, 128), row), pl.BlockSpec((2, BR, 128), full3)],
      out_specs=[smem, smem],
      out_shape=[jax.ShapeDtypeStruct((1, 1), _f32)] * 2)(zA, rA, zB, rB)
  return out[0][0, 0], out[1][0, 0]


NS_ITERS = 25


def _rank_kernel(G, g, r, cnorm2, rnorm2, sgn):
  def body(G_ref, g_ref, r_ref, sc_ref, I_ref, nu_ref, rank_ref):
    I = I_ref[...]

    def trsqrt(S):
      c = jnp.sum(S * I)
      Y, Z = S / c, I
      for _ in range(NS_ITERS):
        T = 0.5 * (3.0 * I - _dot(Z, Y))
        Y, Z = _dot(Y, T), _dot(T, Z)
      return jnp.sum(Y * I) * jnp.sqrt(c)

    G = G_ref[...]
    nu1 = trsqrt(G)
    nu_ref[0, 0] = nu1
    g_ = g_ref[...]
    r_ = r_ref[...]
    cn2, rn2, sg = sc_ref[0], sc_ref[1], sc_ref[2]
    gr = _dot(g_.T, r_) + _dot(r_.T, g_)
    M = (G / (nu1 * nu1)
         - (sg / (nu1 * jnp.sqrt(cn2 * rn2))) * gr
         + _dot(r_.T, r_) / rn2)
    rank_ref[0, 0] = trsqrt(M)

  nu, rank = pl.pallas_call(
      body,
      in_specs=[pl.BlockSpec(memory_space=pltpu.VMEM),
                pl.BlockSpec(memory_space=pltpu.VMEM),
                pl.BlockSpec(memory_space=pltpu.VMEM),
                pl.BlockSpec(memory_space=pltpu.SMEM),
                pl.BlockSpec(memory_space=pltpu.VMEM)],
      out_specs=[pl.BlockSpec(memory_space=pltpu.SMEM),
                 pl.BlockSpec(memory_space=pltpu.SMEM)],
      out_shape=[jax.ShapeDtypeStruct((1, 1), _f32)] * 2,
  )(G, g.reshape(1, 128), r.reshape(1, 128), jnp.stack([cnorm2, rnorm2, sgn]),
    jnp.eye(128, dtype=_f32))
  return rank[0, 0]


def _rank_diff(h, G, rs, cs):
  i = jnp.argmax(rs[:, 0])
  j = jnp.argmax(cs[0, :])
  r = lax.dynamic_slice(h, (i, 0), (1, 128))[0]
  g = lax.dynamic_slice(G, (0, j), (128, 1))[:, 0]
  cnorm2 = G[j, j]
  rnorm2 = jnp.sum(r * r)
  sgn = jnp.where(h[i, j] < 0, -1.0, 1.0)
  return _rank_kernel(G, g, r, cnorm2, rnorm2, sgn)


def kernel(x, edge_index, W_enc, b_enc, W0, b0, W1, b1):
  src_t = edge_index[0].reshape(NW, K, EB)
  dst_t = edge_index[1].reshape(NW, K, EB)

  dout_p, din_p = _deg_kernel(src_t, dst_t)
  dout = (dout_p[0] + dout_p[1])[:N, 0] + 1.0
  din = (din_p[0] + din_p[1])[:N, 0] + 1.0
  v = lax.rsqrt(dout)
  dinv = lax.rsqrt(din)
  dout_inv = 1.0 / dout
  col = lambda a: a.reshape(N, 1)

  v16 = jnp.zeros((NP, 16), _f32).at[:N, :].set(v[:, None])
  win_p, wout_p = _vpass_kernel(v16, src_t, dst_t)
  wsum = v * ((win_p[0] + win_p[1])[:N, 0] + (wout_p[0] + wout_p[1])[:N, 0])
  degsum = (dout - 1.0) + (din - 1.0)

  def e_rw(n, nf, cross):
    return 0.5 * (_vdot(degsum, n) - 2.0 * cross) / nf

  def e_sym(n, nf, cross):
    return 0.5 * (_vdot(n * dout_inv, wsum) - 2.0 * cross) / nf

  enc = _make_stage_kernel(True, True, False, 0)
  x0, n0c, rs0, xh0, zc0, zs0 = enc(x, W_enc, b_enc.reshape(1, 128),
                                    col(dinv), col(dout_inv), W0)
  n0 = n0c[:, 0]
  nf0 = jnp.sum(n0)
  (rc0,) = _make_block_kernel(1)(zc0, src_t, dst_t)
  (rsym0,) = _make_block_kernel(1)(zs0, src_t, dst_t)

  st1 = _make_stage_kernel(False, True, True, 1)
  (h1, n1c, rs1, xh1, zc1, zs1, G1, cs1, x_sym0) = st1(
      rc0, xh0, b0.reshape(1, 128), col(dinv), col(dout_inv), W1, zs0, rsym0)
  n1 = n1c[:, 0]
  nf1 = jnp.sum(n1)
  e0 = e_sym(n0, nf0, x_sym0[0, 0])
  (rc1,) = _make_block_kernel(1)(zc1, src_t, dst_t)
  rrw1, rsym1 = _make_block_kernel(2)(h1, zs1, src_t, dst_t)
  rank1 = _rank_diff(h1, G1, rs1, cs1)

  st2 = _make_stage_kernel(False, False, True, 2)
  (h2, n2c, rs2, zs2, G2, cs2, x_rw1, x_sym1) = st2(
      rc1, xh1, b1.reshape(1, 128), col(dinv), col(dout_inv),
      h1, rrw1, zs1, rsym1)
  n2 = n2c[:, 0]
  nf2 = jnp.sum(n2)
  erw1 = e_rw(n1, nf1, x_rw1[0, 0])
  esym1 = e_sym(n1, nf1, x_sym1[0, 0])
  rrw2, rsym2 = _make_block_kernel(2)(h2, zs2, src_t, dst_t)
  rank2 = _rank_diff(h2, G2, rs2, cs2)
  x_rw2, x_sym2 = _cross2(h2, rrw2, zs2, rsym2)
  erw2 = e_rw(n2, nf2, x_rw2)
  esym2 = e_sym(n2, nf2, x_sym2)

  return (h2,
          jnp.stack([e0, erw1, erw2]),
          jnp.stack([esym1, esym2]),
          jnp.stack([rank1, rank2]))


# speedup vs baseline: 30.7430x; 1.0009x over previous
import functools

import jax
import jax.numpy as jnp
from jax import lax
from jax.experimental import pallas as pl
from jax.experimental.pallas import tpu as pltpu
from jax.experimental.pallas import tpu_sc as plsc

N = 10000
E = 320000
NC, NS = 2, 16
NW = NC * NS
EB = 80
K = E // (NW * EB)
NP = 10112
RPW = NP // NS
T64 = RPW // 64
REM = RPW - T64 * 64

_mesh = plsc.VectorSubcoreMesh(
    core_axis_name="c", subcore_axis_name="s", num_cores=NC, num_subcores=NS)

_f32 = jnp.float32
_sc_params = pltpu.CompilerParams(use_tc_tiling_on_sc=False)
_HI = jax.lax.Precision.HIGHEST


def _vdot(a, b):
  return jnp.sum(a * b)


def _fill(ref, rows, cols, value):
  def body(i, _):
    r = i // (cols // 16)
    c0 = (i % (cols // 16)) * 16
    ref[r, pl.ds(c0, 16)] = jnp.full((16,), value, _f32)
    return _
  lax.fori_loop(0, rows * (cols // 16), body, 0)


def _zero_acc(acc, s, zero_v):
  base = s * RPW
  for t in range(T64):
    pltpu.sync_copy(zero_v, acc.at[pl.ds(base + t * 64, 64)])
  pltpu.sync_copy(zero_v.at[pl.ds(0, REM)],
                  acc.at[pl.ds(base + T64 * 64, REM)])


def _writeback(acc, out, c, s):
  base = s * RPW
  pltpu.sync_copy(acc.at[pl.ds(base, RPW)], out.at[c, pl.ds(base, RPW)])


@functools.partial(
    pl.kernel,
    out_type=(jax.ShapeDtypeStruct((NC, NP, 16), _f32),
              jax.ShapeDtypeStruct((NC, NP, 16), _f32)),
    mesh=_mesh,
    compiler_params=_sc_params,
    scratch_types=[
        pltpu.VMEM((K, EB), jnp.int32),
        pltpu.VMEM((K, EB), jnp.int32),
        pltpu.VMEM((EB, 16), _f32),
        pltpu.VMEM((64, 16), _f32),
        pltpu.VMEM_SHARED((NP, 16), _f32),
        pltpu.VMEM_SHARED((NP, 16), _f32),
    ],
)
def _deg_kernel(src_hbm, dst_hbm, dout_hbm, din_hbm,
                sidx, didx, ones_v, zero16, acc_o, acc_i):
  c = lax.axis_index("c")
  s = lax.axis_index("s")
  w = s * NC + c
  pltpu.sync_copy(src_hbm.at[w], sidx)
  pltpu.sync_copy(dst_hbm.at[w], didx)
  _fill(ones_v, EB, 16, 1.0)
  _fill(zero16, 64, 16, 0.0)
  base = s * RPW
  for t in range(T64):
    pltpu.sync_copy(zero16, acc_o.at[pl.ds(base + t * 64, 64)])
    pltpu.sync_copy(zero16, acc_i.at[pl.ds(base + t * 64, 64)])
  pltpu.sync_copy(zero16.at[pl.ds(0, REM)], acc_o.at[pl.ds(base + T64 * 64, REM)])
  pltpu.sync_copy(zero16.at[pl.ds(0, REM)], acc_i.at[pl.ds(base + T64 * 64, REM)])
  plsc.subcore_barrier()

  def body(k, _):
    pltpu.sync_copy(ones_v, acc_o.at[sidx.at[k]], add=True)
    pltpu.sync_copy(ones_v, acc_i.at[didx.at[k]], add=True)
    return _
  lax.fori_loop(0, K, body, 0)
  plsc.subcore_barrier()
  pltpu.sync_copy(acc_o.at[pl.ds(base, RPW)], dout_hbm.at[c, pl.ds(base, RPW)])
  pltpu.sync_copy(acc_i.at[pl.ds(base, RPW)], din_hbm.at[c, pl.ds(base, RPW)])


@functools.partial(
    pl.kernel,
    out_type=(jax.ShapeDtypeStruct((NC, NP, 16), _f32),
              jax.ShapeDtypeStruct((NC, NP, 16), _f32)),
    mesh=_mesh,
    compiler_params=_sc_params,
    scratch_types=[
        pltpu.VMEM((K, EB), jnp.int32),
        pltpu.VMEM((K, EB), jnp.int32),
        pltpu.VMEM((EB, 16), _f32),
        pltpu.VMEM((EB, 16), _f32),
        pltpu.VMEM((64, 16), _f32),
        pltpu.VMEM_SHARED((NP, 16), _f32),
        pltpu.VMEM_SHARED((NP, 16), _f32),
        pltpu.SemaphoreType.DMA,
        pltpu.SemaphoreType.DMA,
    ],
)
def _vpass_kernel(v16_hbm, src_hbm, dst_hbm, win_hbm, wout_hbm,
                  sidx, didx, rows_f, rows_r, zero16, acc_f, acc_r, sem_f, sem_r):
  c = lax.axis_index("c")
  s = lax.axis_index("s")
  w = s * NC + c
  pltpu.sync_copy(src_hbm.at[w], sidx)
  pltpu.sync_copy(dst_hbm.at[w], didx)
  _fill(zero16, 64, 16, 0.0)
  base = s * RPW
  for t in range(T64):
    pltpu.sync_copy(zero16, acc_f.at[pl.ds(base + t * 64, 64)])
    pltpu.sync_copy(zero16, acc_r.at[pl.ds(base + t * 64, 64)])
  pltpu.sync_copy(zero16.at[pl.ds(0, REM)], acc_f.at[pl.ds(base + T64 * 64, REM)])
  pltpu.sync_copy(zero16.at[pl.ds(0, REM)], acc_r.at[pl.ds(base + T64 * 64, REM)])
  plsc.subcore_barrier()

  pltpu.async_copy(v16_hbm.at[sidx.at[0]], rows_f, sem_f)
  pltpu.async_copy(v16_hbm.at[didx.at[0]], rows_r, sem_r)

  def body(k, _):
    pltpu.make_async_copy(v16_hbm.at[sidx.at[k]], rows_f, sem_f).wait()
    pltpu.sync_copy(rows_f, acc_f.at[didx.at[k]], add=True)

    @pl.when(k + 1 < K)
    def _f():
      pltpu.async_copy(v16_hbm.at[sidx.at[k + 1]], rows_f, sem_f)
    pltpu.make_async_copy(v16_hbm.at[didx.at[k]], rows_r, sem_r).wait()
    pltpu.sync_copy(rows_r, acc_r.at[sidx.at[k]], add=True)

    @pl.when(k + 1 < K)
    def _r():
      pltpu.async_copy(v16_hbm.at[didx.at[k + 1]], rows_r, sem_r)
    return _
  lax.fori_loop(0, K, body, 0)
  plsc.subcore_barrier()
  pltpu.sync_copy(acc_f.at[pl.ds(base, RPW)], win_hbm.at[c, pl.ds(base, RPW)])
  pltpu.sync_copy(acc_r.at[pl.ds(base, RPW)], wout_hbm.at[c, pl.ds(base, RPW)])


@functools.lru_cache(maxsize=None)
def _make_block_kernel(nblocks):
  out_type = tuple(jax.ShapeDtypeStruct((NC, NP, 128), _f32)
                   for _ in range(nblocks))

  @functools.partial(
      pl.kernel,
      out_type=out_type,
      mesh=_mesh,
      compiler_params=_sc_params,
      scratch_types=[
          pltpu.VMEM((K, EB), jnp.int32),
          pltpu.VMEM((K, EB), jnp.int32),
          pltpu.VMEM((EB, 128), _f32),
          pltpu.VMEM((EB, 128), _f32),
          pltpu.VMEM((64, 128), _f32),
          pltpu.VMEM_SHARED((NP, 128), _f32),
          pltpu.SemaphoreType.DMA,
          pltpu.SemaphoreType.DMA,
          pltpu.SemaphoreType.DMA,
          pltpu.SemaphoreType.DMA,
      ],
  )
  def _block_kernel(*refs):
    z_hbm = refs[:nblocks]
    src_hbm, dst_hbm = refs[nblocks], refs[nblocks + 1]
    outs = refs[nblocks + 2:2 * nblocks + 2]
    (sidx, didx, rows0, rows1, zero_v, acc,
     sem0, sem1, semw0, semw1) = refs[2 * nblocks + 2:]
    c = lax.axis_index("c")
    s = lax.axis_index("s")
    w = s * NC + c
    pltpu.sync_copy(src_hbm.at[w], sidx)
    pltpu.sync_copy(dst_hbm.at[w], didx)
    _fill(zero_v, 64, 128, 0.0)
    for b in range(nblocks):
      _zero_acc(acc, s, zero_v)
      plsc.subcore_barrier()
      zb = z_hbm[b]
      pltpu.async_copy(zb.at[sidx.at[0]], rows0, sem0)
      pltpu.async_copy(zb.at[sidx.at[1]], rows1, sem1)

      def pair(t, _, zb=zb):
        k0 = 2 * t
        pltpu.make_async_copy(zb.at[sidx.at[k0]], rows0, sem0).wait()
        pltpu.async_copy(rows0, acc.at[didx.at[k0]], semw0, add=True)

        @pl.when(k0 + 2 < K)
        def _fire():
          pltpu.make_async_copy(rows0, acc.at[didx.at[k0]], semw0).wait()
          pltpu.async_copy(zb.at[sidx.at[k0 + 2]], rows0, sem0)
        pltpu.make_async_copy(zb.at[sidx.at[k0 + 1]], rows1, sem1).wait()
        pltpu.async_copy(rows1, acc.at[didx.at[k0 + 1]], semw1, add=True)

        @pl.when(k0 + 3 < K)
        def _fire1():
          pltpu.make_async_copy(rows1, acc.at[didx.at[k0 + 1]], semw1).wait()
          pltpu.async_copy(zb.at[sidx.at[k0 + 3]], rows1, sem1)
        return _
      lax.fori_loop(0, K // 2, pair, 0)
      pltpu.make_async_copy(zb.at[sidx.at[K - 1]], rows0, sem0).wait()
      pltpu.async_copy(rows0, acc.at[didx.at[K - 1]], semw0, add=True)
      pltpu.make_async_copy(rows0, acc.at[didx.at[K - 1]], semw0).wait()
      pltpu.make_async_copy(rows1, acc.at[didx.at[K - 2]], semw1).wait()
      plsc.subcore_barrier()
      _writeback(acc, outs[b], c, s)
      plsc.subcore_barrier()

  return _block_kernel


def _msgpass(blocks, src_t, dst_t):
  kern = _make_block_kernel(len(blocks))
  outs = kern(*blocks, src_t, dst_t)
  return [(o[0] + o[1])[:N] for o in outs]


NBLK = 25
BR = N // NBLK


def _dot(a, b):
  return jnp.dot(a, b, precision=_HI)


@functools.lru_cache(maxsize=None)
def _make_stage_kernel(encoder, with_next, with_rank, ncross):
  def body(*refs):
    idx = 0
    if encoder:
      x_ref, W_ref, b_ref = refs[0:3]
    else:
      rc_ref, xhp_ref, b_ref = refs[0:3]
    idx = 3
    dinv_ref, dout_inv_ref = refs[idx:idx + 2]; idx += 2
    if with_next:
      Wn_ref = refs[idx]; idx += 1
    cross_in = []
    for _ in range(ncross):
      cross_in.append((refs[idx], refs[idx + 1])); idx += 2
    h_ref, n_ref, rs_ref = refs[idx:idx + 3]; idx += 3
    if with_next:
      xh_ref, zc_ref = refs[idx:idx + 2]; idx += 2
    zs_ref = refs[idx]; idx += 1
    if with_rank:
      G_ref, cs_ref = refs[idx:idx + 2]; idx += 2
    cross_out = refs[idx:idx + ncross]

    i = pl.program_id(0)
    dinv = dinv_ref[...]
    if encoder:
      h = _dot(x_ref[...], W_ref[...]) + b_ref[...]
    else:
      pre = dinv * (rc_ref[0] + rc_ref[1]) + dinv * dinv * xhp_ref[...] + b_ref[...]
      h = jnp.maximum(pre, 0.0)
    h_ref[...] = h
    n_ref[...] = jnp.sum(h * h, axis=1, keepdims=True)
    rs_ref[...] = jnp.sum(jnp.abs(h), axis=1, keepdims=True)
    if with_next:
      xh = _dot(h, Wn_ref[...])
      xh_ref[...] = xh
      zc_ref[...] = dinv * xh
    zs_ref[...] = h * dout_inv_ref[...]

    @pl.when(i == 0)
    def _init():
      if with_rank:
        G_ref[...] = jnp.zeros_like(G_ref)
        cs_ref[...] = jnp.zeros_like(cs_ref)
      for co in cross_out:
        co[0, 0] = 0.0

    if with_rank:
      G_ref[...] += _dot(h.T, h)
      cs_ref[...] += jnp.sum(jnp.abs(h), axis=0, keepdims=True)
    for (zp_ref, r_ref), co in zip(cross_in, cross_out):
      co[0, 0] += jnp.sum(zp_ref[...] * (r_ref[0] + r_ref[1]))

  row = lambda i: (i, 0)
  full = lambda i: (0, 0)
  full3 = lambda i: (0, i, 0)
  in_specs = []
  if encoder:
    in_specs += [pl.BlockSpec((BR, 128), row), pl.BlockSpec((128, 128), full),
                 pl.BlockSpec((1, 128), full)]
  else:
    in_specs += [pl.BlockSpec((2, BR, 128), full3), pl.BlockSpec((BR, 128), row),
                 pl.BlockSpec((1, 128), full)]
  in_specs += [pl.BlockSpec((BR, 1), row)] * 2
  if with_next:
    in_specs += [pl.BlockSpec((128, 128), full)]
  for _ in range(ncross):
    in_specs += [pl.BlockSpec((BR, 128), row), pl.BlockSpec((2, BR, 128), full3)]

  out_specs = [pl.BlockSpec((BR, 128), row), pl.BlockSpec((BR, 1), row),
               pl.BlockSpec((BR, 1), row)]
  out_shape = [jax.ShapeDtypeStruct((N, 128), _f32),
               jax.ShapeDtypeStruct((N, 1), _f32),
               jax.ShapeDtypeStruct((N, 1), _f32)]
  if with_next:
    out_specs += [pl.BlockSpec((BR, 128), row)] * 2
    out_shape += [jax.ShapeDtypeStruct((N, 128), _f32)] * 2
  out_specs += [pl.BlockSpec((BR, 128), row)]
  out_shape += [jax.ShapeDtypeStruct((N, 128), _f32)]
  if with_rank:
    out_specs += [pl.BlockSpec((128, 128), full), pl.BlockSpec((1, 128), full)]
    out_shape += [jax.ShapeDtypeStruct((128, 128), _f32),
                  jax.ShapeDtypeStruct((1, 128), _f32)]
  out_specs += [pl.BlockSpec((1, 1), full, memory_space=pltpu.SMEM)] * ncross
  out_shape += [jax.ShapeDtypeStruct((1, 1), _f32)] * ncross

  return pl.pallas_call(body, grid=(NBLK,), in_specs=in_specs,
                        out_specs=out_specs, out_shape=out_shape)


def _cross2(zA, rA, zB, rB):
  def body(zA_ref, rA_ref, zB_ref, rB_ref, a_ref, b_ref):
    i = pl.program_id(0)

    @pl.when(i == 0)
    def _init():
      a_ref[0, 0] = 0.0
      b_ref[0, 0] = 0.0
    a_ref[0, 0] += jnp.sum(zA_ref[...] * (rA_ref[0] + rA_ref[1]))
    b_ref[0, 0] += jnp.sum(zB_ref[...] * (rB_ref[0] + rB_ref[1]))

  row = lambda i: (i, 0)
  full3 = lambda i: (0, i, 0)
  smem = pl.BlockSpec((1, 1), lambda i: (0, 0), memory_space=pltpu.SMEM)
  out = pl.pallas_call(
      body, grid=(NBLK,),
      in_specs=[pl.BlockSpec((BR, 128), row), pl.BlockSpec((2, BR, 128), full3),
                pl.BlockSpec((BR, 128), row), pl.BlockSpec((2, BR, 128), full3)],
      out_specs=[smem, smem],
      out_shape=[jax.ShapeDtypeStruct((1, 1), _f32)] * 2)(zA, rA, zB, rB)
  return out[0][0, 0], out[1][0, 0]


NS_ITERS = 25


def _rank_kernel(G, g, r, cnorm2, rnorm2, sgn):
  def body(G_ref, g_ref, r_ref, sc_ref, I_ref, nu_ref, rank_ref):
    I = I_ref[...]

    def trsqrt(S):
      c = jnp.sum(S * I)
      Y, Z = S / c, I
      for _ in range(NS_ITERS):
        T = 0.5 * (3.0 * I - _dot(Z, Y))
        Y, Z = _dot(Y, T), _dot(T, Z)
      return jnp.sum(Y * I) * jnp.sqrt(c)

    G = G_ref[...]
    nu1 = trsqrt(G)
    nu_ref[0, 0] = nu1
    g_ = g_ref[...]
    r_ = r_ref[...]
    cn2, rn2, sg = sc_ref[0], sc_ref[1], sc_ref[2]
    gr = _dot(g_.T, r_) + _dot(r_.T, g_)
    M = (G / (nu1 * nu1)
         - (sg / (nu1 * jnp.sqrt(cn2 * rn2))) * gr
         + _dot(r_.T, r_) / rn2)
    rank_ref[0, 0] = trsqrt(M)

  nu, rank = pl.pallas_call(
      body,
      in_specs=[pl.BlockSpec(memory_space=pltpu.VMEM),
                pl.BlockSpec(memory_space=pltpu.VMEM),
                pl.BlockSpec(memory_space=pltpu.VMEM),
                pl.BlockSpec(memory_space=pltpu.SMEM),
                pl.BlockSpec(memory_space=pltpu.VMEM)],
      out_specs=[pl.BlockSpec(memory_space=pltpu.SMEM),
                 pl.BlockSpec(memory_space=pltpu.SMEM)],
      out_shape=[jax.ShapeDtypeStruct((1, 1), _f32)] * 2,
  )(G, g.reshape(1, 128), r.reshape(1, 128), jnp.stack([cnorm2, rnorm2, sgn]),
    jnp.eye(128, dtype=_f32))
  return rank[0, 0]


def _rank_diff(h, G, rs, cs):
  i = jnp.argmax(rs[:, 0])
  j = jnp.argmax(cs[0, :])
  r = lax.dynamic_slice(h, (i, 0), (1, 128))[0]
  g = lax.dynamic_slice(G, (0, j), (128, 1))[:, 0]
  cnorm2 = G[j, j]
  rnorm2 = jnp.sum(r * r)
  sgn = jnp.where(h[i, j] < 0, -1.0, 1.0)
  return _rank_kernel(G, g, r, cnorm2, rnorm2, sgn)


@functools.lru_cache(maxsize=None)
def _make_enc_kernel():
  def body(x_ref, W_ref, b_ref, Wn_ref, x0_ref, n_ref, xh_ref):
    x0 = _dot(x_ref[...], W_ref[...]) + b_ref[...]
    x0_ref[...] = x0
    n_ref[...] = jnp.sum(x0 * x0, axis=1, keepdims=True)
    xh_ref[...] = _dot(x0, Wn_ref[...])

  row = lambda i: (i, 0)
  full = lambda i: (0, 0)
  return pl.pallas_call(
      body, grid=(NBLK,),
      in_specs=[pl.BlockSpec((BR, 128), row), pl.BlockSpec((128, 128), full),
                pl.BlockSpec((1, 128), full), pl.BlockSpec((128, 128), full)],
      out_specs=[pl.BlockSpec((BR, 128), row), pl.BlockSpec((BR, 1), row),
                 pl.BlockSpec((BR, 128), row)],
      out_shape=[jax.ShapeDtypeStruct((N, 128), _f32),
                 jax.ShapeDtypeStruct((N, 1), _f32),
                 jax.ShapeDtypeStruct((N, 128), _f32)])


@functools.lru_cache(maxsize=None)
def _make_scale_kernel():
  def body(x0_ref, xh_ref, dinv_ref, di_ref, zc_ref, zs_ref):
    zc_ref[...] = dinv_ref[...] * xh_ref[...]
    zs_ref[...] = x0_ref[...] * di_ref[...]

  row = lambda i: (i, 0)
  return pl.pallas_call(
      body, grid=(NBLK,),
      in_specs=[pl.BlockSpec((BR, 128), row), pl.BlockSpec((BR, 128), row),
                pl.BlockSpec((BR, 1), row), pl.BlockSpec((BR, 1), row)],
      out_specs=[pl.BlockSpec((BR, 128), row), pl.BlockSpec((BR, 128), row)],
      out_shape=[jax.ShapeDtypeStruct((N, 128), _f32)] * 2)


def kernel(x, edge_index, W_enc, b_enc, W0, b0, W1, b1):
  src_t = edge_index[0].reshape(NW, K, EB)
  dst_t = edge_index[1].reshape(NW, K, EB)

  dout_p, din_p = _deg_kernel(src_t, dst_t)
  dout = (dout_p[0] + dout_p[1])[:N, 0] + 1.0
  din = (din_p[0] + din_p[1])[:N, 0] + 1.0
  v = lax.rsqrt(dout)
  dinv = lax.rsqrt(din)
  dout_inv = 1.0 / dout
  col = lambda a: a.reshape(N, 1)

  v16 = jnp.zeros((NP, 16), _f32).at[:N, :].set(v[:, None])
  win_p, wout_p = _vpass_kernel(v16, src_t, dst_t)
  wsum = v * ((win_p[0] + win_p[1])[:N, 0] + (wout_p[0] + wout_p[1])[:N, 0])
  degsum = (dout - 1.0) + (din - 1.0)

  def e_rw(n, nf, cross):
    return 0.5 * (_vdot(degsum, n) - 2.0 * cross) / nf

  def e_sym(n, nf, cross):
    return 0.5 * (_vdot(n * dout_inv, wsum) - 2.0 * cross) / nf

  x0, n0c, xh0 = _make_enc_kernel()(x, W_enc, b_enc.reshape(1, 128), W0)
  zc0, zs0 = _make_scale_kernel()(x0, xh0, col(dinv), col(dout_inv))
  n0 = n0c[:, 0]
  nf0 = jnp.sum(n0)
  (rc0,) = _make_block_kernel(1)(zc0, src_t, dst_t)
  (rsym0,) = _make_block_kernel(1)(zs0, src_t, dst_t)

  st1 = _make_stage_kernel(False, True, True, 1)
  (h1, n1c, rs1, xh1, zc1, zs1, G1, cs1, x_sym0) = st1(
      rc0, xh0, b0.reshape(1, 128), col(dinv), col(dout_inv), W1, zs0, rsym0)
  n1 = n1c[:, 0]
  nf1 = jnp.sum(n1)
  e0 = e_sym(n0, nf0, x_sym0[0, 0])
  (rc1,) = _make_block_kernel(1)(zc1, src_t, dst_t)
  rrw1, rsym1 = _make_block_kernel(2)(h1, zs1, src_t, dst_t)
  rank1 = _rank_diff(h1, G1, rs1, cs1)

  st2 = _make_stage_kernel(False, False, True, 2)
  (h2, n2c, rs2, zs2, G2, cs2, x_rw1, x_sym1) = st2(
      rc1, xh1, b1.reshape(1, 128), col(dinv), col(dout_inv),
      h1, rrw1, zs1, rsym1)
  n2 = n2c[:, 0]
  nf2 = jnp.sum(n2)
  erw1 = e_rw(n1, nf1, x_rw1[0, 0])
  esym1 = e_sym(n1, nf1, x_sym1[0, 0])
  rrw2, rsym2 = _make_block_kernel(2)(h2, zs2, src_t, dst_t)
  rank2 = _rank_diff(h2, G2, rs2, cs2)
  x_rw2, x_sym2 = _cross2(h2, rrw2, zs2, rsym2)
  erw2 = e_rw(n2, nf2, x_rw2)
  esym2 = e_sym(n2, nf2, x_sym2)

  return (h2,
          jnp.stack([e0, erw1, erw2]),
          jnp.stack([esym1, esym2]),
          jnp.stack([rank1, rank2]))


# speedup vs baseline: 32.9618x; 1.0722x over previous
import functools

import jax
import jax.numpy as jnp
from jax import lax
from jax.experimental import pallas as pl
from jax.experimental.pallas import tpu as pltpu
from jax.experimental.pallas import tpu_sc as plsc

N = 10000
E = 320000
NC, NS = 2, 16
NW = NC * NS
EB = 80
K = E // (NW * EB)
EBD = 1000
KD = E // (NW * EBD)
EBV = 500
KV = E // (NW * EBV)
NP = 10112
RPW = NP // NS
T64 = RPW // 64
REM = RPW - T64 * 64

_mesh = plsc.VectorSubcoreMesh(
    core_axis_name="c", subcore_axis_name="s", num_cores=NC, num_subcores=NS)

_f32 = jnp.float32
_sc_params = pltpu.CompilerParams(use_tc_tiling_on_sc=False)
_HI = jax.lax.Precision.HIGHEST


def _vdot(a, b):
  return jnp.sum(a * b)


def _fill(ref, rows, cols, value):
  def body(i, _):
    r = i // (cols // 16)
    c0 = (i % (cols // 16)) * 16
    ref[r, pl.ds(c0, 16)] = jnp.full((16,), value, _f32)
    return _
  lax.fori_loop(0, rows * (cols // 16), body, 0)


def _zero_acc(acc, s, zero_v):
  base = s * RPW
  for t in range(T64):
    pltpu.sync_copy(zero_v, acc.at[pl.ds(base + t * 64, 64)])
  pltpu.sync_copy(zero_v.at[pl.ds(0, REM)],
                  acc.at[pl.ds(base + T64 * 64, REM)])


def _writeback(acc, out, c, s):
  base = s * RPW
  pltpu.sync_copy(acc.at[pl.ds(base, RPW)], out.at[c, pl.ds(base, RPW)])


@functools.partial(
    pl.kernel,
    out_type=(jax.ShapeDtypeStruct((NC, NP, 16), _f32),
              jax.ShapeDtypeStruct((NC, NP, 16), _f32)),
    mesh=_mesh,
    compiler_params=_sc_params,
    scratch_types=[
        pltpu.VMEM((KD, EBD), jnp.int32),
        pltpu.VMEM((KD, EBD), jnp.int32),
        pltpu.VMEM((EBD, 16), _f32),
        pltpu.VMEM((64, 16), _f32),
        pltpu.VMEM_SHARED((NP, 16), _f32),
        pltpu.VMEM_SHARED((NP, 16), _f32),
    ],
)
def _deg_kernel(src_hbm, dst_hbm, dout_hbm, din_hbm,
                sidx, didx, ones_v, zero16, acc_o, acc_i):
  c = lax.axis_index("c")
  s = lax.axis_index("s")
  w = s * NC + c
  pltpu.sync_copy(src_hbm.at[w], sidx)
  pltpu.sync_copy(dst_hbm.at[w], didx)
  _fill(ones_v, EBD, 16, 1.0)
  _fill(zero16, 64, 16, 0.0)
  base = s * RPW
  for t in range(T64):
    pltpu.sync_copy(zero16, acc_o.at[pl.ds(base + t * 64, 64)])
    pltpu.sync_copy(zero16, acc_i.at[pl.ds(base + t * 64, 64)])
  pltpu.sync_copy(zero16.at[pl.ds(0, REM)], acc_o.at[pl.ds(base + T64 * 64, REM)])
  pltpu.sync_copy(zero16.at[pl.ds(0, REM)], acc_i.at[pl.ds(base + T64 * 64, REM)])
  plsc.subcore_barrier()

  def body(k, _):
    pltpu.sync_copy(ones_v, acc_o.at[sidx.at[k]], add=True)
    pltpu.sync_copy(ones_v, acc_i.at[didx.at[k]], add=True)
    return _
  lax.fori_loop(0, KD, body, 0)
  plsc.subcore_barrier()
  pltpu.sync_copy(acc_o.at[pl.ds(base, RPW)], dout_hbm.at[c, pl.ds(base, RPW)])
  pltpu.sync_copy(acc_i.at[pl.ds(base, RPW)], din_hbm.at[c, pl.ds(base, RPW)])


@functools.partial(
    pl.kernel,
    out_type=(jax.ShapeDtypeStruct((NC, NP, 16), _f32),
              jax.ShapeDtypeStruct((NC, NP, 16), _f32)),
    mesh=_mesh,
    compiler_params=_sc_params,
    scratch_types=[
        pltpu.VMEM((KV, EBV), jnp.int32),
        pltpu.VMEM((KV, EBV), jnp.int32),
        pltpu.VMEM((EBV, 16), _f32),
        pltpu.VMEM((EBV, 16), _f32),
        pltpu.VMEM((64, 16), _f32),
        pltpu.VMEM_SHARED((NP, 16), _f32),
        pltpu.VMEM_SHARED((NP, 16), _f32),
        pltpu.SemaphoreType.DMA,
        pltpu.SemaphoreType.DMA,
    ],
)
def _vpass_kernel(v16_hbm, src_hbm, dst_hbm, win_hbm, wout_hbm,
                  sidx, didx, rows_f, rows_r, zero16, acc_f, acc_r, sem_f, sem_r):
  c = lax.axis_index("c")
  s = lax.axis_index("s")
  w = s * NC + c
  pltpu.sync_copy(src_hbm.at[w], sidx)
  pltpu.sync_copy(dst_hbm.at[w], didx)
  _fill(zero16, 64, 16, 0.0)
  base = s * RPW
  for t in range(T64):
    pltpu.sync_copy(zero16, acc_f.at[pl.ds(base + t * 64, 64)])
    pltpu.sync_copy(zero16, acc_r.at[pl.ds(base + t * 64, 64)])
  pltpu.sync_copy(zero16.at[pl.ds(0, REM)], acc_f.at[pl.ds(base + T64 * 64, REM)])
  pltpu.sync_copy(zero16.at[pl.ds(0, REM)], acc_r.at[pl.ds(base + T64 * 64, REM)])
  plsc.subcore_barrier()

  pltpu.async_copy(v16_hbm.at[sidx.at[0]], rows_f, sem_f)
  pltpu.async_copy(v16_hbm.at[didx.at[0]], rows_r, sem_r)

  def body(k, _):
    pltpu.make_async_copy(v16_hbm.at[sidx.at[k]], rows_f, sem_f).wait()
    pltpu.sync_copy(rows_f, acc_f.at[didx.at[k]], add=True)

    @pl.when(k + 1 < KV)
    def _f():
      pltpu.async_copy(v16_hbm.at[sidx.at[k + 1]], rows_f, sem_f)
    pltpu.make_async_copy(v16_hbm.at[didx.at[k]], rows_r, sem_r).wait()
    pltpu.sync_copy(rows_r, acc_r.at[sidx.at[k]], add=True)

    @pl.when(k + 1 < KV)
    def _r():
      pltpu.async_copy(v16_hbm.at[didx.at[k + 1]], rows_r, sem_r)
    return _
  lax.fori_loop(0, KV, body, 0)
  plsc.subcore_barrier()
  pltpu.sync_copy(acc_f.at[pl.ds(base, RPW)], win_hbm.at[c, pl.ds(base, RPW)])
  pltpu.sync_copy(acc_r.at[pl.ds(base, RPW)], wout_hbm.at[c, pl.ds(base, RPW)])


@functools.lru_cache(maxsize=None)
def _make_block_kernel(nblocks):
  out_type = tuple(jax.ShapeDtypeStruct((NC, NP, 128), _f32)
                   for _ in range(nblocks))

  @functools.partial(
      pl.kernel,
      out_type=out_type,
      mesh=_mesh,
      compiler_params=_sc_params,
      scratch_types=[
          pltpu.VMEM((K, EB), jnp.int32),
          pltpu.VMEM((K, EB), jnp.int32),
          pltpu.VMEM((EB, 128), _f32),
          pltpu.VMEM((EB, 128), _f32),
          pltpu.VMEM((64, 128), _f32),
          pltpu.VMEM_SHARED((NP, 128), _f32),
          pltpu.SemaphoreType.DMA,
          pltpu.SemaphoreType.DMA,
          pltpu.SemaphoreType.DMA,
          pltpu.SemaphoreType.DMA,
      ],
  )
  def _block_kernel(*refs):
    z_hbm = refs[:nblocks]
    src_hbm, dst_hbm = refs[nblocks], refs[nblocks + 1]
    outs = refs[nblocks + 2:2 * nblocks + 2]
    (sidx, didx, rows0, rows1, zero_v, acc,
     sem0, sem1, semw0, semw1) = refs[2 * nblocks + 2:]
    c = lax.axis_index("c")
    s = lax.axis_index("s")
    w = s * NC + c
    pltpu.sync_copy(src_hbm.at[w], sidx)
    pltpu.sync_copy(dst_hbm.at[w], didx)
    _fill(zero_v, 64, 128, 0.0)
    for b in range(nblocks):
      _zero_acc(acc, s, zero_v)
      plsc.subcore_barrier()
      zb = z_hbm[b]
      pltpu.async_copy(zb.at[sidx.at[0]], rows0, sem0)
      pltpu.async_copy(zb.at[sidx.at[1]], rows1, sem1)

      def pair(t, _, zb=zb):
        k0 = 2 * t
        pltpu.make_async_copy(zb.at[sidx.at[k0]], rows0, sem0).wait()
        pltpu.async_copy(rows0, acc.at[didx.at[k0]], semw0, add=True)

        @pl.when(k0 + 2 < K)
        def _fire():
          pltpu.make_async_copy(rows0, acc.at[didx.at[k0]], semw0).wait()
          pltpu.async_copy(zb.at[sidx.at[k0 + 2]], rows0, sem0)
        pltpu.make_async_copy(zb.at[sidx.at[k0 + 1]], rows1, sem1).wait()
        pltpu.async_copy(rows1, acc.at[didx.at[k0 + 1]], semw1, add=True)

        @pl.when(k0 + 3 < K)
        def _fire1():
          pltpu.make_async_copy(rows1, acc.at[didx.at[k0 + 1]], semw1).wait()
          pltpu.async_copy(zb.at[sidx.at[k0 + 3]], rows1, sem1)
        return _
      lax.fori_loop(0, K // 2, pair, 0)
      pltpu.make_async_copy(zb.at[sidx.at[K - 1]], rows0, sem0).wait()
      pltpu.async_copy(rows0, acc.at[didx.at[K - 1]], semw0, add=True)
      pltpu.make_async_copy(rows0, acc.at[didx.at[K - 1]], semw0).wait()
      pltpu.make_async_copy(rows1, acc.at[didx.at[K - 2]], semw1).wait()
      plsc.subcore_barrier()
      _writeback(acc, outs[b], c, s)
      plsc.subcore_barrier()

  return _block_kernel


def _msgpass(blocks, src_t, dst_t):
  kern = _make_block_kernel(len(blocks))
  outs = kern(*blocks, src_t, dst_t)
  return [(o[0] + o[1])[:N] for o in outs]


NBLK = 25
BR = N // NBLK


def _dot(a, b):
  return jnp.dot(a, b, precision=_HI)


@functools.lru_cache(maxsize=None)
def _make_stage_kernel(encoder, with_next, with_rank, ncross):
  def body(*refs):
    idx = 0
    if encoder:
      x_ref, W_ref, b_ref = refs[0:3]
    else:
      rc_ref, xhp_ref, b_ref = refs[0:3]
    idx = 3
    dinv_ref, dout_inv_ref = refs[idx:idx + 2]; idx += 2
    if with_next:
      Wn_ref = refs[idx]; idx += 1
    cross_in = []
    for _ in range(ncross):
      cross_in.append((refs[idx], refs[idx + 1])); idx += 2
    h_ref, n_ref, rs_ref = refs[idx:idx + 3]; idx += 3
    if with_next:
      xh_ref, zc_ref = refs[idx:idx + 2]; idx += 2
    zs_ref = refs[idx]; idx += 1
    if with_rank:
      G_ref, cs_ref = refs[idx:idx + 2]; idx += 2
    cross_out = refs[idx:idx + ncross]

    i = pl.program_id(0)
    dinv = dinv_ref[...]
    if encoder:
      h = _dot(x_ref[...], W_ref[...]) + b_ref[...]
    else:
      pre = dinv * (rc_ref[0] + rc_ref[1]) + dinv * dinv * xhp_ref[...] + b_ref[...]
      h = jnp.maximum(pre, 0.0)
    h_ref[...] = h
    n_ref[...] = jnp.sum(h * h, axis=1, keepdims=True)
    rs_ref[...] = jnp.sum(jnp.abs(h), axis=1, keepdims=True)
    if with_next:
      xh = _dot(h, Wn_ref[...])
      xh_ref[...] = xh
      zc_ref[...] = dinv * xh
    zs_ref[...] = h * dout_inv_ref[...]

    @pl.when(i == 0)
    def _init():
      if with_rank:
        G_ref[...] = jnp.zeros_like(G_ref)
        cs_ref[...] = jnp.zeros_like(cs_ref)
      for co in cross_out:
        co[0, 0] = 0.0

    if with_rank:
      G_ref[...] += _dot(h.T, h)
      cs_ref[...] += jnp.sum(jnp.abs(h), axis=0, keepdims=True)
    for (zp_ref, r_ref), co in zip(cross_in, cross_out):
      co[0, 0] += jnp.sum(zp_ref[...] * (r_ref[0] + r_ref[1]))

  row = lambda i: (i, 0)
  full = lambda i: (0, 0)
  full3 = lambda i: (0, i, 0)
  in_specs = []
  if encoder:
    in_specs += [pl.BlockSpec((BR, 128), row), pl.BlockSpec((128, 128), full),
                 pl.BlockSpec((1, 128), full)]
  else:
    in_specs += [pl.BlockSpec((2, BR, 128), full3), pl.BlockSpec((BR, 128), row),
                 pl.BlockSpec((1, 128), full)]
  in_specs += [pl.BlockSpec((BR, 1), row)] * 2
  if with_next:
    in_specs += [pl.BlockSpec((128, 128), full)]
  for _ in range(ncross):
    in_specs += [pl.BlockSpec((BR, 128), row), pl.BlockSpec((2, BR, 128), full3)]

  out_specs = [pl.BlockSpec((BR, 128), row), pl.BlockSpec((BR, 1), row),
               pl.BlockSpec((BR, 1), row)]
  out_shape = [jax.ShapeDtypeStruct((N, 128), _f32),
               jax.ShapeDtypeStruct((N, 1), _f32),
               jax.ShapeDtypeStruct((N, 1), _f32)]
  if with_next:
    out_specs += [pl.BlockSpec((BR, 128), row)] * 2
    out_shape += [jax.ShapeDtypeStruct((N, 128), _f32)] * 2
  out_specs += [pl.BlockSpec((BR, 128), row)]
  out_shape += [jax.ShapeDtypeStruct((N, 128), _f32)]
  if with_rank:
    out_specs += [pl.BlockSpec((128, 128), full), pl.BlockSpec((1, 128), full)]
    out_shape += [jax.ShapeDtypeStruct((128, 128), _f32),
                  jax.ShapeDtypeStruct((1, 128), _f32)]
  out_specs += [pl.BlockSpec((1, 1), full, memory_space=pltpu.SMEM)] * ncross
  out_shape += [jax.ShapeDtypeStruct((1, 1), _f32)] * ncross

  return pl.pallas_call(body, grid=(NBLK,), in_specs=in_specs,
                        out_specs=out_specs, out_shape=out_shape)


def _cross2(zA, rA, zB, rB):
  def body(zA_ref, rA_ref, zB_ref, rB_ref, a_ref, b_ref):
    i = pl.program_id(0)

    @pl.when(i == 0)
    def _init():
      a_ref[0, 0] = 0.0
      b_ref[0, 0] = 0.0
    a_ref[0, 0] += jnp.sum(zA_ref[...] * (rA_ref[0] + rA_ref[1]))
    b_ref[0, 0] += jnp.sum(zB_ref[...] * (rB_ref[0] + rB_ref[1]))

  row = lambda i: (i, 0)
  full3 = lambda i: (0, i, 0)
  smem = pl.BlockSpec((1, 1), lambda i: (0, 0), memory_space=pltpu.SMEM)
  out = pl.pallas_call(
      body, grid=(NBLK,),
      in_specs=[pl.BlockSpec((BR, 128), row), pl.BlockSpec((2, BR, 128), full3),
                pl.BlockSpec((BR, 128), row), pl.BlockSpec((2, BR, 128), full3)],
      out_specs=[smem, smem],
      out_shape=[jax.ShapeDtypeStruct((1, 1), _f32)] * 2)(zA, rA, zB, rB)
  return out[0][0, 0], out[1][0, 0]


NS_ITERS = 25


def _rank_kernel(G, g, r, cnorm2, rnorm2, sgn):
  def body(G_ref, g_ref, r_ref, sc_ref, I_ref, nu_ref, rank_ref):
    I = I_ref[...]

    def trsqrt(S):
      c = jnp.sum(S * I)
      Y, Z = S / c, I
      for _ in range(NS_ITERS):
        T = 0.5 * (3.0 * I - _dot(Z, Y))
        Y, Z = _dot(Y, T), _dot(T, Z)
      return jnp.sum(Y * I) * jnp.sqrt(c)

    G = G_ref[...]
    nu1 = trsqrt(G)
    nu_ref[0, 0] = nu1
    g_ = g_ref[...]
    r_ = r_ref[...]
    cn2, rn2, sg = sc_ref[0], sc_ref[1], sc_ref[2]
    gr = _dot(g_.T, r_) + _dot(r_.T, g_)
    M = (G / (nu1 * nu1)
         - (sg / (nu1 * jnp.sqrt(cn2 * rn2))) * gr
         + _dot(r_.T, r_) / rn2)
    rank_ref[0, 0] = trsqrt(M)

  nu, rank = pl.pallas_call(
      body,
      in_specs=[pl.BlockSpec(memory_space=pltpu.VMEM),
                pl.BlockSpec(memory_space=pltpu.VMEM),
                pl.BlockSpec(memory_space=pltpu.VMEM),
                pl.BlockSpec(memory_space=pltpu.SMEM),
                pl.BlockSpec(memory_space=pltpu.VMEM)],
      out_specs=[pl.BlockSpec(memory_space=pltpu.SMEM),
                 pl.BlockSpec(memory_space=pltpu.SMEM)],
      out_shape=[jax.ShapeDtypeStruct((1, 1), _f32)] * 2,
  )(G, g.reshape(1, 128), r.reshape(1, 128), jnp.stack([cnorm2, rnorm2, sgn]),
    jnp.eye(128, dtype=_f32))
  return rank[0, 0]


def _rank_diff(h, G, rs, cs):
  i = jnp.argmax(rs[:, 0])
  j = jnp.argmax(cs[0, :])
  r = lax.dynamic_slice(h, (i, 0), (1, 128))[0]
  g = lax.dynamic_slice(G, (0, j), (128, 1))[:, 0]
  cnorm2 = G[j, j]
  rnorm2 = jnp.sum(r * r)
  sgn = jnp.where(h[i, j] < 0, -1.0, 1.0)
  return _rank_kernel(G, g, r, cnorm2, rnorm2, sgn)


@functools.lru_cache(maxsize=None)
def _make_enc_kernel():
  def body(x_ref, W_ref, b_ref, Wn_ref, x0_ref, n_ref, xh_ref):
    x0 = _dot(x_ref[...], W_ref[...]) + b_ref[...]
    x0_ref[...] = x0
    n_ref[...] = jnp.sum(x0 * x0, axis=1, keepdims=True)
    xh_ref[...] = _dot(x0, Wn_ref[...])

  row = lambda i: (i, 0)
  full = lambda i: (0, 0)
  return pl.pallas_call(
      body, grid=(NBLK,),
      in_specs=[pl.BlockSpec((BR, 128), row), pl.BlockSpec((128, 128), full),
                pl.BlockSpec((1, 128), full), pl.BlockSpec((128, 128), full)],
      out_specs=[pl.BlockSpec((BR, 128), row), pl.BlockSpec((BR, 1), row),
                 pl.BlockSpec((BR, 128), row)],
      out_shape=[jax.ShapeDtypeStruct((N, 128), _f32),
                 jax.ShapeDtypeStruct((N, 1), _f32),
                 jax.ShapeDtypeStruct((N, 128), _f32)])


@functools.lru_cache(maxsize=None)
def _make_scale_kernel():
  def body(x0_ref, xh_ref, dinv_ref, di_ref, zc_ref, zs_ref):
    zc_ref[...] = dinv_ref[...] * xh_ref[...]
    zs_ref[...] = x0_ref[...] * di_ref[...]

  row = lambda i: (i, 0)
  return pl.pallas_call(
      body, grid=(NBLK,),
      in_specs=[pl.BlockSpec((BR, 128), row), pl.BlockSpec((BR, 128), row),
                pl.BlockSpec((BR, 1), row), pl.BlockSpec((BR, 1), row)],
      out_specs=[pl.BlockSpec((BR, 128), row), pl.BlockSpec((BR, 128), row)],
      out_shape=[jax.ShapeDtypeStruct((N, 128), _f32)] * 2)


def kernel(x, edge_index, W_enc, b_enc, W0, b0, W1, b1):
  src_t = edge_index[0].reshape(NW, K, EB)
  dst_t = edge_index[1].reshape(NW, K, EB)
  src_d = edge_index[0].reshape(NW, KD, EBD)
  dst_d = edge_index[1].reshape(NW, KD, EBD)
  src_v = edge_index[0].reshape(NW, KV, EBV)
  dst_v = edge_index[1].reshape(NW, KV, EBV)

  dout_p, din_p = _deg_kernel(src_d, dst_d)
  dout = (dout_p[0] + dout_p[1])[:N, 0] + 1.0
  din = (din_p[0] + din_p[1])[:N, 0] + 1.0
  v = lax.rsqrt(dout)
  dinv = lax.rsqrt(din)
  dout_inv = 1.0 / dout
  col = lambda a: a.reshape(N, 1)

  v16 = jnp.zeros((NP, 16), _f32).at[:N, :].set(v[:, None])
  win_p, wout_p = _vpass_kernel(v16, src_v, dst_v)
  wsum = v * ((win_p[0] + win_p[1])[:N, 0] + (wout_p[0] + wout_p[1])[:N, 0])
  degsum = (dout - 1.0) + (din - 1.0)

  def e_rw(n, nf, cross):
    return 0.5 * (_vdot(degsum, n) - 2.0 * cross) / nf

  def e_sym(n, nf, cross):
    return 0.5 * (_vdot(n * dout_inv, wsum) - 2.0 * cross) / nf

  x0, n0c, xh0 = _make_enc_kernel()(x, W_enc, b_enc.reshape(1, 128), W0)
  zc0, zs0 = _make_scale_kernel()(x0, xh0, col(dinv), col(dout_inv))
  n0 = n0c[:, 0]
  nf0 = jnp.sum(n0)
  (rc0,) = _make_block_kernel(1)(zc0, src_t, dst_t)
  (rsym0,) = _make_block_kernel(1)(zs0, src_t, dst_t)

  st1 = _make_stage_kernel(False, True, True, 1)
  (h1, n1c, rs1, xh1, zc1, zs1, G1, cs1, x_sym0) = st1(
      rc0, xh0, b0.reshape(1, 128), col(dinv), col(dout_inv), W1, zs0, rsym0)
  n1 = n1c[:, 0]
  nf1 = jnp.sum(n1)
  e0 = e_sym(n0, nf0, x_sym0[0, 0])
  (rc1,) = _make_block_kernel(1)(zc1, src_t, dst_t)
  rrw1, rsym1 = _make_block_kernel(2)(h1, zs1, src_t, dst_t)
  rank1 = _rank_diff(h1, G1, rs1, cs1)

  st2 = _make_stage_kernel(False, False, True, 2)
  (h2, n2c, rs2, zs2, G2, cs2, x_rw1, x_sym1) = st2(
      rc1, xh1, b1.reshape(1, 128), col(dinv), col(dout_inv),
      h1, rrw1, zs1, rsym1)
  n2 = n2c[:, 0]
  nf2 = jnp.sum(n2)
  erw1 = e_rw(n1, nf1, x_rw1[0, 0])
  esym1 = e_sym(n1, nf1, x_sym1[0, 0])
  rrw2, rsym2 = _make_block_kernel(2)(h2, zs2, src_t, dst_t)
  rank2 = _rank_diff(h2, G2, rs2, cs2)
  x_rw2, x_sym2 = _cross2(h2, rrw2, zs2, rsym2)
  erw2 = e_rw(n2, nf2, x_rw2)
  esym2 = e_sym(n2, nf2, x_sym2)

  return (h2,
          jnp.stack([e0, erw1, erw2]),
          jnp.stack([esym1, esym2]),
          jnp.stack([rank1, rank2]))


# speedup vs baseline: 33.4852x; 1.0159x over previous
import functools

import jax
import jax.numpy as jnp
from jax import lax
from jax.experimental import pallas as pl
from jax.experimental.pallas import tpu as pltpu
from jax.experimental.pallas import tpu_sc as plsc

N = 10000
E = 320000
NC, NS = 2, 16
NW = NC * NS
EB = 80
K = E // (NW * EB)
EBD = 1000
KD = E // (NW * EBD)
EBV = 500
KV = E // (NW * EBV)
NP = 10112
RPW = NP // NS
T64 = RPW // 64
REM = RPW - T64 * 64

_mesh = plsc.VectorSubcoreMesh(
    core_axis_name="c", subcore_axis_name="s", num_cores=NC, num_subcores=NS)

_f32 = jnp.float32
_sc_params = pltpu.CompilerParams(use_tc_tiling_on_sc=False)
_HI = jax.lax.Precision.HIGHEST


def _vdot(a, b):
  return jnp.sum(a * b)


def _fill(ref, rows, cols, value):
  def body(i, _):
    r = i // (cols // 16)
    c0 = (i % (cols // 16)) * 16
    ref[r, pl.ds(c0, 16)] = jnp.full((16,), value, _f32)
    return _
  lax.fori_loop(0, rows * (cols // 16), body, 0)


def _zero_acc(acc, s, zero_v):
  base = s * RPW
  for t in range(T64):
    pltpu.sync_copy(zero_v, acc.at[pl.ds(base + t * 64, 64)])
  pltpu.sync_copy(zero_v.at[pl.ds(0, REM)],
                  acc.at[pl.ds(base + T64 * 64, REM)])


def _writeback(acc, out, c, s):
  base = s * RPW
  pltpu.sync_copy(acc.at[pl.ds(base, RPW)], out.at[c, pl.ds(base, RPW)])


@functools.partial(
    pl.kernel,
    out_type=(jax.ShapeDtypeStruct((NC, NP, 16), _f32),
              jax.ShapeDtypeStruct((NC, NP, 16), _f32)),
    mesh=_mesh,
    compiler_params=_sc_params,
    scratch_types=[
        pltpu.VMEM((KD, EBD), jnp.int32),
        pltpu.VMEM((KD, EBD), jnp.int32),
        pltpu.VMEM((EBD, 16), _f32),
        pltpu.VMEM((64, 16), _f32),
        pltpu.VMEM_SHARED((NP, 16), _f32),
        pltpu.VMEM_SHARED((NP, 16), _f32),
    ],
)
def _deg_kernel(src_hbm, dst_hbm, dout_hbm, din_hbm,
                sidx, didx, ones_v, zero16, acc_o, acc_i):
  c = lax.axis_index("c")
  s = lax.axis_index("s")
  w = s * NC + c
  pltpu.sync_copy(src_hbm.at[w], sidx)
  pltpu.sync_copy(dst_hbm.at[w], didx)
  _fill(ones_v, EBD, 16, 1.0)
  _fill(zero16, 64, 16, 0.0)
  base = s * RPW
  for t in range(T64):
    pltpu.sync_copy(zero16, acc_o.at[pl.ds(base + t * 64, 64)])
    pltpu.sync_copy(zero16, acc_i.at[pl.ds(base + t * 64, 64)])
  pltpu.sync_copy(zero16.at[pl.ds(0, REM)], acc_o.at[pl.ds(base + T64 * 64, REM)])
  pltpu.sync_copy(zero16.at[pl.ds(0, REM)], acc_i.at[pl.ds(base + T64 * 64, REM)])
  plsc.subcore_barrier()

  def body(k, _):
    pltpu.sync_copy(ones_v, acc_o.at[sidx.at[k]], add=True)
    pltpu.sync_copy(ones_v, acc_i.at[didx.at[k]], add=True)
    return _
  lax.fori_loop(0, KD, body, 0)
  plsc.subcore_barrier()
  pltpu.sync_copy(acc_o.at[pl.ds(base, RPW)], dout_hbm.at[c, pl.ds(base, RPW)])
  pltpu.sync_copy(acc_i.at[pl.ds(base, RPW)], din_hbm.at[c, pl.ds(base, RPW)])


@functools.partial(
    pl.kernel,
    out_type=(jax.ShapeDtypeStruct((NC, NP, 16), _f32),
              jax.ShapeDtypeStruct((NC, NP, 16), _f32)),
    mesh=_mesh,
    compiler_params=_sc_params,
    scratch_types=[
        pltpu.VMEM((KV, EBV), jnp.int32),
        pltpu.VMEM((KV, EBV), jnp.int32),
        pltpu.VMEM((EBV, 16), _f32),
        pltpu.VMEM((EBV, 16), _f32),
        pltpu.VMEM((64, 16), _f32),
        pltpu.VMEM_SHARED((NP, 16), _f32),
        pltpu.VMEM_SHARED((NP, 16), _f32),
        pltpu.SemaphoreType.DMA,
        pltpu.SemaphoreType.DMA,
    ],
)
def _vpass_kernel(v16_hbm, src_hbm, dst_hbm, win_hbm, wout_hbm,
                  sidx, didx, rows_f, rows_r, zero16, acc_f, acc_r, sem_f, sem_r):
  c = lax.axis_index("c")
  s = lax.axis_index("s")
  w = s * NC + c
  pltpu.sync_copy(src_hbm.at[w], sidx)
  pltpu.sync_copy(dst_hbm.at[w], didx)
  _fill(zero16, 64, 16, 0.0)
  base = s * RPW
  for t in range(T64):
    pltpu.sync_copy(zero16, acc_f.at[pl.ds(base + t * 64, 64)])
    pltpu.sync_copy(zero16, acc_r.at[pl.ds(base + t * 64, 64)])
  pltpu.sync_copy(zero16.at[pl.ds(0, REM)], acc_f.at[pl.ds(base + T64 * 64, REM)])
  pltpu.sync_copy(zero16.at[pl.ds(0, REM)], acc_r.at[pl.ds(base + T64 * 64, REM)])
  plsc.subcore_barrier()

  pltpu.async_copy(v16_hbm.at[sidx.at[0]], rows_f, sem_f)
  pltpu.async_copy(v16_hbm.at[didx.at[0]], rows_r, sem_r)

  def body(k, _):
    pltpu.make_async_copy(v16_hbm.at[sidx.at[k]], rows_f, sem_f).wait()
    pltpu.sync_copy(rows_f, acc_f.at[didx.at[k]], add=True)

    @pl.when(k + 1 < KV)
    def _f():
      pltpu.async_copy(v16_hbm.at[sidx.at[k + 1]], rows_f, sem_f)
    pltpu.make_async_copy(v16_hbm.at[didx.at[k]], rows_r, sem_r).wait()
    pltpu.sync_copy(rows_r, acc_r.at[sidx.at[k]], add=True)

    @pl.when(k + 1 < KV)
    def _r():
      pltpu.async_copy(v16_hbm.at[didx.at[k + 1]], rows_r, sem_r)
    return _
  lax.fori_loop(0, KV, body, 0)
  plsc.subcore_barrier()
  pltpu.sync_copy(acc_f.at[pl.ds(base, RPW)], win_hbm.at[c, pl.ds(base, RPW)])
  pltpu.sync_copy(acc_r.at[pl.ds(base, RPW)], wout_hbm.at[c, pl.ds(base, RPW)])


@functools.lru_cache(maxsize=None)
def _make_block_kernel(nblocks):
  out_type = tuple(jax.ShapeDtypeStruct((NC, NP, 128), _f32)
                   for _ in range(nblocks))

  @functools.partial(
      pl.kernel,
      out_type=out_type,
      mesh=_mesh,
      compiler_params=_sc_params,
      scratch_types=[
          pltpu.VMEM((K, EB), jnp.int32),
          pltpu.VMEM((K, EB), jnp.int32),
          pltpu.VMEM((EB, 128), _f32),
          pltpu.VMEM((EB, 128), _f32),
          pltpu.VMEM((64, 128), _f32),
          pltpu.VMEM_SHARED((NP, 128), _f32),
          pltpu.SemaphoreType.DMA,
          pltpu.SemaphoreType.DMA,
          pltpu.SemaphoreType.DMA,
          pltpu.SemaphoreType.DMA,
      ],
  )
  def _block_kernel(*refs):
    z_hbm = refs[:nblocks]
    src_hbm, dst_hbm = refs[nblocks], refs[nblocks + 1]
    outs = refs[nblocks + 2:2 * nblocks + 2]
    (sidx, didx, rows0, rows1, zero_v, acc,
     sem0, sem1, semw0, semw1) = refs[2 * nblocks + 2:]
    c = lax.axis_index("c")
    s = lax.axis_index("s")
    w = s * NC + c
    pltpu.sync_copy(src_hbm.at[w], sidx)
    pltpu.sync_copy(dst_hbm.at[w], didx)
    _fill(zero_v, 64, 128, 0.0)
    for b in range(nblocks):
      _zero_acc(acc, s, zero_v)
      plsc.subcore_barrier()
      zb = z_hbm[b]
      pltpu.async_copy(zb.at[sidx.at[0]], rows0, sem0)
      pltpu.async_copy(zb.at[sidx.at[1]], rows1, sem1)

      def pair(t, _, zb=zb):
        k0 = 2 * t
        pltpu.make_async_copy(zb.at[sidx.at[k0]], rows0, sem0).wait()
        pltpu.async_copy(rows0, acc.at[didx.at[k0]], semw0, add=True)

        @pl.when(k0 + 2 < K)
        def _fire():
          pltpu.make_async_copy(rows0, acc.at[didx.at[k0]], semw0).wait()
          pltpu.async_copy(zb.at[sidx.at[k0 + 2]], rows0, sem0)
        pltpu.make_async_copy(zb.at[sidx.at[k0 + 1]], rows1, sem1).wait()
        pltpu.async_copy(rows1, acc.at[didx.at[k0 + 1]], semw1, add=True)

        @pl.when(k0 + 3 < K)
        def _fire1():
          pltpu.make_async_copy(rows1, acc.at[didx.at[k0 + 1]], semw1).wait()
          pltpu.async_copy(zb.at[sidx.at[k0 + 3]], rows1, sem1)
        return _
      lax.fori_loop(0, K // 2, pair, 0)
      pltpu.make_async_copy(zb.at[sidx.at[K - 1]], rows0, sem0).wait()
      pltpu.async_copy(rows0, acc.at[didx.at[K - 1]], semw0, add=True)
      pltpu.make_async_copy(rows0, acc.at[didx.at[K - 1]], semw0).wait()
      pltpu.make_async_copy(rows1, acc.at[didx.at[K - 2]], semw1).wait()
      plsc.subcore_barrier()
      _writeback(acc, outs[b], c, s)
      plsc.subcore_barrier()

  return _block_kernel


def _msgpass(blocks, src_t, dst_t):
  kern = _make_block_kernel(len(blocks))
  outs = kern(*blocks, src_t, dst_t)
  return [(o[0] + o[1])[:N] for o in outs]


NBLK = 25
BR = N // NBLK


def _dot(a, b):
  return jnp.dot(a, b, precision=_HI)


@functools.lru_cache(maxsize=None)
def _make_stage_kernel(encoder, with_next, with_rank, ncross):
  def body(*refs):
    idx = 0
    if encoder:
      x_ref, W_ref, b_ref = refs[0:3]
    else:
      rc_ref, xhp_ref, b_ref = refs[0:3]
    idx = 3
    dinv_ref, dout_inv_ref = refs[idx:idx + 2]; idx += 2
    if with_next:
      Wn_ref = refs[idx]; idx += 1
    cross_in = []
    for _ in range(ncross):
      cross_in.append((refs[idx], refs[idx + 1])); idx += 2
    h_ref, n_ref, rs_ref = refs[idx:idx + 3]; idx += 3
    if with_next:
      xh_ref, zc_ref = refs[idx:idx + 2]; idx += 2
    zs_ref = refs[idx]; idx += 1
    if with_rank:
      G_ref, cs_ref = refs[idx:idx + 2]; idx += 2
    cross_out = refs[idx:idx + ncross]

    i = pl.program_id(0)
    dinv = dinv_ref[...]
    if encoder:
      h = _dot(x_ref[...], W_ref[...]) + b_ref[...]
    else:
      pre = dinv * (rc_ref[0] + rc_ref[1]) + dinv * dinv * xhp_ref[...] + b_ref[...]
      h = jnp.maximum(pre, 0.0)
    h_ref[...] = h
    n_ref[...] = jnp.sum(h * h, axis=1, keepdims=True)
    rs_ref[...] = jnp.sum(jnp.abs(h), axis=1, keepdims=True)
    if with_next:
      xh = _dot(h, Wn_ref[...])
      xh_ref[...] = xh
      zc_ref[...] = dinv * xh
    zs_ref[...] = h * dout_inv_ref[...]

    @pl.when(i == 0)
    def _init():
      if with_rank:
        G_ref[...] = jnp.zeros_like(G_ref)
        cs_ref[...] = jnp.zeros_like(cs_ref)
      for co in cross_out:
        co[0, 0] = 0.0

    if with_rank:
      G_ref[...] += _dot(h.T, h)
      cs_ref[...] += jnp.sum(jnp.abs(h), axis=0, keepdims=True)
    for (zp_ref, r_ref), co in zip(cross_in, cross_out):
      co[0, 0] += jnp.sum(zp_ref[...] * (r_ref[0] + r_ref[1]))

  row = lambda i: (i, 0)
  full = lambda i: (0, 0)
  full3 = lambda i: (0, i, 0)
  in_specs = []
  if encoder:
    in_specs += [pl.BlockSpec((BR, 128), row), pl.BlockSpec((128, 128), full),
                 pl.BlockSpec((1, 128), full)]
  else:
    in_specs += [pl.BlockSpec((2, BR, 128), full3), pl.BlockSpec((BR, 128), row),
                 pl.BlockSpec((1, 128), full)]
  in_specs += [pl.BlockSpec((BR, 1), row)] * 2
  if with_next:
    in_specs += [pl.BlockSpec((128, 128), full)]
  for _ in range(ncross):
    in_specs += [pl.BlockSpec((BR, 128), row), pl.BlockSpec((2, BR, 128), full3)]

  out_specs = [pl.BlockSpec((BR, 128), row), pl.BlockSpec((BR, 1), row),
               pl.BlockSpec((BR, 1), row)]
  out_shape = [jax.ShapeDtypeStruct((N, 128), _f32),
               jax.ShapeDtypeStruct((N, 1), _f32),
               jax.ShapeDtypeStruct((N, 1), _f32)]
  if with_next:
    out_specs += [pl.BlockSpec((BR, 128), row)] * 2
    out_shape += [jax.ShapeDtypeStruct((N, 128), _f32)] * 2
  out_specs += [pl.BlockSpec((BR, 128), row)]
  out_shape += [jax.ShapeDtypeStruct((N, 128), _f32)]
  if with_rank:
    out_specs += [pl.BlockSpec((128, 128), full), pl.BlockSpec((1, 128), full)]
    out_shape += [jax.ShapeDtypeStruct((128, 128), _f32),
                  jax.ShapeDtypeStruct((1, 128), _f32)]
  out_specs += [pl.BlockSpec((1, 1), full, memory_space=pltpu.SMEM)] * ncross
  out_shape += [jax.ShapeDtypeStruct((1, 1), _f32)] * ncross

  return pl.pallas_call(body, grid=(NBLK,), in_specs=in_specs,
                        out_specs=out_specs, out_shape=out_shape)


def _cross2(zA, rA, zB, rB):
  def body(zA_ref, rA_ref, zB_ref, rB_ref, a_ref, b_ref):
    i = pl.program_id(0)

    @pl.when(i == 0)
    def _init():
      a_ref[0, 0] = 0.0
      b_ref[0, 0] = 0.0
    a_ref[0, 0] += jnp.sum(zA_ref[...] * (rA_ref[0] + rA_ref[1]))
    b_ref[0, 0] += jnp.sum(zB_ref[...] * (rB_ref[0] + rB_ref[1]))

  row = lambda i: (i, 0)
  full3 = lambda i: (0, i, 0)
  smem = pl.BlockSpec((1, 1), lambda i: (0, 0), memory_space=pltpu.SMEM)
  out = pl.pallas_call(
      body, grid=(NBLK,),
      in_specs=[pl.BlockSpec((BR, 128), row), pl.BlockSpec((2, BR, 128), full3),
                pl.BlockSpec((BR, 128), row), pl.BlockSpec((2, BR, 128), full3)],
      out_specs=[smem, smem],
      out_shape=[jax.ShapeDtypeStruct((1, 1), _f32)] * 2)(zA, rA, zB, rB)
  return out[0][0, 0], out[1][0, 0]


NS_ITERS = 25


def _rank_kernel(G, g, r, cnorm2, rnorm2, sgn):
  def body(G_ref, g_ref, r_ref, sc_ref, I_ref, nu_ref, rank_ref):
    I = I_ref[...]

    def trsqrt(S):
      c = jnp.sum(S * I)
      Y, Z = S / c, I
      for _ in range(NS_ITERS):
        T = 0.5 * (3.0 * I - _dot(Z, Y))
        Y, Z = _dot(Y, T), _dot(T, Z)
      return jnp.sum(Y * I) * jnp.sqrt(c)

    G = G_ref[...]
    nu1 = trsqrt(G)
    nu_ref[0, 0] = nu1
    g_ = g_ref[...]
    r_ = r_ref[...]
    cn2, rn2, sg = sc_ref[0], sc_ref[1], sc_ref[2]
    gr = _dot(g_.T, r_) + _dot(r_.T, g_)
    M = (G / (nu1 * nu1)
         - (sg / (nu1 * jnp.sqrt(cn2 * rn2))) * gr
         + _dot(r_.T, r_) / rn2)
    rank_ref[0, 0] = trsqrt(M)

  nu, rank = pl.pallas_call(
      body,
      in_specs=[pl.BlockSpec(memory_space=pltpu.VMEM),
                pl.BlockSpec(memory_space=pltpu.VMEM),
                pl.BlockSpec(memory_space=pltpu.VMEM),
                pl.BlockSpec(memory_space=pltpu.SMEM),
                pl.BlockSpec(memory_space=pltpu.VMEM)],
      out_specs=[pl.BlockSpec(memory_space=pltpu.SMEM),
                 pl.BlockSpec(memory_space=pltpu.SMEM)],
      out_shape=[jax.ShapeDtypeStruct((1, 1), _f32)] * 2,
  )(G, g.reshape(1, 128), r.reshape(1, 128), jnp.stack([cnorm2, rnorm2, sgn]),
    jnp.eye(128, dtype=_f32))
  return rank[0, 0]


def _rank_diff(h, G, rs, cs):
  i = jnp.argmax(rs[:, 0])
  j = jnp.argmax(cs[0, :])
  r = lax.dynamic_slice(h, (i, 0), (1, 128))[0]
  g = lax.dynamic_slice(G, (0, j), (128, 1))[:, 0]
  cnorm2 = G[j, j]
  rnorm2 = jnp.sum(r * r)
  sgn = jnp.where(h[i, j] < 0, -1.0, 1.0)
  return _rank_kernel(G, g, r, cnorm2, rnorm2, sgn)


@functools.lru_cache(maxsize=None)
def _make_enc_kernel():
  def body(x_ref, W_ref, b_ref, Wn_ref, x0_ref, n_ref, xh_ref):
    x0 = _dot(x_ref[...], W_ref[...]) + b_ref[...]
    x0_ref[...] = x0
    n_ref[...] = jnp.sum(x0 * x0, axis=1, keepdims=True)
    xh_ref[...] = _dot(x0, Wn_ref[...])

  row = lambda i: (i, 0)
  full = lambda i: (0, 0)
  return pl.pallas_call(
      body, grid=(NBLK,),
      in_specs=[pl.BlockSpec((BR, 128), row), pl.BlockSpec((128, 128), full),
                pl.BlockSpec((1, 128), full), pl.BlockSpec((128, 128), full)],
      out_specs=[pl.BlockSpec((BR, 128), row), pl.BlockSpec((BR, 1), row),
                 pl.BlockSpec((BR, 128), row)],
      out_shape=[jax.ShapeDtypeStruct((N, 128), _f32),
                 jax.ShapeDtypeStruct((N, 1), _f32),
                 jax.ShapeDtypeStruct((N, 128), _f32)])


@functools.lru_cache(maxsize=None)
def _make_scale_kernel():
  def body(x0_ref, xh_ref, dinv_ref, di_ref, zc_ref, zs_ref):
    zc_ref[...] = dinv_ref[...] * xh_ref[...]
    zs_ref[...] = x0_ref[...] * di_ref[...]

  row = lambda i: (i, 0)
  return pl.pallas_call(
      body, grid=(NBLK,),
      in_specs=[pl.BlockSpec((BR, 128), row), pl.BlockSpec((BR, 128), row),
                pl.BlockSpec((BR, 1), row), pl.BlockSpec((BR, 1), row)],
      out_specs=[pl.BlockSpec((BR, 128), row), pl.BlockSpec((BR, 128), row)],
      out_shape=[jax.ShapeDtypeStruct((N, 128), _f32)] * 2)


def kernel(x, edge_index, W_enc, b_enc, W0, b0, W1, b1):
  src_t = edge_index[0].reshape(NW, K, EB)
  dst_t = edge_index[1].reshape(NW, K, EB)
  src_d = edge_index[0].reshape(NW, KD, EBD)
  dst_d = edge_index[1].reshape(NW, KD, EBD)
  src_v = edge_index[0].reshape(NW, KV, EBV)
  dst_v = edge_index[1].reshape(NW, KV, EBV)

  dout_p, din_p = _deg_kernel(src_d, dst_d)
  dout = (dout_p[0] + dout_p[1])[:N, 0] + 1.0
  din = (din_p[0] + din_p[1])[:N, 0] + 1.0
  v = lax.rsqrt(dout)
  dinv = lax.rsqrt(din)
  dout_inv = 1.0 / dout
  col = lambda a: a.reshape(N, 1)

  v16 = jnp.zeros((NP, 16), _f32).at[:N, :].set(v[:, None])
  win_p, wout_p = _vpass_kernel(v16, src_v, dst_v)
  wsum = v * ((win_p[0] + win_p[1])[:N, 0] + (wout_p[0] + wout_p[1])[:N, 0])
  degsum = (dout - 1.0) + (din - 1.0)

  def e_rw(n, nf, cross):
    return 0.5 * (_vdot(degsum, n) - 2.0 * cross) / nf

  def e_sym(n, nf, cross):
    return 0.5 * (_vdot(n * dout_inv, wsum) - 2.0 * cross) / nf

  x0, n0c, xh0 = _make_enc_kernel()(x, W_enc, b_enc.reshape(1, 128), W0)
  zc0, zs0 = _make_scale_kernel()(x0, xh0, col(dinv), col(dout_inv))
  n0 = n0c[:, 0]
  nf0 = jnp.sum(n0)
  rc0, rsym0 = _make_block_kernel(2)(zc0, zs0, src_t, dst_t)

  st1 = _make_stage_kernel(False, True, True, 1)
  (h1, n1c, rs1, xh1, zc1, zs1, G1, cs1, x_sym0) = st1(
      rc0, xh0, b0.reshape(1, 128), col(dinv), col(dout_inv), W1, zs0, rsym0)
  n1 = n1c[:, 0]
  nf1 = jnp.sum(n1)
  e0 = e_sym(n0, nf0, x_sym0[0, 0])
  rc1, rrw1, rsym1 = _make_block_kernel(3)(zc1, h1, zs1, src_t, dst_t)
  rank1 = _rank_diff(h1, G1, rs1, cs1)

  st2 = _make_stage_kernel(False, False, True, 2)
  (h2, n2c, rs2, zs2, G2, cs2, x_rw1, x_sym1) = st2(
      rc1, xh1, b1.reshape(1, 128), col(dinv), col(dout_inv),
      h1, rrw1, zs1, rsym1)
  n2 = n2c[:, 0]
  nf2 = jnp.sum(n2)
  erw1 = e_rw(n1, nf1, x_rw1[0, 0])
  esym1 = e_sym(n1, nf1, x_sym1[0, 0])
  rrw2, rsym2 = _make_block_kernel(2)(h2, zs2, src_t, dst_t)
  rank2 = _rank_diff(h2, G2, rs2, cs2)
  x_rw2, x_sym2 = _cross2(h2, rrw2, zs2, rsym2)
  erw2 = e_rw(n2, nf2, x_rw2)
  esym2 = e_sym(n2, nf2, x_sym2)

  return (h2,
          jnp.stack([e0, erw1, erw2]),
          jnp.stack([esym1, esym2]),
          jnp.stack([rank1, rank2]))


# speedup vs baseline: 34.5193x; 1.0309x over previous
import functools

import jax
import jax.numpy as jnp
from jax import lax
from jax.experimental import pallas as pl
from jax.experimental.pallas import tpu as pltpu
from jax.experimental.pallas import tpu_sc as plsc

N = 10000
E = 320000
NC, NS = 2, 16
NW = NC * NS
EB = 100
K = E // (NW * EB)
EBD = 1000
KD = E // (NW * EBD)
EBV = 500
KV = E // (NW * EBV)
NP = 10112
RPW = NP // NS
T64 = RPW // 64
REM = RPW - T64 * 64
ZR = 24
TZ = RPW // ZR
RZ = RPW - TZ * ZR

_mesh = plsc.VectorSubcoreMesh(
    core_axis_name="c", subcore_axis_name="s", num_cores=NC, num_subcores=NS)

_f32 = jnp.float32
_sc_params = pltpu.CompilerParams(use_tc_tiling_on_sc=False)
_HI = jax.lax.Precision.HIGHEST


def _vdot(a, b):
  return jnp.sum(a * b)


def _fill(ref, rows, cols, value):
  def body(i, _):
    r = i // (cols // 16)
    c0 = (i % (cols // 16)) * 16
    ref[r, pl.ds(c0, 16)] = jnp.full((16,), value, _f32)
    return _
  lax.fori_loop(0, rows * (cols // 16), body, 0)


def _zero_acc(acc, s, zero_v):
  base = s * RPW
  for t in range(TZ):
    pltpu.sync_copy(zero_v, acc.at[pl.ds(base + t * ZR, ZR)])
  if RZ:
    pltpu.sync_copy(zero_v.at[pl.ds(0, RZ)],
                    acc.at[pl.ds(base + TZ * ZR, RZ)])


def _writeback(acc, out, c, s):
  base = s * RPW
  pltpu.sync_copy(acc.at[pl.ds(base, RPW)], out.at[c, pl.ds(base, RPW)])


@functools.partial(
    pl.kernel,
    out_type=(jax.ShapeDtypeStruct((NC, NP, 16), _f32),
              jax.ShapeDtypeStruct((NC, NP, 16), _f32)),
    mesh=_mesh,
    compiler_params=_sc_params,
    scratch_types=[
        pltpu.VMEM((KD, EBD), jnp.int32),
        pltpu.VMEM((KD, EBD), jnp.int32),
        pltpu.VMEM((EBD, 16), _f32),
        pltpu.VMEM((64, 16), _f32),
        pltpu.VMEM_SHARED((NP, 16), _f32),
        pltpu.VMEM_SHARED((NP, 16), _f32),
    ],
)
def _deg_kernel(src_hbm, dst_hbm, dout_hbm, din_hbm,
                sidx, didx, ones_v, zero16, acc_o, acc_i):
  c = lax.axis_index("c")
  s = lax.axis_index("s")
  w = s * NC + c
  pltpu.sync_copy(src_hbm.at[w], sidx)
  pltpu.sync_copy(dst_hbm.at[w], didx)
  _fill(ones_v, EBD, 16, 1.0)
  _fill(zero16, 64, 16, 0.0)
  base = s * RPW
  for t in range(T64):
    pltpu.sync_copy(zero16, acc_o.at[pl.ds(base + t * 64, 64)])
    pltpu.sync_copy(zero16, acc_i.at[pl.ds(base + t * 64, 64)])
  pltpu.sync_copy(zero16.at[pl.ds(0, REM)], acc_o.at[pl.ds(base + T64 * 64, REM)])
  pltpu.sync_copy(zero16.at[pl.ds(0, REM)], acc_i.at[pl.ds(base + T64 * 64, REM)])
  plsc.subcore_barrier()

  def body(k, _):
    pltpu.sync_copy(ones_v, acc_o.at[sidx.at[k]], add=True)
    pltpu.sync_copy(ones_v, acc_i.at[didx.at[k]], add=True)
    return _
  lax.fori_loop(0, KD, body, 0)
  plsc.subcore_barrier()
  pltpu.sync_copy(acc_o.at[pl.ds(base, RPW)], dout_hbm.at[c, pl.ds(base, RPW)])
  pltpu.sync_copy(acc_i.at[pl.ds(base, RPW)], din_hbm.at[c, pl.ds(base, RPW)])


@functools.partial(
    pl.kernel,
    out_type=(jax.ShapeDtypeStruct((NC, NP, 16), _f32),
              jax.ShapeDtypeStruct((NC, NP, 16), _f32)),
    mesh=_mesh,
    compiler_params=_sc_params,
    scratch_types=[
        pltpu.VMEM((KV, EBV), jnp.int32),
        pltpu.VMEM((KV, EBV), jnp.int32),
        pltpu.VMEM((EBV, 16), _f32),
        pltpu.VMEM((EBV, 16), _f32),
        pltpu.VMEM((64, 16), _f32),
        pltpu.VMEM_SHARED((NP, 16), _f32),
        pltpu.VMEM_SHARED((NP, 16), _f32),
        pltpu.SemaphoreType.DMA,
        pltpu.SemaphoreType.DMA,
    ],
)
def _vpass_kernel(v16_hbm, src_hbm, dst_hbm, win_hbm, wout_hbm,
                  sidx, didx, rows_f, rows_r, zero16, acc_f, acc_r, sem_f, sem_r):
  c = lax.axis_index("c")
  s = lax.axis_index("s")
  w = s * NC + c
  pltpu.sync_copy(src_hbm.at[w], sidx)
  pltpu.sync_copy(dst_hbm.at[w], didx)
  _fill(zero16, 64, 16, 0.0)
  base = s * RPW
  for t in range(T64):
    pltpu.sync_copy(zero16, acc_f.at[pl.ds(base + t * 64, 64)])
    pltpu.sync_copy(zero16, acc_r.at[pl.ds(base + t * 64, 64)])
  pltpu.sync_copy(zero16.at[pl.ds(0, REM)], acc_f.at[pl.ds(base + T64 * 64, REM)])
  pltpu.sync_copy(zero16.at[pl.ds(0, REM)], acc_r.at[pl.ds(base + T64 * 64, REM)])
  plsc.subcore_barrier()

  pltpu.async_copy(v16_hbm.at[sidx.at[0]], rows_f, sem_f)
  pltpu.async_copy(v16_hbm.at[didx.at[0]], rows_r, sem_r)

  def body(k, _):
    pltpu.make_async_copy(v16_hbm.at[sidx.at[k]], rows_f, sem_f).wait()
    pltpu.sync_copy(rows_f, acc_f.at[didx.at[k]], add=True)

    @pl.when(k + 1 < KV)
    def _f():
      pltpu.async_copy(v16_hbm.at[sidx.at[k + 1]], rows_f, sem_f)
    pltpu.make_async_copy(v16_hbm.at[didx.at[k]], rows_r, sem_r).wait()
    pltpu.sync_copy(rows_r, acc_r.at[sidx.at[k]], add=True)

    @pl.when(k + 1 < KV)
    def _r():
      pltpu.async_copy(v16_hbm.at[didx.at[k + 1]], rows_r, sem_r)
    return _
  lax.fori_loop(0, KV, body, 0)
  plsc.subcore_barrier()
  pltpu.sync_copy(acc_f.at[pl.ds(base, RPW)], win_hbm.at[c, pl.ds(base, RPW)])
  pltpu.sync_copy(acc_r.at[pl.ds(base, RPW)], wout_hbm.at[c, pl.ds(base, RPW)])


@functools.lru_cache(maxsize=None)
def _make_block_kernel(nblocks):
  out_type = tuple(jax.ShapeDtypeStruct((NC, NP, 128), _f32)
                   for _ in range(nblocks))

  @functools.partial(
      pl.kernel,
      out_type=out_type,
      mesh=_mesh,
      compiler_params=_sc_params,
      scratch_types=[
          pltpu.VMEM((K, EB), jnp.int32),
          pltpu.VMEM((K, EB), jnp.int32),
          pltpu.VMEM((EB, 128), _f32),
          pltpu.VMEM((EB, 128), _f32),
          pltpu.VMEM((ZR, 128), _f32),
          pltpu.VMEM_SHARED((NP, 128), _f32),
          pltpu.SemaphoreType.DMA,
          pltpu.SemaphoreType.DMA,
          pltpu.SemaphoreType.DMA,
          pltpu.SemaphoreType.DMA,
      ],
  )
  def _block_kernel(*refs):
    z_hbm = refs[:nblocks]
    src_hbm, dst_hbm = refs[nblocks], refs[nblocks + 1]
    outs = refs[nblocks + 2:2 * nblocks + 2]
    (sidx, didx, rows0, rows1, zero_v, acc,
     sem0, sem1, semw0, semw1) = refs[2 * nblocks + 2:]
    c = lax.axis_index("c")
    s = lax.axis_index("s")
    w = s * NC + c
    pltpu.sync_copy(src_hbm.at[w], sidx)
    pltpu.sync_copy(dst_hbm.at[w], didx)
    _fill(zero_v, ZR, 128, 0.0)
    for b in range(nblocks):
      _zero_acc(acc, s, zero_v)
      plsc.subcore_barrier()
      zb = z_hbm[b]
      pltpu.async_copy(zb.at[sidx.at[0]], rows0, sem0)
      pltpu.async_copy(zb.at[sidx.at[1]], rows1, sem1)

      def pair(t, _, zb=zb):
        k0 = 2 * t
        pltpu.make_async_copy(zb.at[sidx.at[k0]], rows0, sem0).wait()
        pltpu.async_copy(rows0, acc.at[didx.at[k0]], semw0, add=True)

        @pl.when(k0 + 2 < K)
        def _fire():
          pltpu.make_async_copy(rows0, acc.at[didx.at[k0]], semw0).wait()
          pltpu.async_copy(zb.at[sidx.at[k0 + 2]], rows0, sem0)
        pltpu.make_async_copy(zb.at[sidx.at[k0 + 1]], rows1, sem1).wait()
        pltpu.async_copy(rows1, acc.at[didx.at[k0 + 1]], semw1, add=True)

        @pl.when(k0 + 3 < K)
        def _fire1():
          pltpu.make_async_copy(rows1, acc.at[didx.at[k0 + 1]], semw1).wait()
          pltpu.async_copy(zb.at[sidx.at[k0 + 3]], rows1, sem1)
        return _
      lax.fori_loop(0, K // 2, pair, 0)
      if K % 2:
        pltpu.make_async_copy(zb.at[sidx.at[K - 1]], rows0, sem0).wait()
        pltpu.async_copy(rows0, acc.at[didx.at[K - 1]], semw0, add=True)
        pltpu.make_async_copy(rows0, acc.at[didx.at[K - 1]], semw0).wait()
        pltpu.make_async_copy(rows1, acc.at[didx.at[K - 2]], semw1).wait()
      else:
        pltpu.make_async_copy(rows0, acc.at[didx.at[K - 2]], semw0).wait()
        pltpu.make_async_copy(rows1, acc.at[didx.at[K - 1]], semw1).wait()
      plsc.subcore_barrier()
      _writeback(acc, outs[b], c, s)
      plsc.subcore_barrier()

  return _block_kernel


def _msgpass(blocks, src_t, dst_t):
  kern = _make_block_kernel(len(blocks))
  outs = kern(*blocks, src_t, dst_t)
  return [(o[0] + o[1])[:N] for o in outs]


NBLK = 25
BR = N // NBLK


def _dot(a, b):
  return jnp.dot(a, b, precision=_HI)


@functools.lru_cache(maxsize=None)
def _make_stage_kernel(encoder, with_next, with_rank, ncross):
  def body(*refs):
    idx = 0
    if encoder:
      x_ref, W_ref, b_ref = refs[0:3]
    else:
      rc_ref, xhp_ref, b_ref = refs[0:3]
    idx = 3
    dinv_ref, dout_inv_ref = refs[idx:idx + 2]; idx += 2
    if with_next:
      Wn_ref = refs[idx]; idx += 1
    cross_in = []
    for _ in range(ncross):
      cross_in.append((refs[idx], refs[idx + 1])); idx += 2
    h_ref, n_ref, rs_ref = refs[idx:idx + 3]; idx += 3
    if with_next:
      xh_ref, zc_ref = refs[idx:idx + 2]; idx += 2
    zs_ref = refs[idx]; idx += 1
    if with_rank:
      G_ref, cs_ref = refs[idx:idx + 2]; idx += 2
    cross_out = refs[idx:idx + ncross]

    i = pl.program_id(0)
    dinv = dinv_ref[...]
    if encoder:
      h = _dot(x_ref[...], W_ref[...]) + b_ref[...]
    else:
      pre = dinv * (rc_ref[0] + rc_ref[1]) + dinv * dinv * xhp_ref[...] + b_ref[...]
      h = jnp.maximum(pre, 0.0)
    h_ref[...] = h
    n_ref[...] = jnp.sum(h * h, axis=1, keepdims=True)
    rs_ref[...] = jnp.sum(jnp.abs(h), axis=1, keepdims=True)
    if with_next:
      xh = _dot(h, Wn_ref[...])
      xh_ref[...] = xh
      zc_ref[...] = dinv * xh
    zs_ref[...] = h * dout_inv_ref[...]

    @pl.when(i == 0)
    def _init():
      if with_rank:
        G_ref[...] = jnp.zeros_like(G_ref)
        cs_ref[...] = jnp.zeros_like(cs_ref)
      for co in cross_out:
        co[0, 0] = 0.0

    if with_rank:
      G_ref[...] += _dot(h.T, h)
      cs_ref[...] += jnp.sum(jnp.abs(h), axis=0, keepdims=True)
    for (zp_ref, r_ref), co in zip(cross_in, cross_out):
      co[0, 0] += jnp.sum(zp_ref[...] * (r_ref[0] + r_ref[1]))

  row = lambda i: (i, 0)
  full = lambda i: (0, 0)
  full3 = lambda i: (0, i, 0)
  in_specs = []
  if encoder:
    in_specs += [pl.BlockSpec((BR, 128), row), pl.BlockSpec((128, 128), full),
                 pl.BlockSpec((1, 128), full)]
  else:
    in_specs += [pl.BlockSpec((2, BR, 128), full3), pl.BlockSpec((BR, 128), row),
                 pl.BlockSpec((1, 128), full)]
  in_specs += [pl.BlockSpec((BR, 1), row)] * 2
  if with_next:
    in_specs += [pl.BlockSpec((128, 128), full)]
  for _ in range(ncross):
    in_specs += [pl.BlockSpec((BR, 128), row), pl.BlockSpec((2, BR, 128), full3)]

  out_specs = [pl.BlockSpec((BR, 128), row), pl.BlockSpec((BR, 1), row),
               pl.BlockSpec((BR, 1), row)]
  out_shape = [jax.ShapeDtypeStruct((N, 128), _f32),
               jax.ShapeDtypeStruct((N, 1), _f32),
               jax.ShapeDtypeStruct((N, 1), _f32)]
  if with_next:
    out_specs += [pl.BlockSpec((BR, 128), row)] * 2
    out_shape += [jax.ShapeDtypeStruct((N, 128), _f32)] * 2
  out_specs += [pl.BlockSpec((BR, 128), row)]
  out_shape += [jax.ShapeDtypeStruct((N, 128), _f32)]
  if with_rank:
    out_specs += [pl.BlockSpec((128, 128), full), pl.BlockSpec((1, 128), full)]
    out_shape += [jax.ShapeDtypeStruct((128, 128), _f32),
                  jax.ShapeDtypeStruct((1, 128), _f32)]
  out_specs += [pl.BlockSpec((1, 1), full, memory_space=pltpu.SMEM)] * ncross
  out_shape += [jax.ShapeDtypeStruct((1, 1), _f32)] * ncross

  return pl.pallas_call(body, grid=(NBLK,), in_specs=in_specs,
                        out_specs=out_specs, out_shape=out_shape)


def _cross2(zA, rA, zB, rB):
  def body(zA_ref, rA_ref, zB_ref, rB_ref, a_ref, b_ref):
    i = pl.program_id(0)

    @pl.when(i == 0)
    def _init():
      a_ref[0, 0] = 0.0
      b_ref[0, 0] = 0.0
    a_ref[0, 0] += jnp.sum(zA_ref[...] * (rA_ref[0] + rA_ref[1]))
    b_ref[0, 0] += jnp.sum(zB_ref[...] * (rB_ref[0] + rB_ref[1]))

  row = lambda i: (i, 0)
  full3 = lambda i: (0, i, 0)
  smem = pl.BlockSpec((1, 1), lambda i: (0, 0), memory_space=pltpu.SMEM)
  out = pl.pallas_call(
      body, grid=(NBLK,),
      in_specs=[pl.BlockSpec((BR, 128), row), pl.BlockSpec((2, BR, 128), full3),
                pl.BlockSpec((BR, 128), row), pl.BlockSpec((2, BR, 128), full3)],
      out_specs=[smem, smem],
      out_shape=[jax.ShapeDtypeStruct((1, 1), _f32)] * 2)(zA, rA, zB, rB)
  return out[0][0, 0], out[1][0, 0]


NS_ITERS = 25


def _rank_kernel(G, g, r, cnorm2, rnorm2, sgn):
  def body(G_ref, g_ref, r_ref, sc_ref, I_ref, nu_ref, rank_ref):
    I = I_ref[...]

    def trsqrt(S):
      c = jnp.sum(S * I)
      Y, Z = S / c, I
      for _ in range(NS_ITERS):
        T = 0.5 * (3.0 * I - _dot(Z, Y))
        Y, Z = _dot(Y, T), _dot(T, Z)
      return jnp.sum(Y * I) * jnp.sqrt(c)

    G = G_ref[...]
    nu1 = trsqrt(G)
    nu_ref[0, 0] = nu1
    g_ = g_ref[...]
    r_ = r_ref[...]
    cn2, rn2, sg = sc_ref[0], sc_ref[1], sc_ref[2]
    gr = _dot(g_.T, r_) + _dot(r_.T, g_)
    M = (G / (nu1 * nu1)
         - (sg / (nu1 * jnp.sqrt(cn2 * rn2))) * gr
         + _dot(r_.T, r_) / rn2)
    rank_ref[0, 0] = trsqrt(M)

  nu, rank = pl.pallas_call(
      body,
      in_specs=[pl.BlockSpec(memory_space=pltpu.VMEM),
                pl.BlockSpec(memory_space=pltpu.VMEM),
                pl.BlockSpec(memory_space=pltpu.VMEM),
                pl.BlockSpec(memory_space=pltpu.SMEM),
                pl.BlockSpec(memory_space=pltpu.VMEM)],
      out_specs=[pl.BlockSpec(memory_space=pltpu.SMEM),
                 pl.BlockSpec(memory_space=pltpu.SMEM)],
      out_shape=[jax.ShapeDtypeStruct((1, 1), _f32)] * 2,
  )(G, g.reshape(1, 128), r.reshape(1, 128), jnp.stack([cnorm2, rnorm2, sgn]),
    jnp.eye(128, dtype=_f32))
  return rank[0, 0]


def _rank_diff(h, G, rs, cs):
  i = jnp.argmax(rs[:, 0])
  j = jnp.argmax(cs[0, :])
  r = lax.dynamic_slice(h, (i, 0), (1, 128))[0]
  g = lax.dynamic_slice(G, (0, j), (128, 1))[:, 0]
  cnorm2 = G[j, j]
  rnorm2 = jnp.sum(r * r)
  sgn = jnp.where(h[i, j] < 0, -1.0, 1.0)
  return _rank_kernel(G, g, r, cnorm2, rnorm2, sgn)


@functools.lru_cache(maxsize=None)
def _make_enc_kernel():
  def body(x_ref, W_ref, b_ref, Wn_ref, x0_ref, n_ref, xh_ref):
    x0 = _dot(x_ref[...], W_ref[...]) + b_ref[...]
    x0_ref[...] = x0
    n_ref[...] = jnp.sum(x0 * x0, axis=1, keepdims=True)
    xh_ref[...] = _dot(x0, Wn_ref[...])

  row = lambda i: (i, 0)
  full = lambda i: (0, 0)
  return pl.pallas_call(
      body, grid=(NBLK,),
      in_specs=[pl.BlockSpec((BR, 128), row), pl.BlockSpec((128, 128), full),
                pl.BlockSpec((1, 128), full), pl.BlockSpec((128, 128), full)],
      out_specs=[pl.BlockSpec((BR, 128), row), pl.BlockSpec((BR, 1), row),
                 pl.BlockSpec((BR, 128), row)],
      out_shape=[jax.ShapeDtypeStruct((N, 128), _f32),
                 jax.ShapeDtypeStruct((N, 1), _f32),
                 jax.ShapeDtypeStruct((N, 128), _f32)])


@functools.lru_cache(maxsize=None)
def _make_scale_kernel():
  def body(x0_ref, xh_ref, dinv_ref, di_ref, zc_ref, zs_ref):
    zc_ref[...] = dinv_ref[...] * xh_ref[...]
    zs_ref[...] = x0_ref[...] * di_ref[...]

  row = lambda i: (i, 0)
  return pl.pallas_call(
      body, grid=(NBLK,),
      in_specs=[pl.BlockSpec((BR, 128), row), pl.BlockSpec((BR, 128), row),
                pl.BlockSpec((BR, 1), row), pl.BlockSpec((BR, 1), row)],
      out_specs=[pl.BlockSpec((BR, 128), row), pl.BlockSpec((BR, 128), row)],
      out_shape=[jax.ShapeDtypeStruct((N, 128), _f32)] * 2)


def kernel(x, edge_index, W_enc, b_enc, W0, b0, W1, b1):
  src_t = edge_index[0].reshape(NW, K, EB)
  dst_t = edge_index[1].reshape(NW, K, EB)
  src_d = edge_index[0].reshape(NW, KD, EBD)
  dst_d = edge_index[1].reshape(NW, KD, EBD)
  src_v = edge_index[0].reshape(NW, KV, EBV)
  dst_v = edge_index[1].reshape(NW, KV, EBV)

  dout_p, din_p = _deg_kernel(src_d, dst_d)
  dout = (dout_p[0] + dout_p[1])[:N, 0] + 1.0
  din = (din_p[0] + din_p[1])[:N, 0] + 1.0
  v = lax.rsqrt(dout)
  dinv = lax.rsqrt(din)
  dout_inv = 1.0 / dout
  col = lambda a: a.reshape(N, 1)

  v16 = jnp.zeros((NP, 16), _f32).at[:N, :].set(v[:, None])
  win_p, wout_p = _vpass_kernel(v16, src_v, dst_v)
  wsum = v * ((win_p[0] + win_p[1])[:N, 0] + (wout_p[0] + wout_p[1])[:N, 0])
  degsum = (dout - 1.0) + (din - 1.0)

  def e_rw(n, nf, cross):
    return 0.5 * (_vdot(degsum, n) - 2.0 * cross) / nf

  def e_sym(n, nf, cross):
    return 0.5 * (_vdot(n * dout_inv, wsum) - 2.0 * cross) / nf

  x0, n0c, xh0 = _make_enc_kernel()(x, W_enc, b_enc.reshape(1, 128), W0)
  zc0, zs0 = _make_scale_kernel()(x0, xh0, col(dinv), col(dout_inv))
  n0 = n0c[:, 0]
  nf0 = jnp.sum(n0)
  rc0, rsym0 = _make_block_kernel(2)(zc0, zs0, src_t, dst_t)

  st1 = _make_stage_kernel(False, True, True, 1)
  (h1, n1c, rs1, xh1, zc1, zs1, G1, cs1, x_sym0) = st1(
      rc0, xh0, b0.reshape(1, 128), col(dinv), col(dout_inv), W1, zs0, rsym0)
  n1 = n1c[:, 0]
  nf1 = jnp.sum(n1)
  e0 = e_sym(n0, nf0, x_sym0[0, 0])
  rc1, rrw1, rsym1 = _make_block_kernel(3)(zc1, h1, zs1, src_t, dst_t)
  rank1 = _rank_diff(h1, G1, rs1, cs1)

  st2 = _make_stage_kernel(False, False, True, 2)
  (h2, n2c, rs2, zs2, G2, cs2, x_rw1, x_sym1) = st2(
      rc1, xh1, b1.reshape(1, 128), col(dinv), col(dout_inv),
      h1, rrw1, zs1, rsym1)
  n2 = n2c[:, 0]
  nf2 = jnp.sum(n2)
  erw1 = e_rw(n1, nf1, x_rw1[0, 0])
  esym1 = e_sym(n1, nf1, x_sym1[0, 0])
  rrw2, rsym2 = _make_block_kernel(2)(h2, zs2, src_t, dst_t)
  rank2 = _rank_diff(h2, G2, rs2, cs2)
  x_rw2, x_sym2 = _cross2(h2, rrw2, zs2, rsym2)
  erw2 = e_rw(n2, nf2, x_rw2)
  esym2 = e_sym(n2, nf2, x_sym2)

  return (h2,
          jnp.stack([e0, erw1, erw2]),
          jnp.stack([esym1, esym2]),
          jnp.stack([rank1, rank2]))


# speedup vs baseline: 38.0242x; 1.1015x over previous
import functools

import jax
import jax.numpy as jnp
from jax import lax
from jax.experimental import pallas as pl
from jax.experimental.pallas import tpu as pltpu
from jax.experimental.pallas import tpu_sc as plsc

N = 10000
E = 320000
NC, NS = 2, 16
NW = NC * NS
EB = 50
K = E // (NW * EB)
DEPTH = 4
EBD = 1000
KD = E // (NW * EBD)
EBV = 500
KV = E // (NW * EBV)
NP = 10112
RPW = NP // NS
T64 = RPW // 64
REM = RPW - T64 * 64
ZR = 16
TZ = RPW // ZR
RZ = RPW - TZ * ZR

_mesh = plsc.VectorSubcoreMesh(
    core_axis_name="c", subcore_axis_name="s", num_cores=NC, num_subcores=NS)

_f32 = jnp.float32
_sc_params = pltpu.CompilerParams(use_tc_tiling_on_sc=False)
_HI = jax.lax.Precision.HIGHEST


def _vdot(a, b):
  return jnp.sum(a * b)


def _fill(ref, rows, cols, value):
  def body(i, _):
    r = i // (cols // 16)
    c0 = (i % (cols // 16)) * 16
    ref[r, pl.ds(c0, 16)] = jnp.full((16,), value, _f32)
    return _
  lax.fori_loop(0, rows * (cols // 16), body, 0)


def _zero_acc(acc, s, zero_v):
  base = s * RPW
  for t in range(TZ):
    pltpu.sync_copy(zero_v, acc.at[pl.ds(base + t * ZR, ZR)])
  if RZ:
    pltpu.sync_copy(zero_v.at[pl.ds(0, RZ)],
                    acc.at[pl.ds(base + TZ * ZR, RZ)])


def _writeback(acc, out, c, s):
  base = s * RPW
  pltpu.sync_copy(acc.at[pl.ds(base, RPW)], out.at[c, pl.ds(base, RPW)])


@functools.partial(
    pl.kernel,
    out_type=(jax.ShapeDtypeStruct((NC, NP, 16), _f32),
              jax.ShapeDtypeStruct((NC, NP, 16), _f32)),
    mesh=_mesh,
    compiler_params=_sc_params,
    scratch_types=[
        pltpu.VMEM((KD, EBD), jnp.int32),
        pltpu.VMEM((KD, EBD), jnp.int32),
        pltpu.VMEM((EBD, 16), _f32),
        pltpu.VMEM((64, 16), _f32),
        pltpu.VMEM_SHARED((NP, 16), _f32),
        pltpu.VMEM_SHARED((NP, 16), _f32),
    ],
)
def _deg_kernel(src_hbm, dst_hbm, dout_hbm, din_hbm,
                sidx, didx, ones_v, zero16, acc_o, acc_i):
  c = lax.axis_index("c")
  s = lax.axis_index("s")
  w = s * NC + c
  pltpu.sync_copy(src_hbm.at[w], sidx)
  pltpu.sync_copy(dst_hbm.at[w], didx)
  _fill(ones_v, EBD, 16, 1.0)
  _fill(zero16, 64, 16, 0.0)
  base = s * RPW
  for t in range(T64):
    pltpu.sync_copy(zero16, acc_o.at[pl.ds(base + t * 64, 64)])
    pltpu.sync_copy(zero16, acc_i.at[pl.ds(base + t * 64, 64)])
  pltpu.sync_copy(zero16.at[pl.ds(0, REM)], acc_o.at[pl.ds(base + T64 * 64, REM)])
  pltpu.sync_copy(zero16.at[pl.ds(0, REM)], acc_i.at[pl.ds(base + T64 * 64, REM)])
  plsc.subcore_barrier()

  def body(k, _):
    pltpu.sync_copy(ones_v, acc_o.at[sidx.at[k]], add=True)
    pltpu.sync_copy(ones_v, acc_i.at[didx.at[k]], add=True)
    return _
  lax.fori_loop(0, KD, body, 0)
  plsc.subcore_barrier()
  pltpu.sync_copy(acc_o.at[pl.ds(base, RPW)], dout_hbm.at[c, pl.ds(base, RPW)])
  pltpu.sync_copy(acc_i.at[pl.ds(base, RPW)], din_hbm.at[c, pl.ds(base, RPW)])


@functools.partial(
    pl.kernel,
    out_type=(jax.ShapeDtypeStruct((NC, NP, 16), _f32),
              jax.ShapeDtypeStruct((NC, NP, 16), _f32)),
    mesh=_mesh,
    compiler_params=_sc_params,
    scratch_types=[
        pltpu.VMEM((KV, EBV), jnp.int32),
        pltpu.VMEM((KV, EBV), jnp.int32),
        pltpu.VMEM((EBV, 16), _f32),
        pltpu.VMEM((EBV, 16), _f32),
        pltpu.VMEM((64, 16), _f32),
        pltpu.VMEM_SHARED((NP, 16), _f32),
        pltpu.VMEM_SHARED((NP, 16), _f32),
        pltpu.SemaphoreType.DMA,
        pltpu.SemaphoreType.DMA,
    ],
)
def _vpass_kernel(v16_hbm, src_hbm, dst_hbm, win_hbm, wout_hbm,
                  sidx, didx, rows_f, rows_r, zero16, acc_f, acc_r, sem_f, sem_r):
  c = lax.axis_index("c")
  s = lax.axis_index("s")
  w = s * NC + c
  pltpu.sync_copy(src_hbm.at[w], sidx)
  pltpu.sync_copy(dst_hbm.at[w], didx)
  _fill(zero16, 64, 16, 0.0)
  base = s * RPW
  for t in range(T64):
    pltpu.sync_copy(zero16, acc_f.at[pl.ds(base + t * 64, 64)])
    pltpu.sync_copy(zero16, acc_r.at[pl.ds(base + t * 64, 64)])
  pltpu.sync_copy(zero16.at[pl.ds(0, REM)], acc_f.at[pl.ds(base + T64 * 64, REM)])
  pltpu.sync_copy(zero16.at[pl.ds(0, REM)], acc_r.at[pl.ds(base + T64 * 64, REM)])
  plsc.subcore_barrier()

  pltpu.async_copy(v16_hbm.at[sidx.at[0]], rows_f, sem_f)
  pltpu.async_copy(v16_hbm.at[didx.at[0]], rows_r, sem_r)

  def body(k, _):
    pltpu.make_async_copy(v16_hbm.at[sidx.at[k]], rows_f, sem_f).wait()
    pltpu.sync_copy(rows_f, acc_f.at[didx.at[k]], add=True)

    @pl.when(k + 1 < KV)
    def _f():
      pltpu.async_copy(v16_hbm.at[sidx.at[k + 1]], rows_f, sem_f)
    pltpu.make_async_copy(v16_hbm.at[didx.at[k]], rows_r, sem_r).wait()
    pltpu.sync_copy(rows_r, acc_r.at[sidx.at[k]], add=True)

    @pl.when(k + 1 < KV)
    def _r():
      pltpu.async_copy(v16_hbm.at[didx.at[k + 1]], rows_r, sem_r)
    return _
  lax.fori_loop(0, KV, body, 0)
  plsc.subcore_barrier()
  pltpu.sync_copy(acc_f.at[pl.ds(base, RPW)], win_hbm.at[c, pl.ds(base, RPW)])
  pltpu.sync_copy(acc_r.at[pl.ds(base, RPW)], wout_hbm.at[c, pl.ds(base, RPW)])


@functools.lru_cache(maxsize=None)
def _make_block_kernel(nblocks):
  out_type = tuple(jax.ShapeDtypeStruct((NC, NP, 128), _f32)
                   for _ in range(nblocks))

  @functools.partial(
      pl.kernel,
      out_type=out_type,
      mesh=_mesh,
      compiler_params=_sc_params,
      scratch_types=[
          pltpu.VMEM((K, EB), jnp.int32),
          pltpu.VMEM((K, EB), jnp.int32),
      ] + [pltpu.VMEM((EB, 128), _f32)] * DEPTH + [
          pltpu.VMEM((ZR, 128), _f32),
          pltpu.VMEM_SHARED((NP, 128), _f32),
      ] + [pltpu.SemaphoreType.DMA] * (2 * DEPTH),
  )
  def _block_kernel(*refs):
    z_hbm = refs[:nblocks]
    src_hbm, dst_hbm = refs[nblocks], refs[nblocks + 1]
    outs = refs[nblocks + 2:2 * nblocks + 2]
    rest = refs[2 * nblocks + 2:]
    sidx, didx = rest[0], rest[1]
    rows = rest[2:2 + DEPTH]
    zero_v, acc = rest[2 + DEPTH], rest[3 + DEPTH]
    sems = rest[4 + DEPTH:4 + 2 * DEPTH]
    semw = rest[4 + 2 * DEPTH:4 + 3 * DEPTH]
    c = lax.axis_index("c")
    s = lax.axis_index("s")
    w = s * NC + c
    pltpu.sync_copy(src_hbm.at[w], sidx)
    pltpu.sync_copy(dst_hbm.at[w], didx)
    _fill(zero_v, ZR, 128, 0.0)
    for b in range(nblocks):
      _zero_acc(acc, s, zero_v)
      plsc.subcore_barrier()
      zb = z_hbm[b]
      for d in range(DEPTH):
        pltpu.async_copy(zb.at[sidx.at[d]], rows[d], sems[d])

      def grp(t, _, zb=zb):
        k0 = DEPTH * t
        for d in range(DEPTH):
          k = k0 + d
          pltpu.make_async_copy(zb.at[sidx.at[k]], rows[d], sems[d]).wait()
          pltpu.async_copy(rows[d], acc.at[didx.at[k]], semw[d], add=True)

          @pl.when(k + DEPTH < K)
          def _fire(d=d, k=k):
            pltpu.make_async_copy(rows[d], acc.at[didx.at[k]], semw[d]).wait()
            pltpu.async_copy(zb.at[sidx.at[k + DEPTH]], rows[d], sems[d])
        return _
      lax.fori_loop(0, K // DEPTH, grp, 0)
      for d in range(DEPTH):
        pltpu.make_async_copy(
            rows[d], acc.at[didx.at[K - DEPTH + d]], semw[d]).wait()
      plsc.subcore_barrier()
      _writeback(acc, outs[b], c, s)
      plsc.subcore_barrier()

  return _block_kernel


def _msgpass(blocks, src_t, dst_t):
  kern = _make_block_kernel(len(blocks))
  outs = kern(*blocks, src_t, dst_t)
  return [(o[0] + o[1])[:N] for o in outs]


NBLK = 25
BR = N // NBLK


def _dot(a, b):
  return jnp.dot(a, b, precision=_HI)


@functools.lru_cache(maxsize=None)
def _make_stage_kernel(encoder, with_next, with_rank, ncross):
  def body(*refs):
    idx = 0
    if encoder:
      x_ref, W_ref, b_ref = refs[0:3]
    else:
      rc_ref, xhp_ref, b_ref = refs[0:3]
    idx = 3
    dinv_ref, dout_inv_ref = refs[idx:idx + 2]; idx += 2
    if with_next:
      Wn_ref = refs[idx]; idx += 1
    cross_in = []
    for _ in range(ncross):
      cross_in.append((refs[idx], refs[idx + 1])); idx += 2
    h_ref, n_ref, rs_ref = refs[idx:idx + 3]; idx += 3
    if with_next:
      xh_ref, zc_ref = refs[idx:idx + 2]; idx += 2
    zs_ref = refs[idx]; idx += 1
    if with_rank:
      G_ref, cs_ref = refs[idx:idx + 2]; idx += 2
    cross_out = refs[idx:idx + ncross]

    i = pl.program_id(0)
    dinv = dinv_ref[...]
    if encoder:
      h = _dot(x_ref[...], W_ref[...]) + b_ref[...]
    else:
      pre = dinv * (rc_ref[0] + rc_ref[1]) + dinv * dinv * xhp_ref[...] + b_ref[...]
      h = jnp.maximum(pre, 0.0)
    h_ref[...] = h
    n_ref[...] = jnp.sum(h * h, axis=1, keepdims=True)
    rs_ref[...] = jnp.sum(jnp.abs(h), axis=1, keepdims=True)
    if with_next:
      xh = _dot(h, Wn_ref[...])
      xh_ref[...] = xh
      zc_ref[...] = dinv * xh
    zs_ref[...] = h * dout_inv_ref[...]

    @pl.when(i == 0)
    def _init():
      if with_rank:
        G_ref[...] = jnp.zeros_like(G_ref)
        cs_ref[...] = jnp.zeros_like(cs_ref)
      for co in cross_out:
        co[0, 0] = 0.0

    if with_rank:
      G_ref[...] += _dot(h.T, h)
      cs_ref[...] += jnp.sum(jnp.abs(h), axis=0, keepdims=True)
    for (zp_ref, r_ref), co in zip(cross_in, cross_out):
      co[0, 0] += jnp.sum(zp_ref[...] * (r_ref[0] + r_ref[1]))

  row = lambda i: (i, 0)
  full = lambda i: (0, 0)
  full3 = lambda i: (0, i, 0)
  in_specs = []
  if encoder:
    in_specs += [pl.BlockSpec((BR, 128), row), pl.BlockSpec((128, 128), full),
                 pl.BlockSpec((1, 128), full)]
  else:
    in_specs += [pl.BlockSpec((2, BR, 128), full3), pl.BlockSpec((BR, 128), row),
                 pl.BlockSpec((1, 128), full)]
  in_specs += [pl.BlockSpec((BR, 1), row)] * 2
  if with_next:
    in_specs += [pl.BlockSpec((128, 128), full)]
  for _ in range(ncross):
    in_specs += [pl.BlockSpec((BR, 128), row), pl.BlockSpec((2, BR, 128), full3)]

  out_specs = [pl.BlockSpec((BR, 128), row), pl.BlockSpec((BR, 1), row),
               pl.BlockSpec((BR, 1), row)]
  out_shape = [jax.ShapeDtypeStruct((N, 128), _f32),
               jax.ShapeDtypeStruct((N, 1), _f32),
               jax.ShapeDtypeStruct((N, 1), _f32)]
  if with_next:
    out_specs += [pl.BlockSpec((BR, 128), row)] * 2
    out_shape += [jax.ShapeDtypeStruct((N, 128), _f32)] * 2
  out_specs += [pl.BlockSpec((BR, 128), row)]
  out_shape += [jax.ShapeDtypeStruct((N, 128), _f32)]
  if with_rank:
    out_specs += [pl.BlockSpec((128, 128), full), pl.BlockSpec((1, 128), full)]
    out_shape += [jax.ShapeDtypeStruct((128, 128), _f32),
                  jax.ShapeDtypeStruct((1, 128), _f32)]
  out_specs += [pl.BlockSpec((1, 1), full, memory_space=pltpu.SMEM)] * ncross
  out_shape += [jax.ShapeDtypeStruct((1, 1), _f32)] * ncross

  return pl.pallas_call(body, grid=(NBLK,), in_specs=in_specs,
                        out_specs=out_specs, out_shape=out_shape)


def _cross2(zA, rA, zB, rB):
  def body(zA_ref, rA_ref, zB_ref, rB_ref, a_ref, b_ref):
    i = pl.program_id(0)

    @pl.when(i == 0)
    def _init():
      a_ref[0, 0] = 0.0
      b_ref[0, 0] = 0.0
    a_ref[0, 0] += jnp.sum(zA_ref[...] * (rA_ref[0] + rA_ref[1]))
    b_ref[0, 0] += jnp.sum(zB_ref[...] * (rB_ref[0] + rB_ref[1]))

  row = lambda i: (i, 0)
  full3 = lambda i: (0, i, 0)
  smem = pl.BlockSpec((1, 1), lambda i: (0, 0), memory_space=pltpu.SMEM)
  out = pl.pallas_call(
      body, grid=(NBLK,),
      in_specs=[pl.BlockSpec((BR, 128), row), pl.BlockSpec((2, BR, 128), full3),
                pl.BlockSpec((BR, 128), row), pl.BlockSpec((2, BR, 128), full3)],
      out_specs=[smem, smem],
      out_shape=[jax.ShapeDtypeStruct((1, 1), _f32)] * 2)(zA, rA, zB, rB)
  return out[0][0, 0], out[1][0, 0]


NS_ITERS = 25


def _rank_kernel(G, g, r, cnorm2, rnorm2, sgn):
  def body(G_ref, g_ref, r_ref, sc_ref, I_ref, nu_ref, rank_ref):
    I = I_ref[...]

    def trsqrt(S):
      c = jnp.sum(S * I)
      Y, Z = S / c, I
      for _ in range(NS_ITERS):
        T = 0.5 * (3.0 * I - _dot(Z, Y))
        Y, Z = _dot(Y, T), _dot(T, Z)
      return jnp.sum(Y * I) * jnp.sqrt(c)

    G = G_ref[...]
    nu1 = trsqrt(G)
    nu_ref[0, 0] = nu1
    g_ = g_ref[...]
    r_ = r_ref[...]
    cn2, rn2, sg = sc_ref[0], sc_ref[1], sc_ref[2]
    gr = _dot(g_.T, r_) + _dot(r_.T, g_)
    M = (G / (nu1 * nu1)
         - (sg / (nu1 * jnp.sqrt(cn2 * rn2))) * gr
         + _dot(r_.T, r_) / rn2)
    rank_ref[0, 0] = trsqrt(M)

  nu, rank = pl.pallas_call(
      body,
      in_specs=[pl.BlockSpec(memory_space=pltpu.VMEM),
                pl.BlockSpec(memory_space=pltpu.VMEM),
                pl.BlockSpec(memory_space=pltpu.VMEM),
                pl.BlockSpec(memory_space=pltpu.SMEM),
                pl.BlockSpec(memory_space=pltpu.VMEM)],
      out_specs=[pl.BlockSpec(memory_space=pltpu.SMEM),
                 pl.BlockSpec(memory_space=pltpu.SMEM)],
      out_shape=[jax.ShapeDtypeStruct((1, 1), _f32)] * 2,
  )(G, g.reshape(1, 128), r.reshape(1, 128), jnp.stack([cnorm2, rnorm2, sgn]),
    jnp.eye(128, dtype=_f32))
  return rank[0, 0]


def _rank_diff(h, G, rs, cs):
  i = jnp.argmax(rs[:, 0])
  j = jnp.argmax(cs[0, :])
  r = lax.dynamic_slice(h, (i, 0), (1, 128))[0]
  g = lax.dynamic_slice(G, (0, j), (128, 1))[:, 0]
  cnorm2 = G[j, j]
  rnorm2 = jnp.sum(r * r)
  sgn = jnp.where(h[i, j] < 0, -1.0, 1.0)
  return _rank_kernel(G, g, r, cnorm2, rnorm2, sgn)


@functools.lru_cache(maxsize=None)
def _make_enc_kernel():
  def body(x_ref, W_ref, b_ref, Wn_ref, x0_ref, n_ref, xh_ref):
    x0 = _dot(x_ref[...], W_ref[...]) + b_ref[...]
    x0_ref[...] = x0
    n_ref[...] = jnp.sum(x0 * x0, axis=1, keepdims=True)
    xh_ref[...] = _dot(x0, Wn_ref[...])

  row = lambda i: (i, 0)
  full = lambda i: (0, 0)
  return pl.pallas_call(
      body, grid=(NBLK,),
      in_specs=[pl.BlockSpec((BR, 128), row), pl.BlockSpec((128, 128), full),
                pl.BlockSpec((1, 128), full), pl.BlockSpec((128, 128), full)],
      out_specs=[pl.BlockSpec((BR, 128), row), pl.BlockSpec((BR, 1), row),
                 pl.BlockSpec((BR, 128), row)],
      out_shape=[jax.ShapeDtypeStruct((N, 128), _f32),
                 jax.ShapeDtypeStruct((N, 1), _f32),
                 jax.ShapeDtypeStruct((N, 128), _f32)])


@functools.lru_cache(maxsize=None)
def _make_scale_kernel():
  def body(x0_ref, xh_ref, dinv_ref, di_ref, zc_ref, zs_ref):
    zc_ref[...] = dinv_ref[...] * xh_ref[...]
    zs_ref[...] = x0_ref[...] * di_ref[...]

  row = lambda i: (i, 0)
  return pl.pallas_call(
      body, grid=(NBLK,),
      in_specs=[pl.BlockSpec((BR, 128), row), pl.BlockSpec((BR, 128), row),
                pl.BlockSpec((BR, 1), row), pl.BlockSpec((BR, 1), row)],
      out_specs=[pl.BlockSpec((BR, 128), row), pl.BlockSpec((BR, 128), row)],
      out_shape=[jax.ShapeDtypeStruct((N, 128), _f32)] * 2)


def kernel(x, edge_index, W_enc, b_enc, W0, b0, W1, b1):
  src_t = edge_index[0].reshape(NW, K, EB)
  dst_t = edge_index[1].reshape(NW, K, EB)
  src_d = edge_index[0].reshape(NW, KD, EBD)
  dst_d = edge_index[1].reshape(NW, KD, EBD)
  src_v = edge_index[0].reshape(NW, KV, EBV)
  dst_v = edge_index[1].reshape(NW, KV, EBV)

  dout_p, din_p = _deg_kernel(src_d, dst_d)
  dout = (dout_p[0] + dout_p[1])[:N, 0] + 1.0
  din = (din_p[0] + din_p[1])[:N, 0] + 1.0
  v = lax.rsqrt(dout)
  dinv = lax.rsqrt(din)
  dout_inv = 1.0 / dout
  col = lambda a: a.reshape(N, 1)

  v16 = jnp.zeros((NP, 16), _f32).at[:N, :].set(v[:, None])
  win_p, wout_p = _vpass_kernel(v16, src_v, dst_v)
  wsum = v * ((win_p[0] + win_p[1])[:N, 0] + (wout_p[0] + wout_p[1])[:N, 0])
  degsum = (dout - 1.0) + (din - 1.0)

  def e_rw(n, nf, cross):
    return 0.5 * (_vdot(degsum, n) - 2.0 * cross) / nf

  def e_sym(n, nf, cross):
    return 0.5 * (_vdot(n * dout_inv, wsum) - 2.0 * cross) / nf

  x0, n0c, xh0 = _make_enc_kernel()(x, W_enc, b_enc.reshape(1, 128), W0)
  zc0, zs0 = _make_scale_kernel()(x0, xh0, col(dinv), col(dout_inv))
  n0 = n0c[:, 0]
  nf0 = jnp.sum(n0)
  rc0, rsym0 = _make_block_kernel(2)(zc0, zs0, src_t, dst_t)

  st1 = _make_stage_kernel(False, True, True, 1)
  (h1, n1c, rs1, xh1, zc1, zs1, G1, cs1, x_sym0) = st1(
      rc0, xh0, b0.reshape(1, 128), col(dinv), col(dout_inv), W1, zs0, rsym0)
  n1 = n1c[:, 0]
  nf1 = jnp.sum(n1)
  e0 = e_sym(n0, nf0, x_sym0[0, 0])
  rc1, rrw1, rsym1 = _make_block_kernel(3)(zc1, h1, zs1, src_t, dst_t)
  rank1 = _rank_diff(h1, G1, rs1, cs1)

  st2 = _make_stage_kernel(False, False, True, 2)
  (h2, n2c, rs2, zs2, G2, cs2, x_rw1, x_sym1) = st2(
      rc1, xh1, b1.reshape(1, 128), col(dinv), col(dout_inv),
      h1, rrw1, zs1, rsym1)
  n2 = n2c[:, 0]
  nf2 = jnp.sum(n2)
  erw1 = e_rw(n1, nf1, x_rw1[0, 0])
  esym1 = e_sym(n1, nf1, x_sym1[0, 0])
  rrw2, rsym2 = _make_block_kernel(2)(h2, zs2, src_t, dst_t)
  rank2 = _rank_diff(h2, G2, rs2, cs2)
  x_rw2, x_sym2 = _cross2(h2, rrw2, zs2, rsym2)
  erw2 = e_rw(n2, nf2, x_rw2)
  esym2 = e_sym(n2, nf2, x_sym2)

  return (h2,
          jnp.stack([e0, erw1, erw2]),
          jnp.stack([esym1, esym2]),
          jnp.stack([rank1, rank2]))
